# Initial kernel scaffold; baseline (speedup 1.0000x reference)
import jax
import jax.numpy as jnp
from jax.experimental import pallas as pl


def kernel(x, y, z, edge_index, lg_edge_index, params):
    raise NotImplementedError("write your pallas kernel here")


# speedup vs baseline: 2.1678x; 2.1678x over previous
import functools

import jax
import jax.numpy as jnp
from jax import lax
from jax.experimental import pallas as pl
from jax.experimental.pallas import tpu as pltpu

D = 128
H = 64
BT = 32

EPS_BN = 1e-5
EPS_DIV = 1e-6


def _silu(v):
    return v * (1.0 / (1.0 + jnp.exp(-v)))


def _gram_body(x_ref, s_out, g_out, acc_s, acc_g):
    i = pl.program_id(0)

    @pl.when(i == 0)
    def _init():
        acc_s[...] = jnp.zeros_like(acc_s)
        acc_g[...] = jnp.zeros_like(acc_g)

    x = x_ref[...]
    acc_s[...] += jnp.sum(x, axis=0, keepdims=True)
    acc_g[...] += lax.dot_general(x, x, (((0,), (0,)), ((), ())),
                                  preferred_element_type=jnp.float32)

    @pl.when(i == pl.num_programs(0) - 1)
    def _fin():
        s_out[...] = acc_s[...]
        g_out[...] = acc_g[...]


def _gram_pass(x, R):
    M, Dx = x.shape
    n = M // R
    return pl.pallas_call(
        _gram_body,
        grid=(n,),
        in_specs=[pl.BlockSpec((R, Dx), lambda i: (i, 0))],
        out_specs=[pl.BlockSpec((1, Dx), lambda i: (0, 0)),
                   pl.BlockSpec((Dx, Dx), lambda i: (0, 0))],
        out_shape=[jax.ShapeDtypeStruct((1, Dx), jnp.float32),
                   jax.ShapeDtypeStruct((Dx, Dx), jnp.float32)],
        scratch_shapes=[pltpu.VMEM((1, Dx), jnp.float32),
                        pltpu.VMEM((Dx, Dx), jnp.float32)],
    )(x)


def _mid_body(x_ref, w_ref, b_ref, a_out, s_out, g_out, acc_s, acc_g):
    i = pl.program_id(0)

    @pl.when(i == 0)
    def _init():
        acc_s[...] = jnp.zeros_like(acc_s)
        acc_g[...] = jnp.zeros_like(acc_g)

    a = _silu(jnp.dot(x_ref[...], w_ref[...],
                      preferred_element_type=jnp.float32) + b_ref[...])
    a_out[...] = a
    acc_s[...] += jnp.sum(a, axis=0, keepdims=True)
    acc_g[...] += lax.dot_general(a, a, (((0,), (0,)), ((), ())),
                                  preferred_element_type=jnp.float32)

    @pl.when(i == pl.num_programs(0) - 1)
    def _fin():
        s_out[...] = acc_s[...]
        g_out[...] = acc_g[...]


def _mid_pass(x, Wf, bf, R):
    M, Dx = x.shape
    Hx = Wf.shape[1]
    n = M // R
    return pl.pallas_call(
        _mid_body,
        grid=(n,),
        in_specs=[pl.BlockSpec((R, Dx), lambda i: (i, 0)),
                  pl.BlockSpec((Dx, Hx), lambda i: (0, 0)),
                  pl.BlockSpec((1, Hx), lambda i: (0, 0))],
        out_specs=[pl.BlockSpec((R, Hx), lambda i: (i, 0)),
                   pl.BlockSpec((1, Hx), lambda i: (0, 0)),
                   pl.BlockSpec((Hx, Hx), lambda i: (0, 0))],
        out_shape=[jax.ShapeDtypeStruct((M, Hx), jnp.float32),
                   jax.ShapeDtypeStruct((1, Hx), jnp.float32),
                   jax.ShapeDtypeStruct((Hx, Hx), jnp.float32)],
        scratch_shapes=[pltpu.VMEM((1, Hx), jnp.float32),
                        pltpu.VMEM((Hx, Hx), jnp.float32)],
    )(x, Wf, bf)


def _apply_post_pass(a, W2f, b2f, posts, R):
    M, Hx = a.shape
    Bx = W2f.shape[1]
    n = M // R
    widths = [p.shape[1] for p in posts]
    npost = len(posts)

    def body(*refs):
        a_ref = refs[0]
        w_ref = refs[1]
        b_ref = refs[2]
        post_refs = refs[3:3 + npost]
        out_refs = refs[3 + npost:3 + 2 * npost]
        v = _silu(jnp.dot(a_ref[...], w_ref[...],
                          preferred_element_type=jnp.float32) + b_ref[...])
        for pr, orf in zip(post_refs, out_refs):
            orf[...] = jnp.dot(v, pr[...], preferred_element_type=jnp.float32)

    in_specs = [pl.BlockSpec((R, Hx), lambda i: (i, 0)),
                pl.BlockSpec((Hx, Bx), lambda i: (0, 0)),
                pl.BlockSpec((1, Bx), lambda i: (0, 0))]
    for w in widths:
        in_specs.append(pl.BlockSpec((Bx, w), lambda i: (0, 0)))
    out_specs = [pl.BlockSpec((R, w), lambda i: (i, 0)) for w in widths]
    out_shape = [jax.ShapeDtypeStruct((M, w), jnp.float32) for w in widths]
    outs = pl.pallas_call(
        body,
        grid=(n,),
        in_specs=in_specs,
        out_specs=out_specs,
        out_shape=out_shape,
    )(a, W2f, b2f, *posts)
    return outs


def _ewstats_body(t_ref, sc_ref, sh_ref, v_out, s_out, g_out, acc_s, acc_g):
    i = pl.program_id(0)

    @pl.when(i == 0)
    def _init():
        acc_s[...] = jnp.zeros_like(acc_s)
        acc_g[...] = jnp.zeros_like(acc_g)

    v = _silu(t_ref[...] * sc_ref[...] + sh_ref[...])
    v_out[...] = v
    acc_s[...] += jnp.sum(v, axis=0, keepdims=True)
    acc_g[...] += lax.dot_general(v, v, (((0,), (0,)), ((), ())),
                                  preferred_element_type=jnp.float32)

    @pl.when(i == pl.num_programs(0) - 1)
    def _fin():
        s_out[...] = acc_s[...]
        g_out[...] = acc_g[...]


def _ewstats_pass(t, scale, shift, R):
    M, Bx = t.shape
    n = M // R
    return pl.pallas_call(
        _ewstats_body,
        grid=(n,),
        in_specs=[pl.BlockSpec((R, Bx), lambda i: (i, 0)),
                  pl.BlockSpec((1, Bx), lambda i: (0, 0)),
                  pl.BlockSpec((1, Bx), lambda i: (0, 0))],
        out_specs=[pl.BlockSpec((R, Bx), lambda i: (i, 0)),
                   pl.BlockSpec((1, Bx), lambda i: (0, 0)),
                   pl.BlockSpec((Bx, Bx), lambda i: (0, 0))],
        out_shape=[jax.ShapeDtypeStruct((M, Bx), jnp.float32),
                   jax.ShapeDtypeStruct((1, Bx), jnp.float32),
                   jax.ShapeDtypeStruct((Bx, Bx), jnp.float32)],
        scratch_shapes=[pltpu.VMEM((1, Bx), jnp.float32),
                        pltpu.VMEM((Bx, Bx), jnp.float32)],
    )(t, scale, shift)


def _expand_body(v_ref, w_ref, b_ref, r_ref, o_ref):
    o_ref[...] = _silu(jnp.dot(v_ref[...], w_ref[...],
                               preferred_element_type=jnp.float32)
                       + b_ref[...]) + r_ref[...]


def _expand_pass(v, Wf, bf, resid, R):
    M, Bx = v.shape
    Dx = Wf.shape[1]
    n = M // R
    return pl.pallas_call(
        _expand_body,
        grid=(n,),
        in_specs=[pl.BlockSpec((R, Bx), lambda i: (i, 0)),
                  pl.BlockSpec((Bx, Dx), lambda i: (0, 0)),
                  pl.BlockSpec((1, Dx), lambda i: (0, 0)),
                  pl.BlockSpec((R, Dx), lambda i: (i, 0))],
        out_specs=pl.BlockSpec((R, Dx), lambda i: (i, 0)),
        out_shape=jax.ShapeDtypeStruct((M, Dx), jnp.float32),
    )(v, Wf, bf, resid)


def _tw_body(t_ref, sc_ref, sh_ref, w_ref, o_ref):
    m = _silu(t_ref[...] * sc_ref[...] + sh_ref[...])
    o_ref[...] = jnp.dot(m, w_ref[...], preferred_element_type=jnp.float32)


def _tw_pass(t, scale, shift, Weg, R):
    M, Bx = t.shape
    n = M // R
    return pl.pallas_call(
        _tw_body,
        grid=(n,),
        in_specs=[pl.BlockSpec((R, Bx), lambda i: (i, 0)),
                  pl.BlockSpec((1, Bx), lambda i: (0, 0)),
                  pl.BlockSpec((1, Bx), lambda i: (0, 0)),
                  pl.BlockSpec((Bx, Bx), lambda i: (0, 0))],
        out_specs=pl.BlockSpec((R, Bx), lambda i: (i, 0)),
        out_shape=jax.ShapeDtypeStruct((M, Bx), jnp.float32),
    )(t, scale, shift, Weg)


def _fold_linear_bn(S, G, M, W, b, g, be):
    mu_in = S / M
    mean = mu_in @ W + b
    ex2 = jnp.sum(W * (G @ W), axis=0) / M
    var = ex2 - jnp.square(mu_in @ W)[0]
    scale = g / jnp.sqrt(var + EPS_BN)
    Wf = W * scale[None, :]
    bf = (b - mean[0]) * scale + be
    return Wf, bf[None, :]


def _bn_scale_shift(s, ss, M, g, be):
    mean = s / M
    var = ss / M - jnp.square(mean)
    scale = g / jnp.sqrt(var + EPS_BN)
    shift = be - mean * scale
    return scale, shift


def _bottleneck_heads(x, q, R, posts):
    M = x.shape[0]
    S1, G1 = _gram_pass(x, R)
    W1f, b1f = _fold_linear_bn(S1, G1, M, q["W1"], q["b1"], q["g1"], q["be1"])
    a, S2, G2 = _mid_pass(x, W1f, b1f, R)
    W2f, b2f = _fold_linear_bn(S2, G2, M, q["W2"], q["b2"], q["g2"], q["be2"])
    return _apply_post_pass(a, W2f, b2f, posts, R)


def _sparse_middle_lg(src_tab, dst_tab, zweg, src_l, dst_l):
    E = src_tab.shape[0]
    g_src = src_tab[src_l]
    m = g_src[:, :BT] + dst_tab[dst_l] + zweg
    sigma = 1.0 / (1.0 + jnp.exp(-m))
    v = g_src[:, BT:] * sigma
    ssh = jax.ops.segment_sum(v, dst_l, num_segments=E)
    ss = jax.ops.segment_sum(sigma, dst_l, num_segments=E)
    s = jnp.sum(m, axis=0)
    ssq = jnp.sum(m * m, axis=0)
    return m, ssh, ss, s, ssq


def kernel(x, y, z, edge_index, lg_edge_index, params):
    N = x.shape[0]
    E = y.shape[0]
    EL = z.shape[0]
    R = 2000
    p = params

    qy = p["edge_upd"]
    src_tab_y, dst_tab_y, u_y = _bottleneck_heads(
        y, p["pair_bn"], R,
        [jnp.concatenate([qy["Wsg"], qy["Wdu"]], axis=1), qy["Wdg"], qy["Wsu"]])
    (zweg,) = _bottleneck_heads(z, p["trip_bn"], R, [qy["Weg"]])
    qx = p["node_upd"]
    src_tab_x, dst_tab_x, u_x = _bottleneck_heads(
        x, p["node_bn"], R,
        [jnp.concatenate([qx["Wsg"], qx["Wdu"]], axis=1), qx["Wdg"], qx["Wsu"]])

    src_l, dst_l = lg_edge_index[0], lg_edge_index[1]
    m_l, ssh_l, ss_l, s_ml, ssq_ml = _sparse_middle_lg(
        src_tab_y, dst_tab_y, zweg, src_l, dst_l)

    t = u_y + ssh_l / (ss_l + EPS_DIV)
    St, Gt = _gram_pass(t, R)
    sc_t, sh_t = _bn_scale_shift(St[0], jnp.diag(Gt), E, qy["gn"], qy["bn"])
    w = _tw_pass(t, sc_t[None, :], sh_t[None, :], qx["Weg"], R)

    sc_m, sh_m = _bn_scale_shift(s_ml, ssq_ml, EL, qy["ge"], qy["be"])
    z2, Sz2, Gz2 = _ewstats_pass(m_l, sc_m[None, :], sh_m[None, :], R)

    src_g, dst_g = edge_index[0], edge_index[1]
    m2, ssh_g, ss_g, s_m2, ssq_m2 = _sparse_middle_lg(
        src_tab_x, dst_tab_x, w, src_g, dst_g)

    t2 = u_x + ssh_g / (ss_g + EPS_DIV)
    St2, Gt2 = _gram_pass(t2, R)
    sc_t2, sh_t2 = _bn_scale_shift(St2[0], jnp.diag(Gt2), N, qx["gn"], qx["bn"])
    x2, Sx2, Gx2 = _ewstats_pass(t2, sc_t2[None, :], sh_t2[None, :], R)

    sc_m2, sh_m2 = _bn_scale_shift(s_m2, ssq_m2, E, qx["ge"], qx["be"])
    y2, Sy2, Gy2 = _ewstats_pass(m2, sc_m2[None, :], sh_m2[None, :], R)

    qe = p["node_ex"]
    Wxf, bxf = _fold_linear_bn(Sx2, Gx2, N, qe["W"], qe["b"], qe["g"], qe["be"])
    xo = _expand_pass(x2, Wxf, bxf, x, R)
    qe = p["pair_ex"]
    Wyf, byf = _fold_linear_bn(Sy2, Gy2, E, qe["W"], qe["b"], qe["g"], qe["be"])
    yo = _expand_pass(y2, Wyf, byf, y, R)
    qe = p["trip_ex"]
    Wzf, bzf = _fold_linear_bn(Sz2, Gz2, EL, qe["W"], qe["b"], qe["g"], qe["be"])
    zo = _expand_pass(z2, Wzf, bzf, z, R)

    return xo, yo, zo


# speedup vs baseline: 4.5433x; 2.0959x over previous
import functools

import jax
import jax.numpy as jnp
from jax import lax
from jax.experimental import pallas as pl
from jax.experimental.pallas import tpu as pltpu
from jax.experimental.pallas import tpu_sc as plsc

D = 128
H = 64
BT = 32

NC = 2
NS = 16
L = 16
NW = NC * NS

EPS_BN = 1e-5
EPS_DIV = 1e-6


def _silu(v):
    return v * (1.0 / (1.0 + jnp.exp(-v)))


def _gram_body(x_ref, s_out, g_out, acc_s, acc_g):
    i = pl.program_id(0)

    @pl.when(i == 0)
    def _init():
        acc_s[...] = jnp.zeros_like(acc_s)
        acc_g[...] = jnp.zeros_like(acc_g)

    x = x_ref[...]
    acc_s[...] += jnp.sum(x, axis=0, keepdims=True)
    acc_g[...] += lax.dot_general(x, x, (((0,), (0,)), ((), ())),
                                  preferred_element_type=jnp.float32)

    @pl.when(i == pl.num_programs(0) - 1)
    def _fin():
        s_out[...] = acc_s[...]
        g_out[...] = acc_g[...]


def _gram_pass(x, R):
    M, Dx = x.shape
    n = M // R
    return pl.pallas_call(
        _gram_body,
        grid=(n,),
        in_specs=[pl.BlockSpec((R, Dx), lambda i: (i, 0))],
        out_specs=[pl.BlockSpec((1, Dx), lambda i: (0, 0)),
                   pl.BlockSpec((Dx, Dx), lambda i: (0, 0))],
        out_shape=[jax.ShapeDtypeStruct((1, Dx), jnp.float32),
                   jax.ShapeDtypeStruct((Dx, Dx), jnp.float32)],
        scratch_shapes=[pltpu.VMEM((1, Dx), jnp.float32),
                        pltpu.VMEM((Dx, Dx), jnp.float32)],
    )(x)


def _mid_body(x_ref, w_ref, b_ref, a_out, s_out, g_out, acc_s, acc_g):
    i = pl.program_id(0)

    @pl.when(i == 0)
    def _init():
        acc_s[...] = jnp.zeros_like(acc_s)
        acc_g[...] = jnp.zeros_like(acc_g)

    a = _silu(jnp.dot(x_ref[...], w_ref[...],
                      preferred_element_type=jnp.float32) + b_ref[...])
    a_out[...] = a
    acc_s[...] += jnp.sum(a, axis=0, keepdims=True)
    acc_g[...] += lax.dot_general(a, a, (((0,), (0,)), ((), ())),
                                  preferred_element_type=jnp.float32)

    @pl.when(i == pl.num_programs(0) - 1)
    def _fin():
        s_out[...] = acc_s[...]
        g_out[...] = acc_g[...]


def _mid_pass(x, Wf, bf, R):
    M, Dx = x.shape
    Hx = Wf.shape[1]
    n = M // R
    return pl.pallas_call(
        _mid_body,
        grid=(n,),
        in_specs=[pl.BlockSpec((R, Dx), lambda i: (i, 0)),
                  pl.BlockSpec((Dx, Hx), lambda i: (0, 0)),
                  pl.BlockSpec((1, Hx), lambda i: (0, 0))],
        out_specs=[pl.BlockSpec((R, Hx), lambda i: (i, 0)),
                   pl.BlockSpec((1, Hx), lambda i: (0, 0)),
                   pl.BlockSpec((Hx, Hx), lambda i: (0, 0))],
        out_shape=[jax.ShapeDtypeStruct((M, Hx), jnp.float32),
                   jax.ShapeDtypeStruct((1, Hx), jnp.float32),
                   jax.ShapeDtypeStruct((Hx, Hx), jnp.float32)],
        scratch_shapes=[pltpu.VMEM((1, Hx), jnp.float32),
                        pltpu.VMEM((Hx, Hx), jnp.float32)],
    )(x, Wf, bf)


def _apply_post_pass(a, W2f, b2f, posts, R):
    M, Hx = a.shape
    Bx = W2f.shape[1]
    n = M // R
    widths = [p.shape[1] for p in posts]
    npost = len(posts)

    def body(*refs):
        a_ref = refs[0]
        w_ref = refs[1]
        b_ref = refs[2]
        post_refs = refs[3:3 + npost]
        out_refs = refs[3 + npost:3 + 2 * npost]
        v = _silu(jnp.dot(a_ref[...], w_ref[...],
                          preferred_element_type=jnp.float32) + b_ref[...])
        for pr, orf in zip(post_refs, out_refs):
            orf[...] = jnp.dot(v, pr[...], preferred_element_type=jnp.float32)

    in_specs = [pl.BlockSpec((R, Hx), lambda i: (i, 0)),
                pl.BlockSpec((Hx, Bx), lambda i: (0, 0)),
                pl.BlockSpec((1, Bx), lambda i: (0, 0))]
    for w in widths:
        in_specs.append(pl.BlockSpec((Bx, w), lambda i: (0, 0)))
    out_specs = [pl.BlockSpec((R, w), lambda i: (i, 0)) for w in widths]
    out_shape = [jax.ShapeDtypeStruct((M, w), jnp.float32) for w in widths]
    outs = pl.pallas_call(
        body,
        grid=(n,),
        in_specs=in_specs,
        out_specs=out_specs,
        out_shape=out_shape,
    )(a, W2f, b2f, *posts)
    return outs


def _ewstats_body(t_ref, sc_ref, sh_ref, v_out, s_out, g_out, acc_s, acc_g):
    i = pl.program_id(0)

    @pl.when(i == 0)
    def _init():
        acc_s[...] = jnp.zeros_like(acc_s)
        acc_g[...] = jnp.zeros_like(acc_g)

    v = _silu(t_ref[...] * sc_ref[...] + sh_ref[...])
    v_out[...] = v
    acc_s[...] += jnp.sum(v, axis=0, keepdims=True)
    acc_g[...] += lax.dot_general(v, v, (((0,), (0,)), ((), ())),
                                  preferred_element_type=jnp.float32)

    @pl.when(i == pl.num_programs(0) - 1)
    def _fin():
        s_out[...] = acc_s[...]
        g_out[...] = acc_g[...]


def _ewstats_pass(t, scale, shift, R):
    M, Bx = t.shape
    n = M // R
    return pl.pallas_call(
        _ewstats_body,
        grid=(n,),
        in_specs=[pl.BlockSpec((R, Bx), lambda i: (i, 0)),
                  pl.BlockSpec((1, Bx), lambda i: (0, 0)),
                  pl.BlockSpec((1, Bx), lambda i: (0, 0))],
        out_specs=[pl.BlockSpec((R, Bx), lambda i: (i, 0)),
                   pl.BlockSpec((1, Bx), lambda i: (0, 0)),
                   pl.BlockSpec((Bx, Bx), lambda i: (0, 0))],
        out_shape=[jax.ShapeDtypeStruct((M, Bx), jnp.float32),
                   jax.ShapeDtypeStruct((1, Bx), jnp.float32),
                   jax.ShapeDtypeStruct((Bx, Bx), jnp.float32)],
        scratch_shapes=[pltpu.VMEM((1, Bx), jnp.float32),
                        pltpu.VMEM((Bx, Bx), jnp.float32)],
    )(t, scale, shift)


def _expand_body(v_ref, w_ref, b_ref, r_ref, o_ref):
    o_ref[...] = _silu(jnp.dot(v_ref[...], w_ref[...],
                               preferred_element_type=jnp.float32)
                       + b_ref[...]) + r_ref[...]


def _expand_pass(v, Wf, bf, resid, R):
    M, Bx = v.shape
    Dx = Wf.shape[1]
    n = M // R
    return pl.pallas_call(
        _expand_body,
        grid=(n,),
        in_specs=[pl.BlockSpec((R, Bx), lambda i: (i, 0)),
                  pl.BlockSpec((Bx, Dx), lambda i: (0, 0)),
                  pl.BlockSpec((1, Dx), lambda i: (0, 0)),
                  pl.BlockSpec((R, Dx), lambda i: (i, 0))],
        out_specs=pl.BlockSpec((R, Dx), lambda i: (i, 0)),
        out_shape=jax.ShapeDtypeStruct((M, Dx), jnp.float32),
    )(v, Wf, bf, resid)


def _tw_body(t_ref, sc_ref, sh_ref, w_ref, o_ref):
    m = _silu(t_ref[...] * sc_ref[...] + sh_ref[...])
    o_ref[...] = jnp.dot(m, w_ref[...], preferred_element_type=jnp.float32)


def _tw_pass(t, scale, shift, Weg, R):
    M, Bx = t.shape
    n = M // R
    return pl.pallas_call(
        _tw_body,
        grid=(n,),
        in_specs=[pl.BlockSpec((R, Bx), lambda i: (i, 0)),
                  pl.BlockSpec((1, Bx), lambda i: (0, 0)),
                  pl.BlockSpec((1, Bx), lambda i: (0, 0)),
                  pl.BlockSpec((Bx, Bx), lambda i: (0, 0))],
        out_specs=pl.BlockSpec((R, Bx), lambda i: (i, 0)),
        out_shape=jax.ShapeDtypeStruct((M, Bx), jnp.float32),
    )(t, scale, shift, Weg)


def _fold_linear_bn(S, G, M, W, b, g, be):
    mu_in = S / M
    mean = mu_in @ W + b
    ex2 = jnp.sum(W * (G @ W), axis=0) / M
    var = ex2 - jnp.square(mu_in @ W)[0]
    scale = g / jnp.sqrt(var + EPS_BN)
    Wf = W * scale[None, :]
    bf = (b - mean[0]) * scale + be
    return Wf, bf[None, :]


def _bn_scale_shift(s, ss, M, g, be):
    mean = s / M
    var = ss / M - jnp.square(mean)
    scale = g / jnp.sqrt(var + EPS_BN)
    shift = be - mean * scale
    return scale, shift


def _bottleneck_heads(x, q, R, posts):
    M = x.shape[0]
    S1, G1 = _gram_pass(x, R)
    W1f, b1f = _fold_linear_bn(S1, G1, M, q["W1"], q["b1"], q["g1"], q["be1"])
    a, S2, G2 = _mid_pass(x, W1f, b1f, R)
    W2f, b2f = _fold_linear_bn(S2, G2, M, q["W2"], q["b2"], q["g2"], q["be2"])
    return _apply_post_pass(a, W2f, b2f, posts, R)


def _sc_edge_gate(src_idx, dst_idx, src_tab, dst_tab, ew, CH=80):
    EL = src_idx.shape[0]
    per_w = EL // NW
    nch = per_w // CH
    mesh = plsc.VectorSubcoreMesh(core_axis_name="c", subcore_axis_name="s")

    @functools.partial(
        pl.kernel,
        out_type=[jax.ShapeDtypeStruct((EL, BT), jnp.float32),
                  jax.ShapeDtypeStruct((EL, 2 * BT), jnp.float32),
                  jax.ShapeDtypeStruct((NW, 2, BT), jnp.float32)],
        mesh=mesh,
        compiler_params=pltpu.CompilerParams(use_tc_tiling_on_sc=False),
        scratch_types=[
            pltpu.VMEM((CH,), jnp.int32),
            pltpu.VMEM((CH,), jnp.int32),
            pltpu.VMEM((CH, 2 * BT), jnp.float32),
            pltpu.VMEM((CH, BT), jnp.float32),
            pltpu.VMEM((CH, BT), jnp.float32),
            pltpu.VMEM((CH, BT), jnp.float32),
            pltpu.VMEM((CH, 2 * BT), jnp.float32),
            pltpu.VMEM((2, BT), jnp.float32),
            pltpu.SemaphoreType.DMA,
            pltpu.SemaphoreType.DMA,
            pltpu.SemaphoreType.DMA,
        ],
    )
    def k(src_hbm, dst_hbm, stab_hbm, dtab_hbm, ew_hbm,
          m_hbm, p_hbm, st_hbm,
          idxs_v, idxd_v, srows_v, drows_v, erows_v, mbuf_v, pbuf_v, stat_v,
          sem1, sem2, sem3):
        wid = lax.axis_index("s") * NC + lax.axis_index("c")
        base_w = wid * per_w
        zero = jnp.zeros((L,), jnp.float32)
        for r in range(2):
            for cv in range(BT // L):
                stat_v[r, pl.ds(cv * L, L)] = zero

        def chunk_body(ci, _):
            base = base_w + ci * CH
            pltpu.sync_copy(src_hbm.at[pl.ds(base, CH)], idxs_v)
            pltpu.sync_copy(dst_hbm.at[pl.ds(base, CH)], idxd_v)
            cp1 = pltpu.async_copy(stab_hbm.at[idxs_v], srows_v, sem1)
            cp2 = pltpu.async_copy(dtab_hbm.at[idxd_v], drows_v, sem2)
            cp3 = pltpu.async_copy(ew_hbm.at[pl.ds(base, CH)], erows_v, sem3)
            cp1.wait()
            cp2.wait()
            cp3.wait()

            def row_body(j, carry):
                accs = list(carry)
                for h in range(BT // L):
                    es = srows_v[j, pl.ds(L * h, L)]
                    bh = srows_v[j, pl.ds(BT + L * h, L)]
                    ed = drows_v[j, pl.ds(L * h, L)]
                    zw = erows_v[j, pl.ds(L * h, L)]
                    m = es + ed + zw
                    sg = 1.0 / (1.0 + jnp.exp(-m))
                    v = sg * bh
                    mbuf_v[j, pl.ds(L * h, L)] = m
                    pbuf_v[j, pl.ds(L * h, L)] = v
                    pbuf_v[j, pl.ds(BT + L * h, L)] = sg
                    accs[h] = accs[h] + m
                    accs[2 + h] = accs[2 + h] + m * m
                return tuple(accs)

            accs = lax.fori_loop(0, CH, row_body, (zero, zero, zero, zero))
            for h in range(BT // L):
                stat_v[0, pl.ds(L * h, L)] += accs[h]
                stat_v[1, pl.ds(L * h, L)] += accs[2 + h]
            pltpu.sync_copy(mbuf_v, m_hbm.at[pl.ds(base, CH)])
            pltpu.sync_copy(pbuf_v, p_hbm.at[pl.ds(base, CH)])
            return 0

        lax.fori_loop(0, nch, chunk_body, 0)
        pltpu.sync_copy(stat_v, st_hbm.at[wid])

    return k(src_idx, dst_idx, src_tab, dst_tab, ew)


def _sc_node_conv(src_idx, dst_idx, src_tab, dst_tab, ew, NPAD, CH=80):
    E = src_idx.shape[0]
    per_w = E // NW
    nch = per_w // CH
    stripe = NPAD // NS
    nsch = stripe // CH
    mesh = plsc.VectorSubcoreMesh(core_axis_name="c", subcore_axis_name="s")

    @functools.partial(
        pl.kernel,
        out_type=[jax.ShapeDtypeStruct((E, BT), jnp.float32),
                  jax.ShapeDtypeStruct((NC, NPAD, 2 * BT), jnp.float32),
                  jax.ShapeDtypeStruct((NW, 2, BT), jnp.float32)],
        mesh=mesh,
        compiler_params=pltpu.CompilerParams(use_tc_tiling_on_sc=False),
        scratch_types=[
            pltpu.VMEM((CH,), jnp.int32),
            pltpu.VMEM((CH,), jnp.int32),
            pltpu.VMEM((CH, 2 * BT), jnp.float32),
            pltpu.VMEM((CH, BT), jnp.float32),
            pltpu.VMEM((CH, BT), jnp.float32),
            pltpu.VMEM((CH, BT), jnp.float32),
            pltpu.VMEM((CH, 2 * BT), jnp.float32),
            pltpu.VMEM((2, BT), jnp.float32),
            pltpu.VMEM_SHARED((NPAD, 2 * BT), jnp.float32),
            pltpu.SemaphoreType.DMA,
            pltpu.SemaphoreType.DMA,
            pltpu.SemaphoreType.DMA,
        ],
    )
    def k(src_hbm, dst_hbm, stab_hbm, dtab_hbm, ew_hbm,
          m_hbm, dump_hbm, st_hbm,
          idxs_v, idxd_v, srows_v, drows_v, erows_v, mbuf_v, pbuf_v, stat_v,
          acc_sh, sem1, sem2, sem3):
        cid = lax.axis_index("c")
        sid = lax.axis_index("s")
        wid = sid * NC + cid
        base_w = wid * per_w
        zero = jnp.zeros((L,), jnp.float32)

        def zrow(j, _):
            for h in range(2 * BT // L):
                pbuf_v[j, pl.ds(L * h, L)] = zero
            return 0
        lax.fori_loop(0, CH, zrow, 0)

        def zchunk(c, _):
            pltpu.sync_copy(pbuf_v, acc_sh.at[pl.ds(sid * stripe + c * CH, CH)])
            return 0
        lax.fori_loop(0, nsch, zchunk, 0)
        for r in range(2):
            for cv in range(BT // L):
                stat_v[r, pl.ds(cv * L, L)] = zero
        plsc.subcore_barrier()

        def chunk_body(ci, _):
            base = base_w + ci * CH
            pltpu.sync_copy(src_hbm.at[pl.ds(base, CH)], idxs_v)
            pltpu.sync_copy(dst_hbm.at[pl.ds(base, CH)], idxd_v)
            cp1 = pltpu.async_copy(stab_hbm.at[idxs_v], srows_v, sem1)
            cp2 = pltpu.async_copy(dtab_hbm.at[idxd_v], drows_v, sem2)
            cp3 = pltpu.async_copy(ew_hbm.at[pl.ds(base, CH)], erows_v, sem3)
            cp1.wait()
            cp2.wait()
            cp3.wait()

            def row_body(j, carry):
                accs = list(carry)
                for h in range(BT // L):
                    es = srows_v[j, pl.ds(L * h, L)]
                    bh = srows_v[j, pl.ds(BT + L * h, L)]
                    ed = drows_v[j, pl.ds(L * h, L)]
                    zw = erows_v[j, pl.ds(L * h, L)]
                    m = es + ed + zw
                    sg = 1.0 / (1.0 + jnp.exp(-m))
                    v = sg * bh
                    mbuf_v[j, pl.ds(L * h, L)] = m
                    pbuf_v[j, pl.ds(L * h, L)] = v
                    pbuf_v[j, pl.ds(BT + L * h, L)] = sg
                    accs[h] = accs[h] + m
                    accs[2 + h] = accs[2 + h] + m * m
                return tuple(accs)

            accs = lax.fori_loop(0, CH, row_body, (zero, zero, zero, zero))
            for h in range(BT // L):
                stat_v[0, pl.ds(L * h, L)] += accs[h]
                stat_v[1, pl.ds(L * h, L)] += accs[2 + h]
            pltpu.sync_copy(mbuf_v, m_hbm.at[pl.ds(base, CH)])
            pltpu.sync_copy(pbuf_v, acc_sh.at[idxd_v], add=True)
            return 0

        lax.fori_loop(0, nch, chunk_body, 0)
        pltpu.sync_copy(stat_v, st_hbm.at[wid])
        plsc.subcore_barrier()

        def dchunk(c, _):
            off = sid * stripe + c * CH
            pltpu.sync_copy(acc_sh.at[pl.ds(off, CH)], pbuf_v)
            pltpu.sync_copy(pbuf_v, dump_hbm.at[cid, pl.ds(off, CH)])
            return 0
        lax.fori_loop(0, nsch, dchunk, 0)

    return k(src_idx, dst_idx, src_tab, dst_tab, ew)


def _combine_body(a0_ref, a1_ref, u_ref, t_ref):
    ah = a0_ref[:, :BT] + a1_ref[:, :BT]
    as_ = a0_ref[:, BT:] + a1_ref[:, BT:]
    t_ref[...] = u_ref[...] + ah / (as_ + EPS_DIV)


def _combine_pass(a0, a1, u, R):
    M = u.shape[0]
    n = M // R
    return pl.pallas_call(
        _combine_body,
        grid=(n,),
        in_specs=[pl.BlockSpec((R, 2 * BT), lambda i: (i, 0)),
                  pl.BlockSpec((R, 2 * BT), lambda i: (i, 0)),
                  pl.BlockSpec((R, BT), lambda i: (i, 0))],
        out_specs=pl.BlockSpec((R, BT), lambda i: (i, 0)),
        out_shape=jax.ShapeDtypeStruct((M, BT), jnp.float32),
    )(a0, a1, u)


def _sparse_middle_lg(src_tab, dst_tab, zweg, src_l, dst_l):
    E = src_tab.shape[0]
    m, pay, st = _sc_edge_gate(src_l, dst_l, src_tab, dst_tab, zweg)
    sts = jnp.sum(st, axis=0)
    ssh = jax.ops.segment_sum(pay[:, :BT], dst_l, num_segments=E)
    ss = jax.ops.segment_sum(pay[:, BT:], dst_l, num_segments=E)
    return m, ssh, ss, sts[0], sts[1]


def kernel(x, y, z, edge_index, lg_edge_index, params):
    N = x.shape[0]
    E = y.shape[0]
    EL = z.shape[0]
    R = 2000
    p = params

    qy = p["edge_upd"]
    src_tab_y, dst_tab_y, u_y = _bottleneck_heads(
        y, p["pair_bn"], R,
        [jnp.concatenate([qy["Wsg"], qy["Wdu"]], axis=1), qy["Wdg"], qy["Wsu"]])
    (zweg,) = _bottleneck_heads(z, p["trip_bn"], R, [qy["Weg"]])
    qx = p["node_upd"]
    src_tab_x, dst_tab_x, u_x = _bottleneck_heads(
        x, p["node_bn"], R,
        [jnp.concatenate([qx["Wsg"], qx["Wdu"]], axis=1), qx["Wdg"], qx["Wsu"]])

    src_l, dst_l = lg_edge_index[0], lg_edge_index[1]
    m_l, ssh_l, ss_l, s_ml, ssq_ml = _sparse_middle_lg(
        src_tab_y, dst_tab_y, zweg, src_l, dst_l)

    t = u_y + ssh_l / (ss_l + EPS_DIV)
    St, Gt = _gram_pass(t, R)
    sc_t, sh_t = _bn_scale_shift(St[0], jnp.diag(Gt), E, qy["gn"], qy["bn"])
    w = _tw_pass(t, sc_t[None, :], sh_t[None, :], qx["Weg"], R)

    sc_m, sh_m = _bn_scale_shift(s_ml, ssq_ml, EL, qy["ge"], qy["be"])
    z2, Sz2, Gz2 = _ewstats_pass(m_l, sc_m[None, :], sh_m[None, :], R)

    src_g, dst_g = edge_index[0], edge_index[1]
    NPAD = 10240
    m2, dump, st2 = _sc_node_conv(src_g, dst_g, src_tab_x, dst_tab_x, w, NPAD)
    st2s = jnp.sum(st2, axis=0)
    s_m2, ssq_m2 = st2s[0], st2s[1]

    t2 = _combine_pass(dump[0, :N], dump[1, :N], u_x, 2000)
    St2, Gt2 = _gram_pass(t2, R)
    sc_t2, sh_t2 = _bn_scale_shift(St2[0], jnp.diag(Gt2), N, qx["gn"], qx["bn"])
    x2, Sx2, Gx2 = _ewstats_pass(t2, sc_t2[None, :], sh_t2[None, :], R)

    sc_m2, sh_m2 = _bn_scale_shift(s_m2, ssq_m2, E, qx["ge"], qx["be"])
    y2, Sy2, Gy2 = _ewstats_pass(m2, sc_m2[None, :], sh_m2[None, :], R)

    qe = p["node_ex"]
    Wxf, bxf = _fold_linear_bn(Sx2, Gx2, N, qe["W"], qe["b"], qe["g"], qe["be"])
    xo = _expand_pass(x2, Wxf, bxf, x, R)
    qe = p["pair_ex"]
    Wyf, byf = _fold_linear_bn(Sy2, Gy2, E, qe["W"], qe["b"], qe["g"], qe["be"])
    yo = _expand_pass(y2, Wyf, byf, y, R)
    qe = p["trip_ex"]
    Wzf, bzf = _fold_linear_bn(Sz2, Gz2, EL, qe["W"], qe["b"], qe["g"], qe["be"])
    zo = _expand_pass(z2, Wzf, bzf, z, R)

    return xo, yo, zo


# speedup vs baseline: 5.8696x; 1.2919x over previous
import functools

import jax
import jax.numpy as jnp
from jax import lax
from jax.experimental import pallas as pl
from jax.experimental.pallas import tpu as pltpu
from jax.experimental.pallas import tpu_sc as plsc

D = 128
H = 64
BT = 32

NC = 2
NS = 16
L = 16
NW = NC * NS

EPS_BN = 1e-5
EPS_DIV = 1e-6


def _silu(v):
    return v * (1.0 / (1.0 + jnp.exp(-v)))


def _gram_body(x_ref, s_out, g_out, acc_s, acc_g):
    i = pl.program_id(0)

    @pl.when(i == 0)
    def _init():
        acc_s[...] = jnp.zeros_like(acc_s)
        acc_g[...] = jnp.zeros_like(acc_g)

    x = x_ref[...]
    acc_s[...] += jnp.sum(x, axis=0, keepdims=True)
    acc_g[...] += lax.dot_general(x, x, (((0,), (0,)), ((), ())),
                                  preferred_element_type=jnp.float32)

    @pl.when(i == pl.num_programs(0) - 1)
    def _fin():
        s_out[...] = acc_s[...]
        g_out[...] = acc_g[...]


def _gram_pass(x, R):
    M, Dx = x.shape
    n = M // R
    return pl.pallas_call(
        _gram_body,
        grid=(n,),
        in_specs=[pl.BlockSpec((R, Dx), lambda i: (i, 0))],
        out_specs=[pl.BlockSpec((1, Dx), lambda i: (0, 0)),
                   pl.BlockSpec((Dx, Dx), lambda i: (0, 0))],
        out_shape=[jax.ShapeDtypeStruct((1, Dx), jnp.float32),
                   jax.ShapeDtypeStruct((Dx, Dx), jnp.float32)],
        scratch_shapes=[pltpu.VMEM((1, Dx), jnp.float32),
                        pltpu.VMEM((Dx, Dx), jnp.float32)],
    )(x)


def _mid_body(x_ref, w_ref, b_ref, a_out, s_out, g_out, acc_s, acc_g):
    i = pl.program_id(0)

    @pl.when(i == 0)
    def _init():
        acc_s[...] = jnp.zeros_like(acc_s)
        acc_g[...] = jnp.zeros_like(acc_g)

    a = _silu(jnp.dot(x_ref[...], w_ref[...],
                      preferred_element_type=jnp.float32) + b_ref[...])
    a_out[...] = a
    acc_s[...] += jnp.sum(a, axis=0, keepdims=True)
    acc_g[...] += lax.dot_general(a, a, (((0,), (0,)), ((), ())),
                                  preferred_element_type=jnp.float32)

    @pl.when(i == pl.num_programs(0) - 1)
    def _fin():
        s_out[...] = acc_s[...]
        g_out[...] = acc_g[...]


def _mid_pass(x, Wf, bf, R):
    M, Dx = x.shape
    Hx = Wf.shape[1]
    n = M // R
    return pl.pallas_call(
        _mid_body,
        grid=(n,),
        in_specs=[pl.BlockSpec((R, Dx), lambda i: (i, 0)),
                  pl.BlockSpec((Dx, Hx), lambda i: (0, 0)),
                  pl.BlockSpec((1, Hx), lambda i: (0, 0))],
        out_specs=[pl.BlockSpec((R, Hx), lambda i: (i, 0)),
                   pl.BlockSpec((1, Hx), lambda i: (0, 0)),
                   pl.BlockSpec((Hx, Hx), lambda i: (0, 0))],
        out_shape=[jax.ShapeDtypeStruct((M, Hx), jnp.float32),
                   jax.ShapeDtypeStruct((1, Hx), jnp.float32),
                   jax.ShapeDtypeStruct((Hx, Hx), jnp.float32)],
        scratch_shapes=[pltpu.VMEM((1, Hx), jnp.float32),
                        pltpu.VMEM((Hx, Hx), jnp.float32)],
    )(x, Wf, bf)


def _apply_post_pass(a, W2f, b2f, posts, R):
    M, Hx = a.shape
    Bx = W2f.shape[1]
    n = M // R
    widths = [p.shape[1] for p in posts]
    npost = len(posts)

    def body(*refs):
        a_ref = refs[0]
        w_ref = refs[1]
        b_ref = refs[2]
        post_refs = refs[3:3 + npost]
        out_refs = refs[3 + npost:3 + 2 * npost]
        v = _silu(jnp.dot(a_ref[...], w_ref[...],
                          preferred_element_type=jnp.float32) + b_ref[...])
        for pr, orf in zip(post_refs, out_refs):
            orf[...] = jnp.dot(v, pr[...], preferred_element_type=jnp.float32)

    in_specs = [pl.BlockSpec((R, Hx), lambda i: (i, 0)),
                pl.BlockSpec((Hx, Bx), lambda i: (0, 0)),
                pl.BlockSpec((1, Bx), lambda i: (0, 0))]
    for w in widths:
        in_specs.append(pl.BlockSpec((Bx, w), lambda i: (0, 0)))
    out_specs = [pl.BlockSpec((R, w), lambda i: (i, 0)) for w in widths]
    out_shape = [jax.ShapeDtypeStruct((M, w), jnp.float32) for w in widths]
    outs = pl.pallas_call(
        body,
        grid=(n,),
        in_specs=in_specs,
        out_specs=out_specs,
        out_shape=out_shape,
    )(a, W2f, b2f, *posts)
    return outs


def _ewstats_body(t_ref, sc_ref, sh_ref, v_out, s_out, g_out, acc_s, acc_g):
    i = pl.program_id(0)

    @pl.when(i == 0)
    def _init():
        acc_s[...] = jnp.zeros_like(acc_s)
        acc_g[...] = jnp.zeros_like(acc_g)

    v = _silu(t_ref[...] * sc_ref[...] + sh_ref[...])
    v_out[...] = v
    acc_s[...] += jnp.sum(v, axis=0, keepdims=True)
    acc_g[...] += lax.dot_general(v, v, (((0,), (0,)), ((), ())),
                                  preferred_element_type=jnp.float32)

    @pl.when(i == pl.num_programs(0) - 1)
    def _fin():
        s_out[...] = acc_s[...]
        g_out[...] = acc_g[...]


def _ewstats_pass(t, scale, shift, R):
    M, Bx = t.shape
    n = M // R
    return pl.pallas_call(
        _ewstats_body,
        grid=(n,),
        in_specs=[pl.BlockSpec((R, Bx), lambda i: (i, 0)),
                  pl.BlockSpec((1, Bx), lambda i: (0, 0)),
                  pl.BlockSpec((1, Bx), lambda i: (0, 0))],
        out_specs=[pl.BlockSpec((R, Bx), lambda i: (i, 0)),
                   pl.BlockSpec((1, Bx), lambda i: (0, 0)),
                   pl.BlockSpec((Bx, Bx), lambda i: (0, 0))],
        out_shape=[jax.ShapeDtypeStruct((M, Bx), jnp.float32),
                   jax.ShapeDtypeStruct((1, Bx), jnp.float32),
                   jax.ShapeDtypeStruct((Bx, Bx), jnp.float32)],
        scratch_shapes=[pltpu.VMEM((1, Bx), jnp.float32),
                        pltpu.VMEM((Bx, Bx), jnp.float32)],
    )(t, scale, shift)


def _expand_body(v_ref, w_ref, b_ref, r_ref, o_ref):
    o_ref[...] = _silu(jnp.dot(v_ref[...], w_ref[...],
                               preferred_element_type=jnp.float32)
                       + b_ref[...]) + r_ref[...]


def _expand_pass(v, Wf, bf, resid, R):
    M, Bx = v.shape
    Dx = Wf.shape[1]
    n = M // R
    return pl.pallas_call(
        _expand_body,
        grid=(n,),
        in_specs=[pl.BlockSpec((R, Bx), lambda i: (i, 0)),
                  pl.BlockSpec((Bx, Dx), lambda i: (0, 0)),
                  pl.BlockSpec((1, Dx), lambda i: (0, 0)),
                  pl.BlockSpec((R, Dx), lambda i: (i, 0))],
        out_specs=pl.BlockSpec((R, Dx), lambda i: (i, 0)),
        out_shape=jax.ShapeDtypeStruct((M, Dx), jnp.float32),
    )(v, Wf, bf, resid)


def _tw_body(t_ref, sc_ref, sh_ref, w_ref, o_ref):
    m = _silu(t_ref[...] * sc_ref[...] + sh_ref[...])
    o_ref[...] = jnp.dot(m, w_ref[...], preferred_element_type=jnp.float32)


def _tw_pass(t, scale, shift, Weg, R):
    M, Bx = t.shape
    n = M // R
    return pl.pallas_call(
        _tw_body,
        grid=(n,),
        in_specs=[pl.BlockSpec((R, Bx), lambda i: (i, 0)),
                  pl.BlockSpec((1, Bx), lambda i: (0, 0)),
                  pl.BlockSpec((1, Bx), lambda i: (0, 0)),
                  pl.BlockSpec((Bx, Bx), lambda i: (0, 0))],
        out_specs=pl.BlockSpec((R, Bx), lambda i: (i, 0)),
        out_shape=jax.ShapeDtypeStruct((M, Bx), jnp.float32),
    )(t, scale, shift, Weg)


def _fold_linear_bn(S, G, M, W, b, g, be):
    mu_in = S / M
    mean = mu_in @ W + b
    ex2 = jnp.sum(W * (G @ W), axis=0) / M
    var = ex2 - jnp.square(mu_in @ W)[0]
    scale = g / jnp.sqrt(var + EPS_BN)
    Wf = W * scale[None, :]
    bf = (b - mean[0]) * scale + be
    return Wf, bf[None, :]


def _bn_scale_shift(s, ss, M, g, be):
    mean = s / M
    var = ss / M - jnp.square(mean)
    scale = g / jnp.sqrt(var + EPS_BN)
    shift = be - mean * scale
    return scale, shift


def _bottleneck_heads(x, q, R, posts):
    M = x.shape[0]
    S1, G1 = _gram_pass(x, R)
    W1f, b1f = _fold_linear_bn(S1, G1, M, q["W1"], q["b1"], q["g1"], q["be1"])
    a, S2, G2 = _mid_pass(x, W1f, b1f, R)
    W2f, b2f = _fold_linear_bn(S2, G2, M, q["W2"], q["b2"], q["g2"], q["be2"])
    return _apply_post_pass(a, W2f, b2f, posts, R)


def _sc_edge_gate(src_idx, dst_idx, src_tab, dst_tab, ew, CH=80):
    EL = src_idx.shape[0]
    per_w = EL // NW
    nch = per_w // CH
    mesh = plsc.VectorSubcoreMesh(core_axis_name="c", subcore_axis_name="s")

    @functools.partial(
        pl.kernel,
        out_type=[jax.ShapeDtypeStruct((EL, BT), jnp.float32),
                  jax.ShapeDtypeStruct((EL, 2 * BT), jnp.float32),
                  jax.ShapeDtypeStruct((NW, 2, BT), jnp.float32)],
        mesh=mesh,
        compiler_params=pltpu.CompilerParams(use_tc_tiling_on_sc=False),
        scratch_types=[
            pltpu.VMEM((CH,), jnp.int32),
            pltpu.VMEM((CH,), jnp.int32),
            pltpu.VMEM((CH, 2 * BT), jnp.float32),
            pltpu.VMEM((CH, BT), jnp.float32),
            pltpu.VMEM((CH, BT), jnp.float32),
            pltpu.VMEM((CH, BT), jnp.float32),
            pltpu.VMEM((CH, 2 * BT), jnp.float32),
            pltpu.VMEM((2, BT), jnp.float32),
            pltpu.SemaphoreType.DMA,
            pltpu.SemaphoreType.DMA,
            pltpu.SemaphoreType.DMA,
        ],
    )
    def k(src_hbm, dst_hbm, stab_hbm, dtab_hbm, ew_hbm,
          m_hbm, p_hbm, st_hbm,
          idxs_v, idxd_v, srows_v, drows_v, erows_v, mbuf_v, pbuf_v, stat_v,
          sem1, sem2, sem3):
        wid = lax.axis_index("s") * NC + lax.axis_index("c")
        base_w = wid * per_w
        zero = jnp.zeros((L,), jnp.float32)
        for r in range(2):
            for cv in range(BT // L):
                stat_v[r, pl.ds(cv * L, L)] = zero

        def chunk_body(ci, _):
            base = base_w + ci * CH
            pltpu.sync_copy(src_hbm.at[pl.ds(base, CH)], idxs_v)
            pltpu.sync_copy(dst_hbm.at[pl.ds(base, CH)], idxd_v)
            cp1 = pltpu.async_copy(stab_hbm.at[idxs_v], srows_v, sem1)
            cp2 = pltpu.async_copy(dtab_hbm.at[idxd_v], drows_v, sem2)
            cp3 = pltpu.async_copy(ew_hbm.at[pl.ds(base, CH)], erows_v, sem3)
            cp1.wait()
            cp2.wait()
            cp3.wait()

            def row_body(j, carry):
                accs = list(carry)
                for h in range(BT // L):
                    es = srows_v[j, pl.ds(L * h, L)]
                    bh = srows_v[j, pl.ds(BT + L * h, L)]
                    ed = drows_v[j, pl.ds(L * h, L)]
                    zw = erows_v[j, pl.ds(L * h, L)]
                    m = es + ed + zw
                    sg = 1.0 / (1.0 + jnp.exp(-m))
                    v = sg * bh
                    mbuf_v[j, pl.ds(L * h, L)] = m
                    pbuf_v[j, pl.ds(L * h, L)] = v
                    pbuf_v[j, pl.ds(BT + L * h, L)] = sg
                    accs[h] = accs[h] + m
                    accs[2 + h] = accs[2 + h] + m * m
                return tuple(accs)

            accs = lax.fori_loop(0, CH, row_body, (zero, zero, zero, zero))
            for h in range(BT // L):
                stat_v[0, pl.ds(L * h, L)] += accs[h]
                stat_v[1, pl.ds(L * h, L)] += accs[2 + h]
            pltpu.sync_copy(mbuf_v, m_hbm.at[pl.ds(base, CH)])
            pltpu.sync_copy(pbuf_v, p_hbm.at[pl.ds(base, CH)])
            return 0

        lax.fori_loop(0, nch, chunk_body, 0)
        pltpu.sync_copy(stat_v, st_hbm.at[wid])

    return k(src_idx, dst_idx, src_tab, dst_tab, ew)


def _sc_node_conv(src_idx, dst_idx, src_tab, dst_tab, ew, NPAD, CH=80):
    E = src_idx.shape[0]
    per_w = E // NW
    nch = per_w // CH
    stripe = NPAD // NS
    nsch = stripe // CH
    mesh = plsc.VectorSubcoreMesh(core_axis_name="c", subcore_axis_name="s")

    @functools.partial(
        pl.kernel,
        out_type=[jax.ShapeDtypeStruct((E, BT), jnp.float32),
                  jax.ShapeDtypeStruct((NC, NPAD, 2 * BT), jnp.float32),
                  jax.ShapeDtypeStruct((NW, 2, BT), jnp.float32)],
        mesh=mesh,
        compiler_params=pltpu.CompilerParams(use_tc_tiling_on_sc=False),
        scratch_types=[
            pltpu.VMEM((CH,), jnp.int32),
            pltpu.VMEM((CH,), jnp.int32),
            pltpu.VMEM((CH, 2 * BT), jnp.float32),
            pltpu.VMEM((CH, BT), jnp.float32),
            pltpu.VMEM((CH, BT), jnp.float32),
            pltpu.VMEM((CH, BT), jnp.float32),
            pltpu.VMEM((CH, 2 * BT), jnp.float32),
            pltpu.VMEM((2, BT), jnp.float32),
            pltpu.VMEM_SHARED((NPAD, 2 * BT), jnp.float32),
            pltpu.SemaphoreType.DMA,
            pltpu.SemaphoreType.DMA,
            pltpu.SemaphoreType.DMA,
        ],
    )
    def k(src_hbm, dst_hbm, stab_hbm, dtab_hbm, ew_hbm,
          m_hbm, dump_hbm, st_hbm,
          idxs_v, idxd_v, srows_v, drows_v, erows_v, mbuf_v, pbuf_v, stat_v,
          acc_sh, sem1, sem2, sem3):
        cid = lax.axis_index("c")
        sid = lax.axis_index("s")
        wid = sid * NC + cid
        base_w = wid * per_w
        zero = jnp.zeros((L,), jnp.float32)

        def zrow(j, _):
            for h in range(2 * BT // L):
                pbuf_v[j, pl.ds(L * h, L)] = zero
            return 0
        lax.fori_loop(0, CH, zrow, 0)

        def zchunk(c, _):
            pltpu.sync_copy(pbuf_v, acc_sh.at[pl.ds(sid * stripe + c * CH, CH)])
            return 0
        lax.fori_loop(0, nsch, zchunk, 0)
        for r in range(2):
            for cv in range(BT // L):
                stat_v[r, pl.ds(cv * L, L)] = zero
        plsc.subcore_barrier()

        def chunk_body(ci, _):
            base = base_w + ci * CH
            pltpu.sync_copy(src_hbm.at[pl.ds(base, CH)], idxs_v)
            pltpu.sync_copy(dst_hbm.at[pl.ds(base, CH)], idxd_v)
            cp1 = pltpu.async_copy(stab_hbm.at[idxs_v], srows_v, sem1)
            cp2 = pltpu.async_copy(dtab_hbm.at[idxd_v], drows_v, sem2)
            cp3 = pltpu.async_copy(ew_hbm.at[pl.ds(base, CH)], erows_v, sem3)
            cp1.wait()
            cp2.wait()
            cp3.wait()

            def row_body(j, carry):
                accs = list(carry)
                for h in range(BT // L):
                    es = srows_v[j, pl.ds(L * h, L)]
                    bh = srows_v[j, pl.ds(BT + L * h, L)]
                    ed = drows_v[j, pl.ds(L * h, L)]
                    zw = erows_v[j, pl.ds(L * h, L)]
                    m = es + ed + zw
                    sg = 1.0 / (1.0 + jnp.exp(-m))
                    v = sg * bh
                    mbuf_v[j, pl.ds(L * h, L)] = m
                    pbuf_v[j, pl.ds(L * h, L)] = v
                    pbuf_v[j, pl.ds(BT + L * h, L)] = sg
                    accs[h] = accs[h] + m
                    accs[2 + h] = accs[2 + h] + m * m
                return tuple(accs)

            accs = lax.fori_loop(0, CH, row_body, (zero, zero, zero, zero))
            for h in range(BT // L):
                stat_v[0, pl.ds(L * h, L)] += accs[h]
                stat_v[1, pl.ds(L * h, L)] += accs[2 + h]
            pltpu.sync_copy(mbuf_v, m_hbm.at[pl.ds(base, CH)])
            pltpu.sync_copy(pbuf_v, acc_sh.at[idxd_v], add=True)
            return 0

        lax.fori_loop(0, nch, chunk_body, 0)
        pltpu.sync_copy(stat_v, st_hbm.at[wid])
        plsc.subcore_barrier()

        def dchunk(c, _):
            off = sid * stripe + c * CH
            pltpu.sync_copy(acc_sh.at[pl.ds(off, CH)], pbuf_v)
            pltpu.sync_copy(pbuf_v, dump_hbm.at[cid, pl.ds(off, CH)])
            return 0
        lax.fori_loop(0, nsch, dchunk, 0)

    return k(src_idx, dst_idx, src_tab, dst_tab, ew)


def _sc_seg_sum(dst_idx, pay, zrows, E, K=8, SEGPAD=20480):
    EL = dst_idx.shape[0]
    SEGC = E // (K * NC)
    assert SEGC <= SEGPAD
    DUMMY = SEGPAD - 8
    per_s = EL // NS
    SCH = 2000
    nstage = per_s // SCH
    nvec = SCH // L
    FB = 128
    RING = 1024
    stripe = SEGPAD // NS
    nzch = stripe // 80
    nfch = SEGC // 80
    nfk = (nfch + NS - 1) // NS
    mesh = plsc.VectorSubcoreMesh(core_axis_name="c", subcore_axis_name="s")

    @functools.partial(
        pl.kernel,
        out_type=[jax.ShapeDtypeStruct((E, 2 * BT), jnp.float32)],
        mesh=mesh,
        compiler_params=pltpu.CompilerParams(use_tc_tiling_on_sc=False,
                                             needs_layout_passes=False),
        scratch_types=[
            pltpu.VMEM((SCH,), jnp.int32),
            pltpu.VMEM((RING // FB, FB), jnp.int32),
            pltpu.VMEM((RING // FB, FB), jnp.int32),
            pltpu.VMEM((FB, 2 * BT), jnp.float32),
            pltpu.VMEM((80, 2 * BT), jnp.float32),
            pltpu.VMEM((80, 2 * BT), jnp.float32),
            pltpu.VMEM_SHARED((SEGPAD, 2 * BT), jnp.float32),
            pltpu.SemaphoreType.DMA,
        ],
    )
    def k(dst_hbm, pay_hbm, zrows_hbm, d_hbm,
          dbuf_v, cidx_v, cseg_v, prow_v, zbuf_v, bnc_v,
          acc_sh, semf):
        cid = lax.axis_index("c")
        sid = lax.axis_index("s")
        iota = lax.iota(jnp.int32, L)
        pltpu.sync_copy(zrows_hbm, zbuf_v)

        def flush_one(tail):
            b = lax.rem(lax.div(tail, FB), RING // FB)
            pltpu.async_copy(pay_hbm.at[cidx_v.at[b]], prow_v, semf).wait()
            pltpu.sync_copy(prow_v, acc_sh.at[cseg_v.at[b]], add=True)
            return tail + FB

        def pass_body(p, _):
            lo = (p * NC + cid) * SEGC

            def zchunk(c, _):
                pltpu.sync_copy(zbuf_v, acc_sh.at[pl.ds(sid * stripe + c * 80, 80)])
                return 0
            lax.fori_loop(0, nzch, zchunk, 0)
            plsc.subcore_barrier()

            def stage_body(sc, carry):
                pos, tail = carry
                sbase = sid * per_s + sc * SCH
                pltpu.sync_copy(dst_hbm.at[pl.ds(sbase, SCH)], dbuf_v)

                def vec_body(it, carry):
                    pos, tail = carry
                    dv = dbuf_v[pl.ds(it * L, L)]
                    msk = (dv >= lo) & (dv < lo + SEGC)
                    m01f = jnp.where(msk, 1.0, 0.0)
                    incl = plsc.cumsum(m01f).astype(jnp.int32)
                    cnt = jnp.sum(m01f).astype(jnp.int32)
                    posv = pos + incl - 1
                    rp = jnp.bitwise_and(posv, RING - 1)
                    row = jnp.right_shift(rp, 7)
                    col = jnp.bitwise_and(rp, FB - 1)
                    ids = sbase + it * L + iota
                    plsc.store_scatter(cidx_v, [row, col], ids, mask=msk)
                    plsc.store_scatter(cseg_v, [row, col], dv - lo, mask=msk)
                    pos = pos + cnt
                    pos, tail = lax.while_loop(
                        lambda c: c[0] - c[1] >= FB,
                        lambda c: (c[0], flush_one(c[1])),
                        (pos, tail))
                    return pos, tail

                return lax.fori_loop(0, nvec, vec_body, (pos, tail))

            pos, tail = lax.fori_loop(0, nstage, stage_body,
                                      (jnp.int32(0), jnp.int32(0)))

            nfill = jnp.bitwise_and(-pos, FB - 1)
            for h in range(FB // L):
                fpos = pos + h * L + iota
                fmask = (fpos - pos) < nfill
                rp = jnp.bitwise_and(fpos, RING - 1)
                row = jnp.right_shift(rp, 7)
                col = jnp.bitwise_and(rp, FB - 1)
                plsc.store_scatter(cidx_v, [row, col],
                                   jnp.zeros((L,), jnp.int32), mask=fmask)
                plsc.store_scatter(cseg_v, [row, col],
                                   jnp.full((L,), DUMMY, jnp.int32), mask=fmask)
            pos = pos + nfill
            pos, tail = lax.while_loop(
                lambda c: c[0] - c[1] >= FB,
                lambda c: (c[0], flush_one(c[1])),
                (pos, tail))
            plsc.subcore_barrier()

            def fin_body(kk, _):
                c = kk * NS + sid

                @pl.when(c < nfch)
                def _():
                    off = c * 80
                    pltpu.sync_copy(acc_sh.at[pl.ds(off, 80)], bnc_v)
                    pltpu.sync_copy(bnc_v, d_hbm.at[pl.ds(lo + off, 80)])
                return 0

            lax.fori_loop(0, nfk, fin_body, 0)
            plsc.subcore_barrier()
            return 0

        lax.fori_loop(0, K, pass_body, 0)

    return k(dst_idx, pay, zrows)


def _segfin_body(d_ref, u_ref, t_out, s_out, q_out, acc_s, acc_q):
    i = pl.program_id(0)

    @pl.when(i == 0)
    def _init():
        acc_s[...] = jnp.zeros_like(acc_s)
        acc_q[...] = jnp.zeros_like(acc_q)

    d = d_ref[...]
    t = u_ref[...] + d[:, :BT] / (d[:, BT:] + EPS_DIV)
    t_out[...] = t
    acc_s[...] += jnp.sum(t, axis=0, keepdims=True)
    acc_q[...] += jnp.sum(t * t, axis=0, keepdims=True)

    @pl.when(i == pl.num_programs(0) - 1)
    def _fin():
        s_out[...] = acc_s[...]
        q_out[...] = acc_q[...]


def _segfin_pass(d, u, R):
    M = u.shape[0]
    n = M // R
    return pl.pallas_call(
        _segfin_body,
        grid=(n,),
        in_specs=[pl.BlockSpec((R, 2 * BT), lambda i: (i, 0)),
                  pl.BlockSpec((R, BT), lambda i: (i, 0))],
        out_specs=[pl.BlockSpec((R, BT), lambda i: (i, 0)),
                   pl.BlockSpec((1, BT), lambda i: (0, 0)),
                   pl.BlockSpec((1, BT), lambda i: (0, 0))],
        out_shape=[jax.ShapeDtypeStruct((M, BT), jnp.float32),
                   jax.ShapeDtypeStruct((1, BT), jnp.float32),
                   jax.ShapeDtypeStruct((1, BT), jnp.float32)],
        scratch_shapes=[pltpu.VMEM((1, BT), jnp.float32),
                        pltpu.VMEM((1, BT), jnp.float32)],
    )(d, u)


def _combine_body(a0_ref, a1_ref, u_ref, t_ref):
    ah = a0_ref[:, :BT] + a1_ref[:, :BT]
    as_ = a0_ref[:, BT:] + a1_ref[:, BT:]
    t_ref[...] = u_ref[...] + ah / (as_ + EPS_DIV)


def _combine_pass(a0, a1, u, R):
    M = u.shape[0]
    n = M // R
    return pl.pallas_call(
        _combine_body,
        grid=(n,),
        in_specs=[pl.BlockSpec((R, 2 * BT), lambda i: (i, 0)),
                  pl.BlockSpec((R, 2 * BT), lambda i: (i, 0)),
                  pl.BlockSpec((R, BT), lambda i: (i, 0))],
        out_specs=pl.BlockSpec((R, BT), lambda i: (i, 0)),
        out_shape=jax.ShapeDtypeStruct((M, BT), jnp.float32),
    )(a0, a1, u)


def kernel(x, y, z, edge_index, lg_edge_index, params):
    N = x.shape[0]
    E = y.shape[0]
    EL = z.shape[0]
    R = 2000
    p = params

    qy = p["edge_upd"]
    src_tab_y, dst_tab_y, u_y = _bottleneck_heads(
        y, p["pair_bn"], R,
        [jnp.concatenate([qy["Wsg"], qy["Wdu"]], axis=1), qy["Wdg"], qy["Wsu"]])
    (zweg,) = _bottleneck_heads(z, p["trip_bn"], R, [qy["Weg"]])
    qx = p["node_upd"]
    src_tab_x, dst_tab_x, u_x = _bottleneck_heads(
        x, p["node_bn"], R,
        [jnp.concatenate([qx["Wsg"], qx["Wdu"]], axis=1), qx["Wdg"], qx["Wsu"]])

    src_l, dst_l = lg_edge_index[0], lg_edge_index[1]
    m_l, pay_l, st_l = _sc_edge_gate(src_l, dst_l, src_tab_y, dst_tab_y, zweg)
    stls = jnp.sum(st_l, axis=0)
    s_ml, ssq_ml = stls[0], stls[1]

    zrows = jnp.zeros((80, 2 * BT), jnp.float32)
    (segdump,) = _sc_seg_sum(dst_l, pay_l, zrows, E)
    t, St, Qt = _segfin_pass(segdump, u_y, R)
    sc_t, sh_t = _bn_scale_shift(St[0], Qt[0], E, qy["gn"], qy["bn"])
    w = _tw_pass(t, sc_t[None, :], sh_t[None, :], qx["Weg"], R)

    sc_m, sh_m = _bn_scale_shift(s_ml, ssq_ml, EL, qy["ge"], qy["be"])
    z2, Sz2, Gz2 = _ewstats_pass(m_l, sc_m[None, :], sh_m[None, :], R)

    src_g, dst_g = edge_index[0], edge_index[1]
    NPAD = 10240
    m2, dump, st2 = _sc_node_conv(src_g, dst_g, src_tab_x, dst_tab_x, w, NPAD)
    st2s = jnp.sum(st2, axis=0)
    s_m2, ssq_m2 = st2s[0], st2s[1]

    t2 = _combine_pass(dump[0, :N], dump[1, :N], u_x, 2000)
    St2, Gt2 = _gram_pass(t2, R)
    sc_t2, sh_t2 = _bn_scale_shift(St2[0], jnp.diag(Gt2), N, qx["gn"], qx["bn"])
    x2, Sx2, Gx2 = _ewstats_pass(t2, sc_t2[None, :], sh_t2[None, :], R)

    sc_m2, sh_m2 = _bn_scale_shift(s_m2, ssq_m2, E, qx["ge"], qx["be"])
    y2, Sy2, Gy2 = _ewstats_pass(m2, sc_m2[None, :], sh_m2[None, :], R)

    qe = p["node_ex"]
    Wxf, bxf = _fold_linear_bn(Sx2, Gx2, N, qe["W"], qe["b"], qe["g"], qe["be"])
    xo = _expand_pass(x2, Wxf, bxf, x, R)
    qe = p["pair_ex"]
    Wyf, byf = _fold_linear_bn(Sy2, Gy2, E, qe["W"], qe["b"], qe["g"], qe["be"])
    yo = _expand_pass(y2, Wyf, byf, y, R)
    qe = p["trip_ex"]
    Wzf, bzf = _fold_linear_bn(Sz2, Gz2, EL, qe["W"], qe["b"], qe["g"], qe["be"])
    zo = _expand_pass(z2, Wzf, bzf, z, R)

    return xo, yo, zo


# speedup vs baseline: 6.0764x; 1.0352x over previous
import functools

import jax
import jax.numpy as jnp
from jax import lax
from jax.experimental import pallas as pl
from jax.experimental.pallas import tpu as pltpu
from jax.experimental.pallas import tpu_sc as plsc

D = 128
H = 64
BT = 32

NC = 2
NS = 16
L = 16
NW = NC * NS

EPS_BN = 1e-5
EPS_DIV = 1e-6


def _silu(v):
    return v * (1.0 / (1.0 + jnp.exp(-v)))


def _gram_body(x_ref, s_out, g_out, acc_s, acc_g):
    i = pl.program_id(0)

    @pl.when(i == 0)
    def _init():
        acc_s[...] = jnp.zeros_like(acc_s)
        acc_g[...] = jnp.zeros_like(acc_g)

    x = x_ref[...]
    acc_s[...] += jnp.sum(x, axis=0, keepdims=True)
    acc_g[...] += lax.dot_general(x, x, (((0,), (0,)), ((), ())),
                                  preferred_element_type=jnp.float32)

    @pl.when(i == pl.num_programs(0) - 1)
    def _fin():
        s_out[...] = acc_s[...]
        g_out[...] = acc_g[...]


def _gram_pass(x, R):
    M, Dx = x.shape
    n = M // R
    return pl.pallas_call(
        _gram_body,
        grid=(n,),
        in_specs=[pl.BlockSpec((R, Dx), lambda i: (i, 0))],
        out_specs=[pl.BlockSpec((1, Dx), lambda i: (0, 0)),
                   pl.BlockSpec((Dx, Dx), lambda i: (0, 0))],
        out_shape=[jax.ShapeDtypeStruct((1, Dx), jnp.float32),
                   jax.ShapeDtypeStruct((Dx, Dx), jnp.float32)],
        scratch_shapes=[pltpu.VMEM((1, Dx), jnp.float32),
                        pltpu.VMEM((Dx, Dx), jnp.float32)],
    )(x)


def _mid_body(x_ref, w_ref, b_ref, a_out, s_out, g_out, acc_s, acc_g):
    i = pl.program_id(0)

    @pl.when(i == 0)
    def _init():
        acc_s[...] = jnp.zeros_like(acc_s)
        acc_g[...] = jnp.zeros_like(acc_g)

    a = _silu(jnp.dot(x_ref[...], w_ref[...],
                      preferred_element_type=jnp.float32) + b_ref[...])
    a_out[...] = a
    acc_s[...] += jnp.sum(a, axis=0, keepdims=True)
    acc_g[...] += lax.dot_general(a, a, (((0,), (0,)), ((), ())),
                                  preferred_element_type=jnp.float32)

    @pl.when(i == pl.num_programs(0) - 1)
    def _fin():
        s_out[...] = acc_s[...]
        g_out[...] = acc_g[...]


def _mid_pass(x, Wf, bf, R):
    M, Dx = x.shape
    Hx = Wf.shape[1]
    n = M // R
    return pl.pallas_call(
        _mid_body,
        grid=(n,),
        in_specs=[pl.BlockSpec((R, Dx), lambda i: (i, 0)),
                  pl.BlockSpec((Dx, Hx), lambda i: (0, 0)),
                  pl.BlockSpec((1, Hx), lambda i: (0, 0))],
        out_specs=[pl.BlockSpec((R, Hx), lambda i: (i, 0)),
                   pl.BlockSpec((1, Hx), lambda i: (0, 0)),
                   pl.BlockSpec((Hx, Hx), lambda i: (0, 0))],
        out_shape=[jax.ShapeDtypeStruct((M, Hx), jnp.float32),
                   jax.ShapeDtypeStruct((1, Hx), jnp.float32),
                   jax.ShapeDtypeStruct((Hx, Hx), jnp.float32)],
        scratch_shapes=[pltpu.VMEM((1, Hx), jnp.float32),
                        pltpu.VMEM((Hx, Hx), jnp.float32)],
    )(x, Wf, bf)


def _apply_post_pass(a, W2f, b2f, posts, R):
    M, Hx = a.shape
    Bx = W2f.shape[1]
    n = M // R
    widths = [p.shape[1] for p in posts]
    npost = len(posts)

    def body(*refs):
        a_ref = refs[0]
        w_ref = refs[1]
        b_ref = refs[2]
        post_refs = refs[3:3 + npost]
        out_refs = refs[3 + npost:3 + 2 * npost]
        v = _silu(jnp.dot(a_ref[...], w_ref[...],
                          preferred_element_type=jnp.float32) + b_ref[...])
        for pr, orf in zip(post_refs, out_refs):
            orf[...] = jnp.dot(v, pr[...], preferred_element_type=jnp.float32)

    in_specs = [pl.BlockSpec((R, Hx), lambda i: (i, 0)),
                pl.BlockSpec((Hx, Bx), lambda i: (0, 0)),
                pl.BlockSpec((1, Bx), lambda i: (0, 0))]
    for w in widths:
        in_specs.append(pl.BlockSpec((Bx, w), lambda i: (0, 0)))
    out_specs = [pl.BlockSpec((R, w), lambda i: (i, 0)) for w in widths]
    out_shape = [jax.ShapeDtypeStruct((M, w), jnp.float32) for w in widths]
    outs = pl.pallas_call(
        body,
        grid=(n,),
        in_specs=in_specs,
        out_specs=out_specs,
        out_shape=out_shape,
    )(a, W2f, b2f, *posts)
    return outs


def _ewstats_body(t_ref, sc_ref, sh_ref, v_out, s_out, g_out, acc_s, acc_g):
    i = pl.program_id(0)

    @pl.when(i == 0)
    def _init():
        acc_s[...] = jnp.zeros_like(acc_s)
        acc_g[...] = jnp.zeros_like(acc_g)

    v = _silu(t_ref[...] * sc_ref[...] + sh_ref[...])
    v_out[...] = v
    acc_s[...] += jnp.sum(v, axis=0, keepdims=True)
    acc_g[...] += lax.dot_general(v, v, (((0,), (0,)), ((), ())),
                                  preferred_element_type=jnp.float32)

    @pl.when(i == pl.num_programs(0) - 1)
    def _fin():
        s_out[...] = acc_s[...]
        g_out[...] = acc_g[...]


def _ewstats_pass(t, scale, shift, R):
    M, Bx = t.shape
    n = M // R
    return pl.pallas_call(
        _ewstats_body,
        grid=(n,),
        in_specs=[pl.BlockSpec((R, Bx), lambda i: (i, 0)),
                  pl.BlockSpec((1, Bx), lambda i: (0, 0)),
                  pl.BlockSpec((1, Bx), lambda i: (0, 0))],
        out_specs=[pl.BlockSpec((R, Bx), lambda i: (i, 0)),
                   pl.BlockSpec((1, Bx), lambda i: (0, 0)),
                   pl.BlockSpec((Bx, Bx), lambda i: (0, 0))],
        out_shape=[jax.ShapeDtypeStruct((M, Bx), jnp.float32),
                   jax.ShapeDtypeStruct((1, Bx), jnp.float32),
                   jax.ShapeDtypeStruct((Bx, Bx), jnp.float32)],
        scratch_shapes=[pltpu.VMEM((1, Bx), jnp.float32),
                        pltpu.VMEM((Bx, Bx), jnp.float32)],
    )(t, scale, shift)


def _expand_body(v_ref, w_ref, b_ref, r_ref, o_ref):
    o_ref[...] = _silu(jnp.dot(v_ref[...], w_ref[...],
                               preferred_element_type=jnp.float32)
                       + b_ref[...]) + r_ref[...]


def _expand_pass(v, Wf, bf, resid, R):
    M, Bx = v.shape
    Dx = Wf.shape[1]
    n = M // R
    return pl.pallas_call(
        _expand_body,
        grid=(n,),
        in_specs=[pl.BlockSpec((R, Bx), lambda i: (i, 0)),
                  pl.BlockSpec((Bx, Dx), lambda i: (0, 0)),
                  pl.BlockSpec((1, Dx), lambda i: (0, 0)),
                  pl.BlockSpec((R, Dx), lambda i: (i, 0))],
        out_specs=pl.BlockSpec((R, Dx), lambda i: (i, 0)),
        out_shape=jax.ShapeDtypeStruct((M, Dx), jnp.float32),
    )(v, Wf, bf, resid)


def _tw_body(t_ref, sc_ref, sh_ref, w_ref, o_ref):
    m = _silu(t_ref[...] * sc_ref[...] + sh_ref[...])
    o_ref[...] = jnp.dot(m, w_ref[...], preferred_element_type=jnp.float32)


def _tw_pass(t, scale, shift, Weg, R):
    M, Bx = t.shape
    n = M // R
    return pl.pallas_call(
        _tw_body,
        grid=(n,),
        in_specs=[pl.BlockSpec((R, Bx), lambda i: (i, 0)),
                  pl.BlockSpec((1, Bx), lambda i: (0, 0)),
                  pl.BlockSpec((1, Bx), lambda i: (0, 0)),
                  pl.BlockSpec((Bx, Bx), lambda i: (0, 0))],
        out_specs=pl.BlockSpec((R, Bx), lambda i: (i, 0)),
        out_shape=jax.ShapeDtypeStruct((M, Bx), jnp.float32),
    )(t, scale, shift, Weg)


def _fold_linear_bn(S, G, M, W, b, g, be):
    mu_in = S / M
    mean = mu_in @ W + b
    ex2 = jnp.sum(W * (G @ W), axis=0) / M
    var = ex2 - jnp.square(mu_in @ W)[0]
    scale = g / jnp.sqrt(var + EPS_BN)
    Wf = W * scale[None, :]
    bf = (b - mean[0]) * scale + be
    return Wf, bf[None, :]


def _bn_scale_shift(s, ss, M, g, be):
    mean = s / M
    var = ss / M - jnp.square(mean)
    scale = g / jnp.sqrt(var + EPS_BN)
    shift = be - mean * scale
    return scale, shift


def _bottleneck_heads(x, q, R, posts):
    M = x.shape[0]
    S1, G1 = _gram_pass(x, R)
    W1f, b1f = _fold_linear_bn(S1, G1, M, q["W1"], q["b1"], q["g1"], q["be1"])
    a, S2, G2 = _mid_pass(x, W1f, b1f, R)
    W2f, b2f = _fold_linear_bn(S2, G2, M, q["W2"], q["b2"], q["g2"], q["be2"])
    return _apply_post_pass(a, W2f, b2f, posts, R)


def _sc_edge_gate(src_idx, dst_idx, src_tab, dst_tab, ew, CH=80):
    EL = src_idx.shape[0]
    per_w = EL // NW
    nch = per_w // CH
    mesh = plsc.VectorSubcoreMesh(core_axis_name="c", subcore_axis_name="s")

    @functools.partial(
        pl.kernel,
        out_type=[jax.ShapeDtypeStruct((EL, BT), jnp.float32),
                  jax.ShapeDtypeStruct((EL, 2 * BT), jnp.float32),
                  jax.ShapeDtypeStruct((NW, 2, BT), jnp.float32)],
        mesh=mesh,
        compiler_params=pltpu.CompilerParams(use_tc_tiling_on_sc=False),
        scratch_types=(
            [pltpu.VMEM((CH,), jnp.int32)] * 4
            + [pltpu.VMEM((CH, 2 * BT), jnp.float32)] * 2
            + [pltpu.VMEM((CH, BT), jnp.float32)] * 4
            + [pltpu.VMEM((CH, BT), jnp.float32)] * 2
            + [pltpu.VMEM((CH, 2 * BT), jnp.float32)] * 2
            + [pltpu.VMEM((2, BT), jnp.float32)]
            + [pltpu.SemaphoreType.DMA] * 2
        ),
    )
    def k(src_hbm, dst_hbm, stab_hbm, dtab_hbm, ew_hbm,
          m_hbm, p_hbm, st_hbm,
          idxs0, idxs1, idxd0, idxd1, srows0, srows1,
          drows0, drows1, erows0, erows1, mbuf0, mbuf1, pbuf0, pbuf1, stat_v,
          sem0, sem1):
        wid = lax.axis_index("s") * NC + lax.axis_index("c")
        base_w = wid * per_w
        zero = jnp.zeros((L,), jnp.float32)
        for r in range(2):
            for cv in range(BT // L):
                stat_v[r, pl.ds(cv * L, L)] = zero

        idxs = (idxs0, idxs1)
        idxd = (idxd0, idxd1)
        srows = (srows0, srows1)
        drows = (drows0, drows1)
        erows = (erows0, erows1)
        mbuf = (mbuf0, mbuf1)
        pbuf = (pbuf0, pbuf1)
        sems = (sem0, sem1)

        def start(ci, b):
            base = base_w + ci * CH
            pltpu.sync_copy(src_hbm.at[pl.ds(base, CH)], idxs[b])
            pltpu.sync_copy(dst_hbm.at[pl.ds(base, CH)], idxd[b])
            pltpu.async_copy(stab_hbm.at[idxs[b]], srows[b], sems[b])
            pltpu.async_copy(dtab_hbm.at[idxd[b]], drows[b], sems[b])
            pltpu.async_copy(ew_hbm.at[pl.ds(base, CH)], erows[b], sems[b])

        def finish(ci, b):
            pltpu.make_async_copy(stab_hbm.at[idxs[b]], srows[b], sems[b]).wait()
            pltpu.make_async_copy(dtab_hbm.at[idxd[b]], drows[b], sems[b]).wait()
            base = base_w + ci * CH
            pltpu.make_async_copy(ew_hbm.at[pl.ds(base, CH)], erows[b], sems[b]).wait()

        def compute(ci, b):
            base = base_w + ci * CH

            def row_body(j, carry):
                accs = list(carry)
                for h in range(BT // L):
                    es = srows[b][j, pl.ds(L * h, L)]
                    bh = srows[b][j, pl.ds(BT + L * h, L)]
                    ed = drows[b][j, pl.ds(L * h, L)]
                    zw = erows[b][j, pl.ds(L * h, L)]
                    m = es + ed + zw
                    sg = 1.0 / (1.0 + jnp.exp(-m))
                    v = sg * bh
                    mbuf[b][j, pl.ds(L * h, L)] = m
                    pbuf[b][j, pl.ds(L * h, L)] = v
                    pbuf[b][j, pl.ds(BT + L * h, L)] = sg
                    accs[h] = accs[h] + m
                    accs[2 + h] = accs[2 + h] + m * m
                return tuple(accs)

            accs = lax.fori_loop(0, CH, row_body, (zero, zero, zero, zero))
            for h in range(BT // L):
                stat_v[0, pl.ds(L * h, L)] += accs[h]
                stat_v[1, pl.ds(L * h, L)] += accs[2 + h]
            pltpu.sync_copy(mbuf[b], m_hbm.at[pl.ds(base, CH)])
            pltpu.sync_copy(pbuf[b], p_hbm.at[pl.ds(base, CH)])

        start(0, 0)

        def pair_body(i, _):
            for b in range(2):
                ci = i * 2 + b

                @pl.when(ci + 1 < nch)
                def _():
                    start(ci + 1, 1 - b)
                finish(ci, b)
                compute(ci, b)
            return 0

        lax.fori_loop(0, nch // 2, pair_body, 0)
        if nch % 2:
            finish(nch - 1, (nch - 1) % 2)
            compute(nch - 1, (nch - 1) % 2)
        pltpu.sync_copy(stat_v, st_hbm.at[wid])

    return k(src_idx, dst_idx, src_tab, dst_tab, ew)


def _sc_node_conv(src_idx, dst_idx, src_tab, dst_tab, ew, NPAD, CH=80):
    E = src_idx.shape[0]
    per_w = E // NW
    nch = per_w // CH
    stripe = NPAD // NS
    nsch = stripe // CH
    mesh = plsc.VectorSubcoreMesh(core_axis_name="c", subcore_axis_name="s")

    @functools.partial(
        pl.kernel,
        out_type=[jax.ShapeDtypeStruct((E, BT), jnp.float32),
                  jax.ShapeDtypeStruct((NC, NPAD, 2 * BT), jnp.float32),
                  jax.ShapeDtypeStruct((NW, 2, BT), jnp.float32)],
        mesh=mesh,
        compiler_params=pltpu.CompilerParams(use_tc_tiling_on_sc=False),
        scratch_types=(
            [pltpu.VMEM((CH,), jnp.int32)] * 4
            + [pltpu.VMEM((CH, 2 * BT), jnp.float32)] * 2
            + [pltpu.VMEM((CH, BT), jnp.float32)] * 4
            + [pltpu.VMEM((CH, BT), jnp.float32)] * 2
            + [pltpu.VMEM((CH, 2 * BT), jnp.float32)] * 2
            + [pltpu.VMEM((2, BT), jnp.float32)]
            + [pltpu.VMEM_SHARED((NPAD, 2 * BT), jnp.float32)]
            + [pltpu.SemaphoreType.DMA] * 2
        ),
    )
    def k(src_hbm, dst_hbm, stab_hbm, dtab_hbm, ew_hbm,
          m_hbm, dump_hbm, st_hbm,
          idxs0, idxs1, idxd0, idxd1, srows0, srows1,
          drows0, drows1, erows0, erows1, mbuf0, mbuf1, pbuf0, pbuf1, stat_v,
          acc_sh, sem0, sem1):
        cid = lax.axis_index("c")
        sid = lax.axis_index("s")
        wid = sid * NC + cid
        base_w = wid * per_w
        zero = jnp.zeros((L,), jnp.float32)

        idxs = (idxs0, idxs1)
        idxd = (idxd0, idxd1)
        srows = (srows0, srows1)
        drows = (drows0, drows1)
        erows = (erows0, erows1)
        mbuf = (mbuf0, mbuf1)
        pbuf = (pbuf0, pbuf1)
        sems = (sem0, sem1)

        def zrow(j, _):
            for h in range(2 * BT // L):
                pbuf0[j, pl.ds(L * h, L)] = zero
            return 0
        lax.fori_loop(0, CH, zrow, 0)

        def zchunk(c, _):
            pltpu.sync_copy(pbuf0, acc_sh.at[pl.ds(sid * stripe + c * CH, CH)])
            return 0
        lax.fori_loop(0, nsch, zchunk, 0)
        for r in range(2):
            for cv in range(BT // L):
                stat_v[r, pl.ds(cv * L, L)] = zero
        plsc.subcore_barrier()

        def start(ci, b):
            base = base_w + ci * CH
            pltpu.sync_copy(src_hbm.at[pl.ds(base, CH)], idxs[b])
            pltpu.sync_copy(dst_hbm.at[pl.ds(base, CH)], idxd[b])
            pltpu.async_copy(stab_hbm.at[idxs[b]], srows[b], sems[b])
            pltpu.async_copy(dtab_hbm.at[idxd[b]], drows[b], sems[b])
            pltpu.async_copy(ew_hbm.at[pl.ds(base, CH)], erows[b], sems[b])

        def finish(ci, b):
            pltpu.make_async_copy(stab_hbm.at[idxs[b]], srows[b], sems[b]).wait()
            pltpu.make_async_copy(dtab_hbm.at[idxd[b]], drows[b], sems[b]).wait()
            base = base_w + ci * CH
            pltpu.make_async_copy(ew_hbm.at[pl.ds(base, CH)], erows[b], sems[b]).wait()

        def compute(ci, b):
            base = base_w + ci * CH

            def row_body(j, carry):
                accs = list(carry)
                for h in range(BT // L):
                    es = srows[b][j, pl.ds(L * h, L)]
                    bh = srows[b][j, pl.ds(BT + L * h, L)]
                    ed = drows[b][j, pl.ds(L * h, L)]
                    zw = erows[b][j, pl.ds(L * h, L)]
                    m = es + ed + zw
                    sg = 1.0 / (1.0 + jnp.exp(-m))
                    v = sg * bh
                    mbuf[b][j, pl.ds(L * h, L)] = m
                    pbuf[b][j, pl.ds(L * h, L)] = v
                    pbuf[b][j, pl.ds(BT + L * h, L)] = sg
                    accs[h] = accs[h] + m
                    accs[2 + h] = accs[2 + h] + m * m
                return tuple(accs)

            accs = lax.fori_loop(0, CH, row_body, (zero, zero, zero, zero))
            for h in range(BT // L):
                stat_v[0, pl.ds(L * h, L)] += accs[h]
                stat_v[1, pl.ds(L * h, L)] += accs[2 + h]
            pltpu.sync_copy(mbuf[b], m_hbm.at[pl.ds(base, CH)])
            pltpu.sync_copy(pbuf[b], acc_sh.at[idxd[b]], add=True)

        start(0, 0)

        def pair_body(i, _):
            for b in range(2):
                ci = i * 2 + b

                @pl.when(ci + 1 < nch)
                def _():
                    start(ci + 1, 1 - b)
                finish(ci, b)
                compute(ci, b)
            return 0

        lax.fori_loop(0, nch // 2, pair_body, 0)
        if nch % 2:
            finish(nch - 1, (nch - 1) % 2)
            compute(nch - 1, (nch - 1) % 2)
        pltpu.sync_copy(stat_v, st_hbm.at[wid])
        plsc.subcore_barrier()

        def dchunk(c, _):
            off = sid * stripe + c * CH
            pltpu.sync_copy(acc_sh.at[pl.ds(off, CH)], pbuf0)
            pltpu.sync_copy(pbuf0, dump_hbm.at[cid, pl.ds(off, CH)])
            return 0
        lax.fori_loop(0, nsch, dchunk, 0)

    return k(src_idx, dst_idx, src_tab, dst_tab, ew)


def _sc_seg_sum(dst_idx, pay, zrows, E, K=8, SEGPAD=20480):
    EL = dst_idx.shape[0]
    SEGC = E // (K * NC)
    assert SEGC <= SEGPAD
    DUMMY = SEGPAD - 8
    per_s = EL // NS
    SCH = 2000
    nstage = per_s // SCH
    nvec = SCH // L
    FB = 128
    RING = 1024
    stripe = SEGPAD // NS
    nzch = stripe // 80
    nfch = SEGC // 80
    nfk = (nfch + NS - 1) // NS
    mesh = plsc.VectorSubcoreMesh(core_axis_name="c", subcore_axis_name="s")

    @functools.partial(
        pl.kernel,
        out_type=[jax.ShapeDtypeStruct((E, 2 * BT), jnp.float32)],
        mesh=mesh,
        compiler_params=pltpu.CompilerParams(use_tc_tiling_on_sc=False,
                                             needs_layout_passes=False),
        scratch_types=[
            pltpu.VMEM((SCH,), jnp.int32),
            pltpu.VMEM((RING // FB, FB), jnp.int32),
            pltpu.VMEM((RING // FB, FB), jnp.int32),
            pltpu.VMEM((FB, 2 * BT), jnp.float32),
            pltpu.VMEM((80, 2 * BT), jnp.float32),
            pltpu.VMEM((80, 2 * BT), jnp.float32),
            pltpu.VMEM_SHARED((SEGPAD, 2 * BT), jnp.float32),
            pltpu.SemaphoreType.DMA,
        ],
    )
    def k(dst_hbm, pay_hbm, zrows_hbm, d_hbm,
          dbuf_v, cidx_v, cseg_v, prow_v, zbuf_v, bnc_v,
          acc_sh, semf):
        cid = lax.axis_index("c")
        sid = lax.axis_index("s")
        iota = lax.iota(jnp.int32, L)
        pltpu.sync_copy(zrows_hbm, zbuf_v)

        def flush_one(tail):
            b = lax.rem(lax.div(tail, FB), RING // FB)
            pltpu.async_copy(pay_hbm.at[cidx_v.at[b]], prow_v, semf).wait()
            pltpu.sync_copy(prow_v, acc_sh.at[cseg_v.at[b]], add=True)
            return tail + FB

        def pass_body(p, _):
            lo = (p * NC + cid) * SEGC

            def zchunk(c, _):
                pltpu.sync_copy(zbuf_v, acc_sh.at[pl.ds(sid * stripe + c * 80, 80)])
                return 0
            lax.fori_loop(0, nzch, zchunk, 0)
            plsc.subcore_barrier()

            def stage_body(sc, carry):
                pos, tail = carry
                sbase = sid * per_s + sc * SCH
                pltpu.sync_copy(dst_hbm.at[pl.ds(sbase, SCH)], dbuf_v)

                def vec_body(it, carry):
                    pos, tail = carry
                    dv = dbuf_v[pl.ds(it * L, L)]
                    msk = (dv >= lo) & (dv < lo + SEGC)
                    m01f = jnp.where(msk, 1.0, 0.0)
                    incl = plsc.cumsum(m01f).astype(jnp.int32)
                    cnt = jnp.sum(m01f).astype(jnp.int32)
                    posv = pos + incl - 1
                    rp = jnp.bitwise_and(posv, RING - 1)
                    row = jnp.right_shift(rp, 7)
                    col = jnp.bitwise_and(rp, FB - 1)
                    ids = sbase + it * L + iota
                    plsc.store_scatter(cidx_v, [row, col], ids, mask=msk)
                    plsc.store_scatter(cseg_v, [row, col], dv - lo, mask=msk)
                    pos = pos + cnt
                    pos, tail = lax.while_loop(
                        lambda c: c[0] - c[1] >= FB,
                        lambda c: (c[0], flush_one(c[1])),
                        (pos, tail))
                    return pos, tail

                return lax.fori_loop(0, nvec, vec_body, (pos, tail))

            pos, tail = lax.fori_loop(0, nstage, stage_body,
                                      (jnp.int32(0), jnp.int32(0)))

            nfill = jnp.bitwise_and(-pos, FB - 1)
            for h in range(FB // L):
                fpos = pos + h * L + iota
                fmask = (fpos - pos) < nfill
                rp = jnp.bitwise_and(fpos, RING - 1)
                row = jnp.right_shift(rp, 7)
                col = jnp.bitwise_and(rp, FB - 1)
                plsc.store_scatter(cidx_v, [row, col],
                                   jnp.zeros((L,), jnp.int32), mask=fmask)
                plsc.store_scatter(cseg_v, [row, col],
                                   jnp.full((L,), DUMMY, jnp.int32), mask=fmask)
            pos = pos + nfill
            pos, tail = lax.while_loop(
                lambda c: c[0] - c[1] >= FB,
                lambda c: (c[0], flush_one(c[1])),
                (pos, tail))
            plsc.subcore_barrier()

            def fin_body(kk, _):
                c = kk * NS + sid

                @pl.when(c < nfch)
                def _():
                    off = c * 80
                    pltpu.sync_copy(acc_sh.at[pl.ds(off, 80)], bnc_v)
                    pltpu.sync_copy(bnc_v, d_hbm.at[pl.ds(lo + off, 80)])
                return 0

            lax.fori_loop(0, nfk, fin_body, 0)
            plsc.subcore_barrier()
            return 0

        lax.fori_loop(0, K, pass_body, 0)

    return k(dst_idx, pay, zrows)


def _segfin_body(d_ref, u_ref, t_out, s_out, q_out, acc_s, acc_q):
    i = pl.program_id(0)

    @pl.when(i == 0)
    def _init():
        acc_s[...] = jnp.zeros_like(acc_s)
        acc_q[...] = jnp.zeros_like(acc_q)

    d = d_ref[...]
    t = u_ref[...] + d[:, :BT] / (d[:, BT:] + EPS_DIV)
    t_out[...] = t
    acc_s[...] += jnp.sum(t, axis=0, keepdims=True)
    acc_q[...] += jnp.sum(t * t, axis=0, keepdims=True)

    @pl.when(i == pl.num_programs(0) - 1)
    def _fin():
        s_out[...] = acc_s[...]
        q_out[...] = acc_q[...]


def _segfin_pass(d, u, R):
    M = u.shape[0]
    n = M // R
    return pl.pallas_call(
        _segfin_body,
        grid=(n,),
        in_specs=[pl.BlockSpec((R, 2 * BT), lambda i: (i, 0)),
                  pl.BlockSpec((R, BT), lambda i: (i, 0))],
        out_specs=[pl.BlockSpec((R, BT), lambda i: (i, 0)),
                   pl.BlockSpec((1, BT), lambda i: (0, 0)),
                   pl.BlockSpec((1, BT), lambda i: (0, 0))],
        out_shape=[jax.ShapeDtypeStruct((M, BT), jnp.float32),
                   jax.ShapeDtypeStruct((1, BT), jnp.float32),
                   jax.ShapeDtypeStruct((1, BT), jnp.float32)],
        scratch_shapes=[pltpu.VMEM((1, BT), jnp.float32),
                        pltpu.VMEM((1, BT), jnp.float32)],
    )(d, u)


def _combine_body(a0_ref, a1_ref, u_ref, t_ref):
    ah = a0_ref[:, :BT] + a1_ref[:, :BT]
    as_ = a0_ref[:, BT:] + a1_ref[:, BT:]
    t_ref[...] = u_ref[...] + ah / (as_ + EPS_DIV)


def _combine_pass(a0, a1, u, R):
    M = u.shape[0]
    n = M // R
    return pl.pallas_call(
        _combine_body,
        grid=(n,),
        in_specs=[pl.BlockSpec((R, 2 * BT), lambda i: (i, 0)),
                  pl.BlockSpec((R, 2 * BT), lambda i: (i, 0)),
                  pl.BlockSpec((R, BT), lambda i: (i, 0))],
        out_specs=pl.BlockSpec((R, BT), lambda i: (i, 0)),
        out_shape=jax.ShapeDtypeStruct((M, BT), jnp.float32),
    )(a0, a1, u)


def kernel(x, y, z, edge_index, lg_edge_index, params):
    N = x.shape[0]
    E = y.shape[0]
    EL = z.shape[0]
    R = 2000
    p = params

    qy = p["edge_upd"]
    src_tab_y, dst_tab_y, u_y = _bottleneck_heads(
        y, p["pair_bn"], R,
        [jnp.concatenate([qy["Wsg"], qy["Wdu"]], axis=1), qy["Wdg"], qy["Wsu"]])
    (zweg,) = _bottleneck_heads(z, p["trip_bn"], R, [qy["Weg"]])
    qx = p["node_upd"]
    src_tab_x, dst_tab_x, u_x = _bottleneck_heads(
        x, p["node_bn"], R,
        [jnp.concatenate([qx["Wsg"], qx["Wdu"]], axis=1), qx["Wdg"], qx["Wsu"]])

    src_l, dst_l = lg_edge_index[0], lg_edge_index[1]
    m_l, pay_l, st_l = _sc_edge_gate(src_l, dst_l, src_tab_y, dst_tab_y, zweg)
    stls = jnp.sum(st_l, axis=0)
    s_ml, ssq_ml = stls[0], stls[1]

    zrows = jnp.zeros((80, 2 * BT), jnp.float32)
    (segdump,) = _sc_seg_sum(dst_l, pay_l, zrows, E)
    t, St, Qt = _segfin_pass(segdump, u_y, R)
    sc_t, sh_t = _bn_scale_shift(St[0], Qt[0], E, qy["gn"], qy["bn"])
    w = _tw_pass(t, sc_t[None, :], sh_t[None, :], qx["Weg"], R)

    sc_m, sh_m = _bn_scale_shift(s_ml, ssq_ml, EL, qy["ge"], qy["be"])
    z2, Sz2, Gz2 = _ewstats_pass(m_l, sc_m[None, :], sh_m[None, :], R)

    src_g, dst_g = edge_index[0], edge_index[1]
    NPAD = 10240
    m2, dump, st2 = _sc_node_conv(src_g, dst_g, src_tab_x, dst_tab_x, w, NPAD)
    st2s = jnp.sum(st2, axis=0)
    s_m2, ssq_m2 = st2s[0], st2s[1]

    t2 = _combine_pass(dump[0, :N], dump[1, :N], u_x, 2000)
    St2, Gt2 = _gram_pass(t2, R)
    sc_t2, sh_t2 = _bn_scale_shift(St2[0], jnp.diag(Gt2), N, qx["gn"], qx["bn"])
    x2, Sx2, Gx2 = _ewstats_pass(t2, sc_t2[None, :], sh_t2[None, :], R)

    sc_m2, sh_m2 = _bn_scale_shift(s_m2, ssq_m2, E, qx["ge"], qx["be"])
    y2, Sy2, Gy2 = _ewstats_pass(m2, sc_m2[None, :], sh_m2[None, :], R)

    qe = p["node_ex"]
    Wxf, bxf = _fold_linear_bn(Sx2, Gx2, N, qe["W"], qe["b"], qe["g"], qe["be"])
    xo = _expand_pass(x2, Wxf, bxf, x, R)
    qe = p["pair_ex"]
    Wyf, byf = _fold_linear_bn(Sy2, Gy2, E, qe["W"], qe["b"], qe["g"], qe["be"])
    yo = _expand_pass(y2, Wyf, byf, y, R)
    qe = p["trip_ex"]
    Wzf, bzf = _fold_linear_bn(Sz2, Gz2, EL, qe["W"], qe["b"], qe["g"], qe["be"])
    zo = _expand_pass(z2, Wzf, bzf, z, R)

    return xo, yo, zo


# speedup vs baseline: 6.1404x; 1.0105x over previous
import functools

import jax
import jax.numpy as jnp
from jax import lax
from jax.experimental import pallas as pl
from jax.experimental.pallas import tpu as pltpu
from jax.experimental.pallas import tpu_sc as plsc

D = 128
H = 64
BT = 32

NC = 2
NS = 16
L = 16
NW = NC * NS

EPS_BN = 1e-5
EPS_DIV = 1e-6


def _silu(v):
    return v * (1.0 / (1.0 + jnp.exp(-v)))


def _gram_body(x_ref, s_out, g_out, acc_s, acc_g):
    i = pl.program_id(0)

    @pl.when(i == 0)
    def _init():
        acc_s[...] = jnp.zeros_like(acc_s)
        acc_g[...] = jnp.zeros_like(acc_g)

    x = x_ref[...]
    acc_s[...] += jnp.sum(x, axis=0, keepdims=True)
    acc_g[...] += lax.dot_general(x, x, (((0,), (0,)), ((), ())),
                                  preferred_element_type=jnp.float32)

    @pl.when(i == pl.num_programs(0) - 1)
    def _fin():
        s_out[...] = acc_s[...]
        g_out[...] = acc_g[...]


def _gram_pass(x, R):
    M, Dx = x.shape
    n = M // R
    return pl.pallas_call(
        _gram_body,
        grid=(n,),
        in_specs=[pl.BlockSpec((R, Dx), lambda i: (i, 0))],
        out_specs=[pl.BlockSpec((1, Dx), lambda i: (0, 0)),
                   pl.BlockSpec((Dx, Dx), lambda i: (0, 0))],
        out_shape=[jax.ShapeDtypeStruct((1, Dx), jnp.float32),
                   jax.ShapeDtypeStruct((Dx, Dx), jnp.float32)],
        scratch_shapes=[pltpu.VMEM((1, Dx), jnp.float32),
                        pltpu.VMEM((Dx, Dx), jnp.float32)],
    )(x)


def _mid_body(x_ref, w_ref, b_ref, a_out, s_out, g_out, acc_s, acc_g):
    i = pl.program_id(0)

    @pl.when(i == 0)
    def _init():
        acc_s[...] = jnp.zeros_like(acc_s)
        acc_g[...] = jnp.zeros_like(acc_g)

    a = _silu(jnp.dot(x_ref[...], w_ref[...],
                      preferred_element_type=jnp.float32) + b_ref[...])
    a_out[...] = a
    acc_s[...] += jnp.sum(a, axis=0, keepdims=True)
    acc_g[...] += lax.dot_general(a, a, (((0,), (0,)), ((), ())),
                                  preferred_element_type=jnp.float32)

    @pl.when(i == pl.num_programs(0) - 1)
    def _fin():
        s_out[...] = acc_s[...]
        g_out[...] = acc_g[...]


def _mid_pass(x, Wf, bf, R):
    M, Dx = x.shape
    Hx = Wf.shape[1]
    n = M // R
    return pl.pallas_call(
        _mid_body,
        grid=(n,),
        in_specs=[pl.BlockSpec((R, Dx), lambda i: (i, 0)),
                  pl.BlockSpec((Dx, Hx), lambda i: (0, 0)),
                  pl.BlockSpec((1, Hx), lambda i: (0, 0))],
        out_specs=[pl.BlockSpec((R, Hx), lambda i: (i, 0)),
                   pl.BlockSpec((1, Hx), lambda i: (0, 0)),
                   pl.BlockSpec((Hx, Hx), lambda i: (0, 0))],
        out_shape=[jax.ShapeDtypeStruct((M, Hx), jnp.float32),
                   jax.ShapeDtypeStruct((1, Hx), jnp.float32),
                   jax.ShapeDtypeStruct((Hx, Hx), jnp.float32)],
        scratch_shapes=[pltpu.VMEM((1, Hx), jnp.float32),
                        pltpu.VMEM((Hx, Hx), jnp.float32)],
    )(x, Wf, bf)


def _apply_post_pass(a, W2f, b2f, posts, R):
    M, Hx = a.shape
    Bx = W2f.shape[1]
    n = M // R
    widths = [p.shape[1] for p in posts]
    npost = len(posts)

    def body(*refs):
        a_ref = refs[0]
        w_ref = refs[1]
        b_ref = refs[2]
        post_refs = refs[3:3 + npost]
        out_refs = refs[3 + npost:3 + 2 * npost]
        v = _silu(jnp.dot(a_ref[...], w_ref[...],
                          preferred_element_type=jnp.float32) + b_ref[...])
        for pr, orf in zip(post_refs, out_refs):
            orf[...] = jnp.dot(v, pr[...], preferred_element_type=jnp.float32)

    in_specs = [pl.BlockSpec((R, Hx), lambda i: (i, 0)),
                pl.BlockSpec((Hx, Bx), lambda i: (0, 0)),
                pl.BlockSpec((1, Bx), lambda i: (0, 0))]
    for w in widths:
        in_specs.append(pl.BlockSpec((Bx, w), lambda i: (0, 0)))
    out_specs = [pl.BlockSpec((R, w), lambda i: (i, 0)) for w in widths]
    out_shape = [jax.ShapeDtypeStruct((M, w), jnp.float32) for w in widths]
    outs = pl.pallas_call(
        body,
        grid=(n,),
        in_specs=in_specs,
        out_specs=out_specs,
        out_shape=out_shape,
    )(a, W2f, b2f, *posts)
    return outs


def _ewstats_body(t_ref, sc_ref, sh_ref, v_out, s_out, g_out, acc_s, acc_g):
    i = pl.program_id(0)

    @pl.when(i == 0)
    def _init():
        acc_s[...] = jnp.zeros_like(acc_s)
        acc_g[...] = jnp.zeros_like(acc_g)

    v = _silu(t_ref[...] * sc_ref[...] + sh_ref[...])
    v_out[...] = v
    acc_s[...] += jnp.sum(v, axis=0, keepdims=True)
    acc_g[...] += lax.dot_general(v, v, (((0,), (0,)), ((), ())),
                                  preferred_element_type=jnp.float32)

    @pl.when(i == pl.num_programs(0) - 1)
    def _fin():
        s_out[...] = acc_s[...]
        g_out[...] = acc_g[...]


def _ewstats_pass(t, scale, shift, R):
    M, Bx = t.shape
    n = M // R
    return pl.pallas_call(
        _ewstats_body,
        grid=(n,),
        in_specs=[pl.BlockSpec((R, Bx), lambda i: (i, 0)),
                  pl.BlockSpec((1, Bx), lambda i: (0, 0)),
                  pl.BlockSpec((1, Bx), lambda i: (0, 0))],
        out_specs=[pl.BlockSpec((R, Bx), lambda i: (i, 0)),
                   pl.BlockSpec((1, Bx), lambda i: (0, 0)),
                   pl.BlockSpec((Bx, Bx), lambda i: (0, 0))],
        out_shape=[jax.ShapeDtypeStruct((M, Bx), jnp.float32),
                   jax.ShapeDtypeStruct((1, Bx), jnp.float32),
                   jax.ShapeDtypeStruct((Bx, Bx), jnp.float32)],
        scratch_shapes=[pltpu.VMEM((1, Bx), jnp.float32),
                        pltpu.VMEM((Bx, Bx), jnp.float32)],
    )(t, scale, shift)


def _expand_body(v_ref, w_ref, b_ref, r_ref, o_ref):
    o_ref[...] = _silu(jnp.dot(v_ref[...], w_ref[...],
                               preferred_element_type=jnp.float32)
                       + b_ref[...]) + r_ref[...]


def _expand_pass(v, Wf, bf, resid, R):
    M, Bx = v.shape
    Dx = Wf.shape[1]
    n = M // R
    return pl.pallas_call(
        _expand_body,
        grid=(n,),
        in_specs=[pl.BlockSpec((R, Bx), lambda i: (i, 0)),
                  pl.BlockSpec((Bx, Dx), lambda i: (0, 0)),
                  pl.BlockSpec((1, Dx), lambda i: (0, 0)),
                  pl.BlockSpec((R, Dx), lambda i: (i, 0))],
        out_specs=pl.BlockSpec((R, Dx), lambda i: (i, 0)),
        out_shape=jax.ShapeDtypeStruct((M, Dx), jnp.float32),
    )(v, Wf, bf, resid)


def _tw_body(t_ref, sc_ref, sh_ref, w_ref, o_ref):
    m = _silu(t_ref[...] * sc_ref[...] + sh_ref[...])
    o_ref[...] = jnp.dot(m, w_ref[...], preferred_element_type=jnp.float32)


def _tw_pass(t, scale, shift, Weg, R):
    M, Bx = t.shape
    n = M // R
    return pl.pallas_call(
        _tw_body,
        grid=(n,),
        in_specs=[pl.BlockSpec((R, Bx), lambda i: (i, 0)),
                  pl.BlockSpec((1, Bx), lambda i: (0, 0)),
                  pl.BlockSpec((1, Bx), lambda i: (0, 0)),
                  pl.BlockSpec((Bx, Bx), lambda i: (0, 0))],
        out_specs=pl.BlockSpec((R, Bx), lambda i: (i, 0)),
        out_shape=jax.ShapeDtypeStruct((M, Bx), jnp.float32),
    )(t, scale, shift, Weg)


def _fold_linear_bn(S, G, M, W, b, g, be):
    mu_in = S / M
    mean = mu_in @ W + b
    ex2 = jnp.sum(W * (G @ W), axis=0) / M
    var = ex2 - jnp.square(mu_in @ W)[0]
    scale = g / jnp.sqrt(var + EPS_BN)
    Wf = W * scale[None, :]
    bf = (b - mean[0]) * scale + be
    return Wf, bf[None, :]


def _bn_scale_shift(s, ss, M, g, be):
    mean = s / M
    var = ss / M - jnp.square(mean)
    scale = g / jnp.sqrt(var + EPS_BN)
    shift = be - mean * scale
    return scale, shift


def _bottleneck_heads(x, q, R, posts):
    M = x.shape[0]
    S1, G1 = _gram_pass(x, R)
    W1f, b1f = _fold_linear_bn(S1, G1, M, q["W1"], q["b1"], q["g1"], q["be1"])
    a, S2, G2 = _mid_pass(x, W1f, b1f, R)
    W2f, b2f = _fold_linear_bn(S2, G2, M, q["W2"], q["b2"], q["g2"], q["be2"])
    return _apply_post_pass(a, W2f, b2f, posts, R)


def _sc_edge_gate(src_idx, dst_idx, src_tab, dst_tab, ew, CH=80):
    EL = src_idx.shape[0]
    per_w = EL // NW
    nch = per_w // CH
    mesh = plsc.VectorSubcoreMesh(core_axis_name="c", subcore_axis_name="s")

    @functools.partial(
        pl.kernel,
        out_type=[jax.ShapeDtypeStruct((EL, BT), jnp.float32),
                  jax.ShapeDtypeStruct((EL + 8, 2 * BT), jnp.float32),
                  jax.ShapeDtypeStruct((NW, 2, BT), jnp.float32)],
        mesh=mesh,
        compiler_params=pltpu.CompilerParams(use_tc_tiling_on_sc=False),
        scratch_types=(
            [pltpu.VMEM((CH,), jnp.int32)] * 4
            + [pltpu.VMEM((CH, 2 * BT), jnp.float32)] * 2
            + [pltpu.VMEM((CH, BT), jnp.float32)] * 4
            + [pltpu.VMEM((CH, BT), jnp.float32)] * 2
            + [pltpu.VMEM((CH, 2 * BT), jnp.float32)] * 2
            + [pltpu.VMEM((2, BT), jnp.float32)]
            + [pltpu.SemaphoreType.DMA] * 2
        ),
    )
    def k(src_hbm, dst_hbm, stab_hbm, dtab_hbm, ew_hbm,
          m_hbm, p_hbm, st_hbm,
          idxs0, idxs1, idxd0, idxd1, srows0, srows1,
          drows0, drows1, erows0, erows1, mbuf0, mbuf1, pbuf0, pbuf1, stat_v,
          sem0, sem1):
        wid = lax.axis_index("s") * NC + lax.axis_index("c")
        base_w = wid * per_w
        zero = jnp.zeros((L,), jnp.float32)
        for r in range(2):
            for cv in range(BT // L):
                stat_v[r, pl.ds(cv * L, L)] = zero

        idxs = (idxs0, idxs1)
        idxd = (idxd0, idxd1)
        srows = (srows0, srows1)
        drows = (drows0, drows1)
        erows = (erows0, erows1)
        mbuf = (mbuf0, mbuf1)
        pbuf = (pbuf0, pbuf1)
        sems = (sem0, sem1)

        def start(ci, b):
            base = base_w + ci * CH
            pltpu.sync_copy(src_hbm.at[pl.ds(base, CH)], idxs[b])
            pltpu.sync_copy(dst_hbm.at[pl.ds(base, CH)], idxd[b])
            pltpu.async_copy(stab_hbm.at[idxs[b]], srows[b], sems[b])
            pltpu.async_copy(dtab_hbm.at[idxd[b]], drows[b], sems[b])
            pltpu.async_copy(ew_hbm.at[pl.ds(base, CH)], erows[b], sems[b])

        def finish(ci, b):
            pltpu.make_async_copy(stab_hbm.at[idxs[b]], srows[b], sems[b]).wait()
            pltpu.make_async_copy(dtab_hbm.at[idxd[b]], drows[b], sems[b]).wait()
            base = base_w + ci * CH
            pltpu.make_async_copy(ew_hbm.at[pl.ds(base, CH)], erows[b], sems[b]).wait()

        def compute(ci, b):
            base = base_w + ci * CH

            def row_body(j, carry):
                accs = list(carry)
                for h in range(BT // L):
                    es = srows[b][j, pl.ds(L * h, L)]
                    bh = srows[b][j, pl.ds(BT + L * h, L)]
                    ed = drows[b][j, pl.ds(L * h, L)]
                    zw = erows[b][j, pl.ds(L * h, L)]
                    m = es + ed + zw
                    sg = 1.0 / (1.0 + jnp.exp(-m))
                    v = sg * bh
                    mbuf[b][j, pl.ds(L * h, L)] = m
                    pbuf[b][j, pl.ds(L * h, L)] = v
                    pbuf[b][j, pl.ds(BT + L * h, L)] = sg
                    accs[h] = accs[h] + m
                    accs[2 + h] = accs[2 + h] + m * m
                return tuple(accs)

            accs = lax.fori_loop(0, CH, row_body, (zero, zero, zero, zero))
            for h in range(BT // L):
                stat_v[0, pl.ds(L * h, L)] += accs[h]
                stat_v[1, pl.ds(L * h, L)] += accs[2 + h]
            pltpu.sync_copy(mbuf[b], m_hbm.at[pl.ds(base, CH)])
            pltpu.sync_copy(pbuf[b], p_hbm.at[pl.ds(base, CH)])

        start(0, 0)

        def pair_body(i, _):
            for b in range(2):
                ci = i * 2 + b

                @pl.when(ci + 1 < nch)
                def _():
                    start(ci + 1, 1 - b)
                finish(ci, b)
                compute(ci, b)
            return 0

        lax.fori_loop(0, nch // 2, pair_body, 0)
        if nch % 2:
            finish(nch - 1, (nch - 1) % 2)
            compute(nch - 1, (nch - 1) % 2)
        pltpu.sync_copy(stat_v, st_hbm.at[wid])

        @pl.when(wid == 0)
        def _():
            for j in range(8):
                for h in range(2 * BT // L):
                    pbuf0[j, pl.ds(L * h, L)] = zero
            pltpu.sync_copy(pbuf0.at[pl.ds(0, 8)], p_hbm.at[pl.ds(EL, 8)])

    return k(src_idx, dst_idx, src_tab, dst_tab, ew)


def _sc_node_conv(src_idx, dst_idx, src_tab, dst_tab, ew, NPAD, CH=80):
    E = src_idx.shape[0]
    per_w = E // NW
    nch = per_w // CH
    stripe = NPAD // NS
    nsch = stripe // CH
    mesh = plsc.VectorSubcoreMesh(core_axis_name="c", subcore_axis_name="s")

    @functools.partial(
        pl.kernel,
        out_type=[jax.ShapeDtypeStruct((E, BT), jnp.float32),
                  jax.ShapeDtypeStruct((NC, NPAD, 2 * BT), jnp.float32),
                  jax.ShapeDtypeStruct((NW, 2, BT), jnp.float32)],
        mesh=mesh,
        compiler_params=pltpu.CompilerParams(use_tc_tiling_on_sc=False),
        scratch_types=(
            [pltpu.VMEM((CH,), jnp.int32)] * 4
            + [pltpu.VMEM((CH, 2 * BT), jnp.float32)] * 2
            + [pltpu.VMEM((CH, BT), jnp.float32)] * 4
            + [pltpu.VMEM((CH, BT), jnp.float32)] * 2
            + [pltpu.VMEM((CH, 2 * BT), jnp.float32)] * 2
            + [pltpu.VMEM((2, BT), jnp.float32)]
            + [pltpu.VMEM_SHARED((NPAD, 2 * BT), jnp.float32)]
            + [pltpu.SemaphoreType.DMA] * 2
        ),
    )
    def k(src_hbm, dst_hbm, stab_hbm, dtab_hbm, ew_hbm,
          m_hbm, dump_hbm, st_hbm,
          idxs0, idxs1, idxd0, idxd1, srows0, srows1,
          drows0, drows1, erows0, erows1, mbuf0, mbuf1, pbuf0, pbuf1, stat_v,
          acc_sh, sem0, sem1):
        cid = lax.axis_index("c")
        sid = lax.axis_index("s")
        wid = sid * NC + cid
        base_w = wid * per_w
        zero = jnp.zeros((L,), jnp.float32)

        idxs = (idxs0, idxs1)
        idxd = (idxd0, idxd1)
        srows = (srows0, srows1)
        drows = (drows0, drows1)
        erows = (erows0, erows1)
        mbuf = (mbuf0, mbuf1)
        pbuf = (pbuf0, pbuf1)
        sems = (sem0, sem1)

        def zrow(j, _):
            for h in range(2 * BT // L):
                pbuf0[j, pl.ds(L * h, L)] = zero
            return 0
        lax.fori_loop(0, CH, zrow, 0)

        def zchunk(c, _):
            pltpu.sync_copy(pbuf0, acc_sh.at[pl.ds(sid * stripe + c * CH, CH)])
            return 0
        lax.fori_loop(0, nsch, zchunk, 0)
        for r in range(2):
            for cv in range(BT // L):
                stat_v[r, pl.ds(cv * L, L)] = zero
        plsc.subcore_barrier()

        def start(ci, b):
            base = base_w + ci * CH
            pltpu.sync_copy(src_hbm.at[pl.ds(base, CH)], idxs[b])
            pltpu.sync_copy(dst_hbm.at[pl.ds(base, CH)], idxd[b])
            pltpu.async_copy(stab_hbm.at[idxs[b]], srows[b], sems[b])
            pltpu.async_copy(dtab_hbm.at[idxd[b]], drows[b], sems[b])
            pltpu.async_copy(ew_hbm.at[pl.ds(base, CH)], erows[b], sems[b])

        def finish(ci, b):
            pltpu.make_async_copy(stab_hbm.at[idxs[b]], srows[b], sems[b]).wait()
            pltpu.make_async_copy(dtab_hbm.at[idxd[b]], drows[b], sems[b]).wait()
            base = base_w + ci * CH
            pltpu.make_async_copy(ew_hbm.at[pl.ds(base, CH)], erows[b], sems[b]).wait()

        def compute(ci, b):
            base = base_w + ci * CH

            def row_body(j, carry):
                accs = list(carry)
                for h in range(BT // L):
                    es = srows[b][j, pl.ds(L * h, L)]
                    bh = srows[b][j, pl.ds(BT + L * h, L)]
                    ed = drows[b][j, pl.ds(L * h, L)]
                    zw = erows[b][j, pl.ds(L * h, L)]
                    m = es + ed + zw
                    sg = 1.0 / (1.0 + jnp.exp(-m))
                    v = sg * bh
                    mbuf[b][j, pl.ds(L * h, L)] = m
                    pbuf[b][j, pl.ds(L * h, L)] = v
                    pbuf[b][j, pl.ds(BT + L * h, L)] = sg
                    accs[h] = accs[h] + m
                    accs[2 + h] = accs[2 + h] + m * m
                return tuple(accs)

            accs = lax.fori_loop(0, CH, row_body, (zero, zero, zero, zero))
            for h in range(BT // L):
                stat_v[0, pl.ds(L * h, L)] += accs[h]
                stat_v[1, pl.ds(L * h, L)] += accs[2 + h]
            pltpu.sync_copy(mbuf[b], m_hbm.at[pl.ds(base, CH)])
            pltpu.sync_copy(pbuf[b], acc_sh.at[idxd[b]], add=True)

        start(0, 0)

        def pair_body(i, _):
            for b in range(2):
                ci = i * 2 + b

                @pl.when(ci + 1 < nch)
                def _():
                    start(ci + 1, 1 - b)
                finish(ci, b)
                compute(ci, b)
            return 0

        lax.fori_loop(0, nch // 2, pair_body, 0)
        if nch % 2:
            finish(nch - 1, (nch - 1) % 2)
            compute(nch - 1, (nch - 1) % 2)
        pltpu.sync_copy(stat_v, st_hbm.at[wid])
        plsc.subcore_barrier()

        def dchunk(c, _):
            off = sid * stripe + c * CH
            pltpu.sync_copy(acc_sh.at[pl.ds(off, CH)], pbuf0)
            pltpu.sync_copy(pbuf0, dump_hbm.at[cid, pl.ds(off, CH)])
            return 0
        lax.fori_loop(0, nsch, dchunk, 0)

    return k(src_idx, dst_idx, src_tab, dst_tab, ew)


def _sc_seg_sum(dst_idx, pay, zrows, E, K=8, SEGPAD=20480):
    EL = dst_idx.shape[0]
    SEGC = E // (K * NC)
    assert SEGC <= SEGPAD
    per_s = EL // NS
    SCH = 2000
    nstage = per_s // SCH
    nvec = SCH // L
    FB = 128
    RING = 1024
    stripe = SEGPAD // NS
    nzch = stripe // 80
    nfch = SEGC // 80
    nfk = (nfch + NS - 1) // NS
    mesh = plsc.VectorSubcoreMesh(core_axis_name="c", subcore_axis_name="s")

    @functools.partial(
        pl.kernel,
        out_type=[jax.ShapeDtypeStruct((E, 2 * BT), jnp.float32)],
        mesh=mesh,
        compiler_params=pltpu.CompilerParams(use_tc_tiling_on_sc=False,
                                             needs_layout_passes=False),
        scratch_types=[
            pltpu.VMEM((SCH,), jnp.int32),
            pltpu.VMEM((SCH,), jnp.int32),
            pltpu.VMEM((RING // FB, FB), jnp.int32),
            pltpu.VMEM((RING // FB, FB), jnp.int32),
            pltpu.VMEM((FB, 2 * BT), jnp.float32),
            pltpu.VMEM((80, 2 * BT), jnp.float32),
            pltpu.VMEM((80, 2 * BT), jnp.float32),
            pltpu.VMEM_SHARED((SEGPAD, 2 * BT), jnp.float32),
            pltpu.SemaphoreType.DMA,
            pltpu.SemaphoreType.DMA,
            pltpu.SemaphoreType.DMA,
        ],
    )
    def k(dst_hbm, pay_hbm, zrows_hbm, d_hbm,
          dbuf0, dbuf1, cidx_v, cseg_v, prow_v, zbuf_v, bnc_v,
          acc_sh, semf, dsem0, dsem1):
        cid = lax.axis_index("c")
        sid = lax.axis_index("s")
        iota = lax.iota(jnp.int32, L)
        pltpu.sync_copy(zrows_hbm, zbuf_v)
        dbuf = (dbuf0, dbuf1)
        dsem = (dsem0, dsem1)

        def stage_start(sc, b):
            sbase = sid * per_s + sc * SCH
            pltpu.async_copy(dst_hbm.at[pl.ds(sbase, SCH)], dbuf[b], dsem[b])

        def stage_wait(sc, b):
            sbase = sid * per_s + sc * SCH
            pltpu.make_async_copy(dst_hbm.at[pl.ds(sbase, SCH)],
                                  dbuf[b], dsem[b]).wait()

        def flush_one(tail):
            b = lax.rem(lax.div(tail, FB), RING // FB)
            pltpu.async_copy(pay_hbm.at[cidx_v.at[b]], prow_v, semf).wait()
            pltpu.sync_copy(prow_v, acc_sh.at[cseg_v.at[b]], add=True)
            return tail + FB

        def pass_body(p, _):
            lo = (p * NC + cid) * SEGC

            def zchunk(c, _):
                pltpu.sync_copy(zbuf_v, acc_sh.at[pl.ds(sid * stripe + c * 80, 80)])
                return 0
            lax.fori_loop(0, nzch, zchunk, 0)
            plsc.subcore_barrier()

            def vec_loop(dref, sbase, pos, tail):
                def vec_body(it, carry):
                    pos, tail = carry
                    dv = dref[pl.ds(it * L, L)]
                    msk = (dv >= lo) & (dv < lo + SEGC)
                    m01f = jnp.where(msk, 1.0, 0.0)
                    incl = plsc.cumsum(m01f).astype(jnp.int32)
                    cnt = jnp.sum(m01f).astype(jnp.int32)
                    posv = pos + incl - 1
                    rp = jnp.bitwise_and(posv, RING - 1)
                    row = jnp.right_shift(rp, 7)
                    col = jnp.bitwise_and(rp, FB - 1)
                    ids = sbase + it * L + iota
                    plsc.store_scatter(cidx_v, [row, col], ids, mask=msk)
                    plsc.store_scatter(cseg_v, [row, col], dv - lo, mask=msk)
                    pos = pos + cnt
                    pos, tail = lax.while_loop(
                        lambda c: c[0] - c[1] >= FB,
                        lambda c: (c[0], flush_one(c[1])),
                        (pos, tail))
                    return pos, tail

                return lax.fori_loop(0, nvec, vec_body, (pos, tail))

            stage_start(0, 0)

            def stage_pair(i, carry):
                pos, tail = carry
                for b in range(2):
                    sc = i * 2 + b

                    @pl.when(sc + 1 < nstage)
                    def _():
                        stage_start(sc + 1, 1 - b)
                    stage_wait(sc, b)
                    sbase = sid * per_s + sc * SCH
                    pos, tail = vec_loop(dbuf[b], sbase, pos, tail)
                return pos, tail

            pos, tail = lax.fori_loop(0, nstage // 2, stage_pair,
                                      (jnp.int32(0), jnp.int32(0)))

            nfill = jnp.bitwise_and(-pos, FB - 1)
            for h in range(FB // L):
                fpos = pos + h * L + iota
                fmask = (fpos - pos) < nfill
                rp = jnp.bitwise_and(fpos, RING - 1)
                row = jnp.right_shift(rp, 7)
                col = jnp.bitwise_and(rp, FB - 1)
                plsc.store_scatter(cidx_v, [row, col],
                                   jnp.full((L,), EL, jnp.int32), mask=fmask)
                plsc.store_scatter(cseg_v, [row, col],
                                   jnp.zeros((L,), jnp.int32), mask=fmask)
            pos = pos + nfill
            pos, tail = lax.while_loop(
                lambda c: c[0] - c[1] >= FB,
                lambda c: (c[0], flush_one(c[1])),
                (pos, tail))
            plsc.subcore_barrier()

            def fin_body(kk, _):
                c = kk * NS + sid

                @pl.when(c < nfch)
                def _():
                    off = c * 80
                    pltpu.sync_copy(acc_sh.at[pl.ds(off, 80)], bnc_v)
                    pltpu.sync_copy(bnc_v, d_hbm.at[pl.ds(lo + off, 80)])
                return 0

            lax.fori_loop(0, nfk, fin_body, 0)
            plsc.subcore_barrier()
            return 0

        lax.fori_loop(0, K, pass_body, 0)

    return k(dst_idx, pay, zrows)


def _segfin_body(d_ref, u_ref, t_out, s_out, q_out, acc_s, acc_q):
    i = pl.program_id(0)

    @pl.when(i == 0)
    def _init():
        acc_s[...] = jnp.zeros_like(acc_s)
        acc_q[...] = jnp.zeros_like(acc_q)

    d = d_ref[...]
    t = u_ref[...] + d[:, :BT] / (d[:, BT:] + EPS_DIV)
    t_out[...] = t
    acc_s[...] += jnp.sum(t, axis=0, keepdims=True)
    acc_q[...] += jnp.sum(t * t, axis=0, keepdims=True)

    @pl.when(i == pl.num_programs(0) - 1)
    def _fin():
        s_out[...] = acc_s[...]
        q_out[...] = acc_q[...]


def _segfin_pass(d, u, R):
    M = u.shape[0]
    n = M // R
    return pl.pallas_call(
        _segfin_body,
        grid=(n,),
        in_specs=[pl.BlockSpec((R, 2 * BT), lambda i: (i, 0)),
                  pl.BlockSpec((R, BT), lambda i: (i, 0))],
        out_specs=[pl.BlockSpec((R, BT), lambda i: (i, 0)),
                   pl.BlockSpec((1, BT), lambda i: (0, 0)),
                   pl.BlockSpec((1, BT), lambda i: (0, 0))],
        out_shape=[jax.ShapeDtypeStruct((M, BT), jnp.float32),
                   jax.ShapeDtypeStruct((1, BT), jnp.float32),
                   jax.ShapeDtypeStruct((1, BT), jnp.float32)],
        scratch_shapes=[pltpu.VMEM((1, BT), jnp.float32),
                        pltpu.VMEM((1, BT), jnp.float32)],
    )(d, u)


def _combine_body(a0_ref, a1_ref, u_ref, t_ref):
    ah = a0_ref[:, :BT] + a1_ref[:, :BT]
    as_ = a0_ref[:, BT:] + a1_ref[:, BT:]
    t_ref[...] = u_ref[...] + ah / (as_ + EPS_DIV)


def _combine_pass(a0, a1, u, R):
    M = u.shape[0]
    n = M // R
    return pl.pallas_call(
        _combine_body,
        grid=(n,),
        in_specs=[pl.BlockSpec((R, 2 * BT), lambda i: (i, 0)),
                  pl.BlockSpec((R, 2 * BT), lambda i: (i, 0)),
                  pl.BlockSpec((R, BT), lambda i: (i, 0))],
        out_specs=pl.BlockSpec((R, BT), lambda i: (i, 0)),
        out_shape=jax.ShapeDtypeStruct((M, BT), jnp.float32),
    )(a0, a1, u)


def kernel(x, y, z, edge_index, lg_edge_index, params):
    N = x.shape[0]
    E = y.shape[0]
    EL = z.shape[0]
    R = 2000
    p = params

    qy = p["edge_upd"]
    src_tab_y, dst_tab_y, u_y = _bottleneck_heads(
        y, p["pair_bn"], R,
        [jnp.concatenate([qy["Wsg"], qy["Wdu"]], axis=1), qy["Wdg"], qy["Wsu"]])
    (zweg,) = _bottleneck_heads(z, p["trip_bn"], R, [qy["Weg"]])
    qx = p["node_upd"]
    src_tab_x, dst_tab_x, u_x = _bottleneck_heads(
        x, p["node_bn"], R,
        [jnp.concatenate([qx["Wsg"], qx["Wdu"]], axis=1), qx["Wdg"], qx["Wsu"]])

    src_l, dst_l = lg_edge_index[0], lg_edge_index[1]
    m_l, pay_l, st_l = _sc_edge_gate(src_l, dst_l, src_tab_y, dst_tab_y, zweg)
    stls = jnp.sum(st_l, axis=0)
    s_ml, ssq_ml = stls[0], stls[1]

    zrows = jnp.zeros((80, 2 * BT), jnp.float32)
    (segdump,) = _sc_seg_sum(dst_l, pay_l, zrows, E)
    t, St, Qt = _segfin_pass(segdump, u_y, R)
    sc_t, sh_t = _bn_scale_shift(St[0], Qt[0], E, qy["gn"], qy["bn"])
    w = _tw_pass(t, sc_t[None, :], sh_t[None, :], qx["Weg"], R)

    sc_m, sh_m = _bn_scale_shift(s_ml, ssq_ml, EL, qy["ge"], qy["be"])
    z2, Sz2, Gz2 = _ewstats_pass(m_l, sc_m[None, :], sh_m[None, :], R)

    src_g, dst_g = edge_index[0], edge_index[1]
    NPAD = 10240
    m2, dump, st2 = _sc_node_conv(src_g, dst_g, src_tab_x, dst_tab_x, w, NPAD)
    st2s = jnp.sum(st2, axis=0)
    s_m2, ssq_m2 = st2s[0], st2s[1]

    t2 = _combine_pass(dump[0, :N], dump[1, :N], u_x, 2000)
    St2, Gt2 = _gram_pass(t2, R)
    sc_t2, sh_t2 = _bn_scale_shift(St2[0], jnp.diag(Gt2), N, qx["gn"], qx["bn"])
    x2, Sx2, Gx2 = _ewstats_pass(t2, sc_t2[None, :], sh_t2[None, :], R)

    sc_m2, sh_m2 = _bn_scale_shift(s_m2, ssq_m2, E, qx["ge"], qx["be"])
    y2, Sy2, Gy2 = _ewstats_pass(m2, sc_m2[None, :], sh_m2[None, :], R)

    qe = p["node_ex"]
    Wxf, bxf = _fold_linear_bn(Sx2, Gx2, N, qe["W"], qe["b"], qe["g"], qe["be"])
    xo = _expand_pass(x2, Wxf, bxf, x, R)
    qe = p["pair_ex"]
    Wyf, byf = _fold_linear_bn(Sy2, Gy2, E, qe["W"], qe["b"], qe["g"], qe["be"])
    yo = _expand_pass(y2, Wyf, byf, y, R)
    qe = p["trip_ex"]
    Wzf, bzf = _fold_linear_bn(Sz2, Gz2, EL, qe["W"], qe["b"], qe["g"], qe["be"])
    zo = _expand_pass(z2, Wzf, bzf, z, R)

    return xo, yo, zo


# speedup vs baseline: 6.9767x; 1.1362x over previous
import functools

import jax
import jax.numpy as jnp
from jax import lax
from jax.experimental import pallas as pl
from jax.experimental.pallas import tpu as pltpu
from jax.experimental.pallas import tpu_sc as plsc

D = 128
H = 64
BT = 32

NC = 2
NS = 16
L = 16
NW = NC * NS

EPS_BN = 1e-5
EPS_DIV = 1e-6


def _silu(v):
    return v * (1.0 / (1.0 + jnp.exp(-v)))


def _gram_body(x_ref, s_out, g_out, acc_s, acc_g):
    i = pl.program_id(0)

    @pl.when(i == 0)
    def _init():
        acc_s[...] = jnp.zeros_like(acc_s)
        acc_g[...] = jnp.zeros_like(acc_g)

    x = x_ref[...]
    acc_s[...] += jnp.sum(x, axis=0, keepdims=True)
    acc_g[...] += lax.dot_general(x, x, (((0,), (0,)), ((), ())),
                                  preferred_element_type=jnp.float32)

    @pl.when(i == pl.num_programs(0) - 1)
    def _fin():
        s_out[...] = acc_s[...]
        g_out[...] = acc_g[...]


def _gram_pass(x, R):
    M, Dx = x.shape
    n = M // R
    return pl.pallas_call(
        _gram_body,
        grid=(n,),
        in_specs=[pl.BlockSpec((R, Dx), lambda i: (i, 0))],
        out_specs=[pl.BlockSpec((1, Dx), lambda i: (0, 0)),
                   pl.BlockSpec((Dx, Dx), lambda i: (0, 0))],
        out_shape=[jax.ShapeDtypeStruct((1, Dx), jnp.float32),
                   jax.ShapeDtypeStruct((Dx, Dx), jnp.float32)],
        scratch_shapes=[pltpu.VMEM((1, Dx), jnp.float32),
                        pltpu.VMEM((Dx, Dx), jnp.float32)],
    )(x)


def _mid_body(x_ref, w_ref, b_ref, a_out, s_out, g_out, acc_s, acc_g):
    i = pl.program_id(0)

    @pl.when(i == 0)
    def _init():
        acc_s[...] = jnp.zeros_like(acc_s)
        acc_g[...] = jnp.zeros_like(acc_g)

    a = _silu(jnp.dot(x_ref[...], w_ref[...],
                      preferred_element_type=jnp.float32) + b_ref[...])
    a_out[...] = a
    acc_s[...] += jnp.sum(a, axis=0, keepdims=True)
    acc_g[...] += lax.dot_general(a, a, (((0,), (0,)), ((), ())),
                                  preferred_element_type=jnp.float32)

    @pl.when(i == pl.num_programs(0) - 1)
    def _fin():
        s_out[...] = acc_s[...]
        g_out[...] = acc_g[...]


def _mid_pass(x, Wf, bf, R):
    M, Dx = x.shape
    Hx = Wf.shape[1]
    n = M // R
    return pl.pallas_call(
        _mid_body,
        grid=(n,),
        in_specs=[pl.BlockSpec((R, Dx), lambda i: (i, 0)),
                  pl.BlockSpec((Dx, Hx), lambda i: (0, 0)),
                  pl.BlockSpec((1, Hx), lambda i: (0, 0))],
        out_specs=[pl.BlockSpec((R, Hx), lambda i: (i, 0)),
                   pl.BlockSpec((1, Hx), lambda i: (0, 0)),
                   pl.BlockSpec((Hx, Hx), lambda i: (0, 0))],
        out_shape=[jax.ShapeDtypeStruct((M, Hx), jnp.float32),
                   jax.ShapeDtypeStruct((1, Hx), jnp.float32),
                   jax.ShapeDtypeStruct((Hx, Hx), jnp.float32)],
        scratch_shapes=[pltpu.VMEM((1, Hx), jnp.float32),
                        pltpu.VMEM((Hx, Hx), jnp.float32)],
    )(x, Wf, bf)


def _apply_post_pass(a, W2f, b2f, posts, R):
    M, Hx = a.shape
    Bx = W2f.shape[1]
    n = M // R
    widths = [p.shape[1] for p in posts]
    npost = len(posts)

    def body(*refs):
        a_ref = refs[0]
        w_ref = refs[1]
        b_ref = refs[2]
        post_refs = refs[3:3 + npost]
        out_refs = refs[3 + npost:3 + 2 * npost]
        v = _silu(jnp.dot(a_ref[...], w_ref[...],
                          preferred_element_type=jnp.float32) + b_ref[...])
        for pr, orf in zip(post_refs, out_refs):
            orf[...] = jnp.dot(v, pr[...], preferred_element_type=jnp.float32)

    in_specs = [pl.BlockSpec((R, Hx), lambda i: (i, 0)),
                pl.BlockSpec((Hx, Bx), lambda i: (0, 0)),
                pl.BlockSpec((1, Bx), lambda i: (0, 0))]
    for w in widths:
        in_specs.append(pl.BlockSpec((Bx, w), lambda i: (0, 0)))
    out_specs = [pl.BlockSpec((R, w), lambda i: (i, 0)) for w in widths]
    out_shape = [jax.ShapeDtypeStruct((M, w), jnp.float32) for w in widths]
    outs = pl.pallas_call(
        body,
        grid=(n,),
        in_specs=in_specs,
        out_specs=out_specs,
        out_shape=out_shape,
    )(a, W2f, b2f, *posts)
    return outs


def _ewstats_body(t_ref, sc_ref, sh_ref, v_out, s_out, g_out, acc_s, acc_g):
    i = pl.program_id(0)

    @pl.when(i == 0)
    def _init():
        acc_s[...] = jnp.zeros_like(acc_s)
        acc_g[...] = jnp.zeros_like(acc_g)

    v = _silu(t_ref[...] * sc_ref[...] + sh_ref[...])
    v_out[...] = v
    acc_s[...] += jnp.sum(v, axis=0, keepdims=True)
    acc_g[...] += lax.dot_general(v, v, (((0,), (0,)), ((), ())),
                                  preferred_element_type=jnp.float32)

    @pl.when(i == pl.num_programs(0) - 1)
    def _fin():
        s_out[...] = acc_s[...]
        g_out[...] = acc_g[...]


def _ewstats_pass(t, scale, shift, R):
    M, Bx = t.shape
    n = M // R
    return pl.pallas_call(
        _ewstats_body,
        grid=(n,),
        in_specs=[pl.BlockSpec((R, Bx), lambda i: (i, 0)),
                  pl.BlockSpec((1, Bx), lambda i: (0, 0)),
                  pl.BlockSpec((1, Bx), lambda i: (0, 0))],
        out_specs=[pl.BlockSpec((R, Bx), lambda i: (i, 0)),
                   pl.BlockSpec((1, Bx), lambda i: (0, 0)),
                   pl.BlockSpec((Bx, Bx), lambda i: (0, 0))],
        out_shape=[jax.ShapeDtypeStruct((M, Bx), jnp.float32),
                   jax.ShapeDtypeStruct((1, Bx), jnp.float32),
                   jax.ShapeDtypeStruct((Bx, Bx), jnp.float32)],
        scratch_shapes=[pltpu.VMEM((1, Bx), jnp.float32),
                        pltpu.VMEM((Bx, Bx), jnp.float32)],
    )(t, scale, shift)


def _expand_body(v_ref, w_ref, b_ref, r_ref, o_ref):
    o_ref[...] = _silu(jnp.dot(v_ref[...], w_ref[...],
                               preferred_element_type=jnp.float32)
                       + b_ref[...]) + r_ref[...]


def _expand_pass(v, Wf, bf, resid, R):
    M, Bx = v.shape
    Dx = Wf.shape[1]
    n = M // R
    return pl.pallas_call(
        _expand_body,
        grid=(n,),
        in_specs=[pl.BlockSpec((R, Bx), lambda i: (i, 0)),
                  pl.BlockSpec((Bx, Dx), lambda i: (0, 0)),
                  pl.BlockSpec((1, Dx), lambda i: (0, 0)),
                  pl.BlockSpec((R, Dx), lambda i: (i, 0))],
        out_specs=pl.BlockSpec((R, Dx), lambda i: (i, 0)),
        out_shape=jax.ShapeDtypeStruct((M, Dx), jnp.float32),
    )(v, Wf, bf, resid)


def _tw_body(t_ref, sc_ref, sh_ref, w_ref, o_ref):
    m = _silu(t_ref[...] * sc_ref[...] + sh_ref[...])
    o_ref[...] = jnp.dot(m, w_ref[...], preferred_element_type=jnp.float32)


def _tw_pass(t, scale, shift, Weg, R):
    M, Bx = t.shape
    n = M // R
    return pl.pallas_call(
        _tw_body,
        grid=(n,),
        in_specs=[pl.BlockSpec((R, Bx), lambda i: (i, 0)),
                  pl.BlockSpec((1, Bx), lambda i: (0, 0)),
                  pl.BlockSpec((1, Bx), lambda i: (0, 0)),
                  pl.BlockSpec((Bx, Bx), lambda i: (0, 0))],
        out_specs=pl.BlockSpec((R, Bx), lambda i: (i, 0)),
        out_shape=jax.ShapeDtypeStruct((M, Bx), jnp.float32),
    )(t, scale, shift, Weg)


def _fold_linear_bn(S, G, M, W, b, g, be):
    mu_in = S / M
    mean = mu_in @ W + b
    ex2 = jnp.sum(W * (G @ W), axis=0) / M
    var = ex2 - jnp.square(mu_in @ W)[0]
    scale = g / jnp.sqrt(var + EPS_BN)
    Wf = W * scale[None, :]
    bf = (b - mean[0]) * scale + be
    return Wf, bf[None, :]


def _bn_scale_shift(s, ss, M, g, be):
    mean = s / M
    var = ss / M - jnp.square(mean)
    scale = g / jnp.sqrt(var + EPS_BN)
    shift = be - mean * scale
    return scale, shift


def _bottleneck_heads(x, q, R, posts):
    M = x.shape[0]
    S1, G1 = _gram_pass(x, R)
    W1f, b1f = _fold_linear_bn(S1, G1, M, q["W1"], q["b1"], q["g1"], q["be1"])
    a, S2, G2 = _mid_pass(x, W1f, b1f, R)
    W2f, b2f = _fold_linear_bn(S2, G2, M, q["W2"], q["b2"], q["g2"], q["be2"])
    return _apply_post_pass(a, W2f, b2f, posts, R)


def _sc_edge_gate(src_idx, dst_idx, src_tab, dst_tab, ew, CH=80):
    EL = src_idx.shape[0]
    per_w = EL // NW
    nch = per_w // CH
    mesh = plsc.VectorSubcoreMesh(core_axis_name="c", subcore_axis_name="s")

    @functools.partial(
        pl.kernel,
        out_type=[jax.ShapeDtypeStruct((EL, BT), jnp.float32),
                  jax.ShapeDtypeStruct((EL + 8, 2 * BT), jnp.float32),
                  jax.ShapeDtypeStruct((NW, 2, BT), jnp.float32)],
        mesh=mesh,
        compiler_params=pltpu.CompilerParams(use_tc_tiling_on_sc=False),
        scratch_types=(
            [pltpu.VMEM((CH,), jnp.int32)] * 4
            + [pltpu.VMEM((CH, 2 * BT), jnp.float32)] * 2
            + [pltpu.VMEM((CH, BT), jnp.float32)] * 4
            + [pltpu.VMEM((CH, BT), jnp.float32)] * 2
            + [pltpu.VMEM((CH, 2 * BT), jnp.float32)] * 2
            + [pltpu.VMEM((2, BT), jnp.float32)]
            + [pltpu.SemaphoreType.DMA] * 2
        ),
    )
    def k(src_hbm, dst_hbm, stab_hbm, dtab_hbm, ew_hbm,
          m_hbm, p_hbm, st_hbm,
          idxs0, idxs1, idxd0, idxd1, srows0, srows1,
          drows0, drows1, erows0, erows1, mbuf0, mbuf1, pbuf0, pbuf1, stat_v,
          sem0, sem1):
        wid = lax.axis_index("s") * NC + lax.axis_index("c")
        base_w = wid * per_w
        zero = jnp.zeros((L,), jnp.float32)
        for r in range(2):
            for cv in range(BT // L):
                stat_v[r, pl.ds(cv * L, L)] = zero

        idxs = (idxs0, idxs1)
        idxd = (idxd0, idxd1)
        srows = (srows0, srows1)
        drows = (drows0, drows1)
        erows = (erows0, erows1)
        mbuf = (mbuf0, mbuf1)
        pbuf = (pbuf0, pbuf1)
        sems = (sem0, sem1)

        def start(ci, b):
            base = base_w + ci * CH
            pltpu.sync_copy(src_hbm.at[pl.ds(base, CH)], idxs[b])
            pltpu.sync_copy(dst_hbm.at[pl.ds(base, CH)], idxd[b])
            pltpu.async_copy(stab_hbm.at[idxs[b]], srows[b], sems[b])
            pltpu.async_copy(dtab_hbm.at[idxd[b]], drows[b], sems[b])
            pltpu.async_copy(ew_hbm.at[pl.ds(base, CH)], erows[b], sems[b])

        def finish(ci, b):
            pltpu.make_async_copy(stab_hbm.at[idxs[b]], srows[b], sems[b]).wait()
            pltpu.make_async_copy(dtab_hbm.at[idxd[b]], drows[b], sems[b]).wait()
            base = base_w + ci * CH
            pltpu.make_async_copy(ew_hbm.at[pl.ds(base, CH)], erows[b], sems[b]).wait()

        def compute(ci, b):
            base = base_w + ci * CH

            def row_body(j, carry):
                accs = list(carry)
                for dj in range(2):
                    jj = j * 2 + dj
                    for h in range(BT // L):
                        es = srows[b][jj, pl.ds(L * h, L)]
                        bh = srows[b][jj, pl.ds(BT + L * h, L)]
                        ed = drows[b][jj, pl.ds(L * h, L)]
                        zw = erows[b][jj, pl.ds(L * h, L)]
                        m = es + ed + zw
                        sg = 1.0 / (1.0 + jnp.exp(-m))
                        v = sg * bh
                        mbuf[b][jj, pl.ds(L * h, L)] = m
                        pbuf[b][jj, pl.ds(L * h, L)] = v
                        pbuf[b][jj, pl.ds(BT + L * h, L)] = sg
                        accs[h] = accs[h] + m
                        accs[2 + h] = accs[2 + h] + m * m
                return tuple(accs)

            accs = lax.fori_loop(0, CH // 2, row_body, (zero, zero, zero, zero))
            for h in range(BT // L):
                stat_v[0, pl.ds(L * h, L)] += accs[h]
                stat_v[1, pl.ds(L * h, L)] += accs[2 + h]
            pltpu.sync_copy(mbuf[b], m_hbm.at[pl.ds(base, CH)])
            pltpu.sync_copy(pbuf[b], p_hbm.at[pl.ds(base, CH)])

        start(0, 0)

        def pair_body(i, _):
            for b in range(2):
                ci = i * 2 + b

                @pl.when(ci + 1 < nch)
                def _():
                    start(ci + 1, 1 - b)
                finish(ci, b)
                compute(ci, b)
            return 0

        lax.fori_loop(0, nch // 2, pair_body, 0)
        if nch % 2:
            finish(nch - 1, (nch - 1) % 2)
            compute(nch - 1, (nch - 1) % 2)
        pltpu.sync_copy(stat_v, st_hbm.at[wid])

        @pl.when(wid == 0)
        def _():
            for j in range(8):
                for h in range(2 * BT // L):
                    pbuf0[j, pl.ds(L * h, L)] = zero
            pltpu.sync_copy(pbuf0.at[pl.ds(0, 8)], p_hbm.at[pl.ds(EL, 8)])

    return k(src_idx, dst_idx, src_tab, dst_tab, ew)


def _sc_node_conv(src_idx, dst_idx, src_tab, dst_tab, ew, NPAD, CH=80):
    E = src_idx.shape[0]
    per_w = E // NW
    nch = per_w // CH
    stripe = NPAD // NS
    nsch = stripe // CH
    mesh = plsc.VectorSubcoreMesh(core_axis_name="c", subcore_axis_name="s")

    @functools.partial(
        pl.kernel,
        out_type=[jax.ShapeDtypeStruct((E, BT), jnp.float32),
                  jax.ShapeDtypeStruct((NC, NPAD, 2 * BT), jnp.float32),
                  jax.ShapeDtypeStruct((NW, 2, BT), jnp.float32)],
        mesh=mesh,
        compiler_params=pltpu.CompilerParams(use_tc_tiling_on_sc=False),
        scratch_types=(
            [pltpu.VMEM((CH,), jnp.int32)] * 4
            + [pltpu.VMEM((CH, 2 * BT), jnp.float32)] * 2
            + [pltpu.VMEM((CH, BT), jnp.float32)] * 4
            + [pltpu.VMEM((CH, BT), jnp.float32)] * 2
            + [pltpu.VMEM((CH, 2 * BT), jnp.float32)] * 2
            + [pltpu.VMEM((2, BT), jnp.float32)]
            + [pltpu.VMEM_SHARED((NPAD, 2 * BT), jnp.float32)]
            + [pltpu.SemaphoreType.DMA] * 2
        ),
    )
    def k(src_hbm, dst_hbm, stab_hbm, dtab_hbm, ew_hbm,
          m_hbm, dump_hbm, st_hbm,
          idxs0, idxs1, idxd0, idxd1, srows0, srows1,
          drows0, drows1, erows0, erows1, mbuf0, mbuf1, pbuf0, pbuf1, stat_v,
          acc_sh, sem0, sem1):
        cid = lax.axis_index("c")
        sid = lax.axis_index("s")
        wid = sid * NC + cid
        base_w = wid * per_w
        zero = jnp.zeros((L,), jnp.float32)

        idxs = (idxs0, idxs1)
        idxd = (idxd0, idxd1)
        srows = (srows0, srows1)
        drows = (drows0, drows1)
        erows = (erows0, erows1)
        mbuf = (mbuf0, mbuf1)
        pbuf = (pbuf0, pbuf1)
        sems = (sem0, sem1)

        def zrow(j, _):
            for h in range(2 * BT // L):
                pbuf0[j, pl.ds(L * h, L)] = zero
            return 0
        lax.fori_loop(0, CH, zrow, 0)

        def zchunk(c, _):
            pltpu.sync_copy(pbuf0, acc_sh.at[pl.ds(sid * stripe + c * CH, CH)])
            return 0
        lax.fori_loop(0, nsch, zchunk, 0)
        for r in range(2):
            for cv in range(BT // L):
                stat_v[r, pl.ds(cv * L, L)] = zero
        plsc.subcore_barrier()

        def start(ci, b):
            base = base_w + ci * CH
            pltpu.sync_copy(src_hbm.at[pl.ds(base, CH)], idxs[b])
            pltpu.sync_copy(dst_hbm.at[pl.ds(base, CH)], idxd[b])
            pltpu.async_copy(stab_hbm.at[idxs[b]], srows[b], sems[b])
            pltpu.async_copy(dtab_hbm.at[idxd[b]], drows[b], sems[b])
            pltpu.async_copy(ew_hbm.at[pl.ds(base, CH)], erows[b], sems[b])

        def finish(ci, b):
            pltpu.make_async_copy(stab_hbm.at[idxs[b]], srows[b], sems[b]).wait()
            pltpu.make_async_copy(dtab_hbm.at[idxd[b]], drows[b], sems[b]).wait()
            base = base_w + ci * CH
            pltpu.make_async_copy(ew_hbm.at[pl.ds(base, CH)], erows[b], sems[b]).wait()

        def compute(ci, b):
            base = base_w + ci * CH

            def row_body(j, carry):
                accs = list(carry)
                for dj in range(2):
                    jj = j * 2 + dj
                    for h in range(BT // L):
                        es = srows[b][jj, pl.ds(L * h, L)]
                        bh = srows[b][jj, pl.ds(BT + L * h, L)]
                        ed = drows[b][jj, pl.ds(L * h, L)]
                        zw = erows[b][jj, pl.ds(L * h, L)]
                        m = es + ed + zw
                        sg = 1.0 / (1.0 + jnp.exp(-m))
                        v = sg * bh
                        mbuf[b][jj, pl.ds(L * h, L)] = m
                        pbuf[b][jj, pl.ds(L * h, L)] = v
                        pbuf[b][jj, pl.ds(BT + L * h, L)] = sg
                        accs[h] = accs[h] + m
                        accs[2 + h] = accs[2 + h] + m * m
                return tuple(accs)

            accs = lax.fori_loop(0, CH // 2, row_body, (zero, zero, zero, zero))
            for h in range(BT // L):
                stat_v[0, pl.ds(L * h, L)] += accs[h]
                stat_v[1, pl.ds(L * h, L)] += accs[2 + h]
            pltpu.sync_copy(mbuf[b], m_hbm.at[pl.ds(base, CH)])
            pltpu.sync_copy(pbuf[b], acc_sh.at[idxd[b]], add=True)

        start(0, 0)

        def pair_body(i, _):
            for b in range(2):
                ci = i * 2 + b

                @pl.when(ci + 1 < nch)
                def _():
                    start(ci + 1, 1 - b)
                finish(ci, b)
                compute(ci, b)
            return 0

        lax.fori_loop(0, nch // 2, pair_body, 0)
        if nch % 2:
            finish(nch - 1, (nch - 1) % 2)
            compute(nch - 1, (nch - 1) % 2)
        pltpu.sync_copy(stat_v, st_hbm.at[wid])
        plsc.subcore_barrier()

        def dchunk(c, _):
            off = sid * stripe + c * CH
            pltpu.sync_copy(acc_sh.at[pl.ds(off, CH)], pbuf0)
            pltpu.sync_copy(pbuf0, dump_hbm.at[cid, pl.ds(off, CH)])
            return 0
        lax.fori_loop(0, nsch, dchunk, 0)

    return k(src_idx, dst_idx, src_tab, dst_tab, ew)


def _sc_seg_sum(dst_idx, pay, zrows, E, K=8, SEGPAD=20480):
    EL = dst_idx.shape[0]
    SEGC = E // (K * NC)
    assert SEGC <= SEGPAD
    per_s = EL // NS
    SCH = 2000
    nstage = per_s // SCH
    nvec = SCH // L
    FB = 128
    RING = 1024
    stripe = SEGPAD // NS
    nzch = stripe // 80
    nfch = SEGC // 80
    nfk = (nfch + NS - 1) // NS
    mesh = plsc.VectorSubcoreMesh(core_axis_name="c", subcore_axis_name="s")

    @functools.partial(
        pl.kernel,
        out_type=[jax.ShapeDtypeStruct((E, 2 * BT), jnp.float32)],
        mesh=mesh,
        compiler_params=pltpu.CompilerParams(use_tc_tiling_on_sc=False,
                                             needs_layout_passes=False),
        scratch_types=[
            pltpu.VMEM((SCH,), jnp.int32),
            pltpu.VMEM((SCH,), jnp.int32),
            pltpu.VMEM((RING // FB, FB), jnp.int32),
            pltpu.VMEM((RING // FB, FB), jnp.int32),
            pltpu.VMEM((FB, 2 * BT), jnp.float32),
            pltpu.VMEM((80, 2 * BT), jnp.float32),
            pltpu.VMEM((80, 2 * BT), jnp.float32),
            pltpu.VMEM_SHARED((SEGPAD, 2 * BT), jnp.float32),
            pltpu.SemaphoreType.DMA,
            pltpu.SemaphoreType.DMA,
            pltpu.SemaphoreType.DMA,
        ],
    )
    def k(dst_hbm, pay_hbm, zrows_hbm, d_hbm,
          dbuf0, dbuf1, cidx_v, cseg_v, prow_v, zbuf_v, bnc_v,
          acc_sh, semf, dsem0, dsem1):
        cid = lax.axis_index("c")
        sid = lax.axis_index("s")
        iota = lax.iota(jnp.int32, L)
        pltpu.sync_copy(zrows_hbm, zbuf_v)
        dbuf = (dbuf0, dbuf1)
        dsem = (dsem0, dsem1)

        def stage_start(sc, b):
            sbase = sid * per_s + sc * SCH
            pltpu.async_copy(dst_hbm.at[pl.ds(sbase, SCH)], dbuf[b], dsem[b])

        def stage_wait(sc, b):
            sbase = sid * per_s + sc * SCH
            pltpu.make_async_copy(dst_hbm.at[pl.ds(sbase, SCH)],
                                  dbuf[b], dsem[b]).wait()

        def flush_one(tail):
            b = lax.rem(lax.div(tail, FB), RING // FB)
            pltpu.async_copy(pay_hbm.at[cidx_v.at[b]], prow_v, semf).wait()
            pltpu.sync_copy(prow_v, acc_sh.at[cseg_v.at[b]], add=True)
            return tail + FB

        def pass_body(p, _):
            lo = (p * NC + cid) * SEGC

            def zchunk(c, _):
                pltpu.sync_copy(zbuf_v, acc_sh.at[pl.ds(sid * stripe + c * 80, 80)])
                return 0
            lax.fori_loop(0, nzch, zchunk, 0)
            plsc.subcore_barrier()

            def vec_loop(dref, sbase, pos, tail):
                def vec_body(it, carry):
                    pos, tail = carry
                    dv = dref[pl.ds(it * L, L)]
                    msk = (dv >= lo) & (dv < lo + SEGC)
                    m01f = jnp.where(msk, 1.0, 0.0)
                    incl = plsc.cumsum(m01f).astype(jnp.int32)
                    cnt = jnp.sum(m01f).astype(jnp.int32)
                    posv = pos + incl - 1
                    rp = jnp.bitwise_and(posv, RING - 1)
                    row = jnp.right_shift(rp, 7)
                    col = jnp.bitwise_and(rp, FB - 1)
                    ids = sbase + it * L + iota
                    plsc.store_scatter(cidx_v, [row, col], ids, mask=msk)
                    plsc.store_scatter(cseg_v, [row, col], dv - lo, mask=msk)
                    pos = pos + cnt
                    pos, tail = lax.while_loop(
                        lambda c: c[0] - c[1] >= FB,
                        lambda c: (c[0], flush_one(c[1])),
                        (pos, tail))
                    return pos, tail

                return lax.fori_loop(0, nvec, vec_body, (pos, tail))

            stage_start(0, 0)

            def stage_pair(i, carry):
                pos, tail = carry
                for b in range(2):
                    sc = i * 2 + b

                    @pl.when(sc + 1 < nstage)
                    def _():
                        stage_start(sc + 1, 1 - b)
                    stage_wait(sc, b)
                    sbase = sid * per_s + sc * SCH
                    pos, tail = vec_loop(dbuf[b], sbase, pos, tail)
                return pos, tail

            pos, tail = lax.fori_loop(0, nstage // 2, stage_pair,
                                      (jnp.int32(0), jnp.int32(0)))

            nfill = jnp.bitwise_and(-pos, FB - 1)
            for h in range(FB // L):
                fpos = pos + h * L + iota
                fmask = (fpos - pos) < nfill
                rp = jnp.bitwise_and(fpos, RING - 1)
                row = jnp.right_shift(rp, 7)
                col = jnp.bitwise_and(rp, FB - 1)
                plsc.store_scatter(cidx_v, [row, col],
                                   jnp.full((L,), EL, jnp.int32), mask=fmask)
                plsc.store_scatter(cseg_v, [row, col],
                                   jnp.zeros((L,), jnp.int32), mask=fmask)
            pos = pos + nfill
            pos, tail = lax.while_loop(
                lambda c: c[0] - c[1] >= FB,
                lambda c: (c[0], flush_one(c[1])),
                (pos, tail))
            plsc.subcore_barrier()

            def fin_body(kk, _):
                c = kk * NS + sid

                @pl.when(c < nfch)
                def _():
                    off = c * 80
                    pltpu.sync_copy(acc_sh.at[pl.ds(off, 80)], bnc_v)
                    pltpu.sync_copy(bnc_v, d_hbm.at[pl.ds(lo + off, 80)])
                return 0

            lax.fori_loop(0, nfk, fin_body, 0)
            plsc.subcore_barrier()
            return 0

        lax.fori_loop(0, K, pass_body, 0)

    return k(dst_idx, pay, zrows)


def _segfin_body(d_ref, u_ref, t_out, s_out, q_out, acc_s, acc_q):
    i = pl.program_id(0)

    @pl.when(i == 0)
    def _init():
        acc_s[...] = jnp.zeros_like(acc_s)
        acc_q[...] = jnp.zeros_like(acc_q)

    d = d_ref[...]
    t = u_ref[...] + d[:, :BT] / (d[:, BT:] + EPS_DIV)
    t_out[...] = t
    acc_s[...] += jnp.sum(t, axis=0, keepdims=True)
    acc_q[...] += jnp.sum(t * t, axis=0, keepdims=True)

    @pl.when(i == pl.num_programs(0) - 1)
    def _fin():
        s_out[...] = acc_s[...]
        q_out[...] = acc_q[...]


def _segfin_pass(d, u, R):
    M = u.shape[0]
    n = M // R
    return pl.pallas_call(
        _segfin_body,
        grid=(n,),
        in_specs=[pl.BlockSpec((R, 2 * BT), lambda i: (i, 0)),
                  pl.BlockSpec((R, BT), lambda i: (i, 0))],
        out_specs=[pl.BlockSpec((R, BT), lambda i: (i, 0)),
                   pl.BlockSpec((1, BT), lambda i: (0, 0)),
                   pl.BlockSpec((1, BT), lambda i: (0, 0))],
        out_shape=[jax.ShapeDtypeStruct((M, BT), jnp.float32),
                   jax.ShapeDtypeStruct((1, BT), jnp.float32),
                   jax.ShapeDtypeStruct((1, BT), jnp.float32)],
        scratch_shapes=[pltpu.VMEM((1, BT), jnp.float32),
                        pltpu.VMEM((1, BT), jnp.float32)],
    )(d, u)


def _combine_body(a0_ref, a1_ref, u_ref, t_ref):
    ah = a0_ref[:, :BT] + a1_ref[:, :BT]
    as_ = a0_ref[:, BT:] + a1_ref[:, BT:]
    t_ref[...] = u_ref[...] + ah / (as_ + EPS_DIV)


def _combine_pass(a0, a1, u, R):
    M = u.shape[0]
    n = M // R
    return pl.pallas_call(
        _combine_body,
        grid=(n,),
        in_specs=[pl.BlockSpec((R, 2 * BT), lambda i: (i, 0)),
                  pl.BlockSpec((R, 2 * BT), lambda i: (i, 0)),
                  pl.BlockSpec((R, BT), lambda i: (i, 0))],
        out_specs=pl.BlockSpec((R, BT), lambda i: (i, 0)),
        out_shape=jax.ShapeDtypeStruct((M, BT), jnp.float32),
    )(a0, a1, u)


def kernel(x, y, z, edge_index, lg_edge_index, params):
    N = x.shape[0]
    E = y.shape[0]
    EL = z.shape[0]
    R = 4000
    RN = 2000
    p = params

    qy = p["edge_upd"]
    src_tab_y, dst_tab_y, u_y = _bottleneck_heads(
        y, p["pair_bn"], R,
        [jnp.concatenate([qy["Wsg"], qy["Wdu"]], axis=1), qy["Wdg"], qy["Wsu"]])
    (zweg,) = _bottleneck_heads(z, p["trip_bn"], R, [qy["Weg"]])
    qx = p["node_upd"]
    src_tab_x, dst_tab_x, u_x = _bottleneck_heads(
        x, p["node_bn"], RN,
        [jnp.concatenate([qx["Wsg"], qx["Wdu"]], axis=1), qx["Wdg"], qx["Wsu"]])

    src_l, dst_l = lg_edge_index[0], lg_edge_index[1]
    m_l, pay_l, st_l = _sc_edge_gate(src_l, dst_l, src_tab_y, dst_tab_y, zweg)
    stls = jnp.sum(st_l, axis=0)
    s_ml, ssq_ml = stls[0], stls[1]

    zrows = jnp.zeros((80, 2 * BT), jnp.float32)
    (segdump,) = _sc_seg_sum(dst_l, pay_l, zrows, E)
    t, St, Qt = _segfin_pass(segdump, u_y, R)
    sc_t, sh_t = _bn_scale_shift(St[0], Qt[0], E, qy["gn"], qy["bn"])
    w = _tw_pass(t, sc_t[None, :], sh_t[None, :], qx["Weg"], R)

    sc_m, sh_m = _bn_scale_shift(s_ml, ssq_ml, EL, qy["ge"], qy["be"])
    z2, Sz2, Gz2 = _ewstats_pass(m_l, sc_m[None, :], sh_m[None, :], R)

    src_g, dst_g = edge_index[0], edge_index[1]
    NPAD = 10240
    m2, dump, st2 = _sc_node_conv(src_g, dst_g, src_tab_x, dst_tab_x, w, NPAD)
    st2s = jnp.sum(st2, axis=0)
    s_m2, ssq_m2 = st2s[0], st2s[1]

    t2 = _combine_pass(dump[0, :N], dump[1, :N], u_x, RN)
    St2, Gt2 = _gram_pass(t2, RN)
    sc_t2, sh_t2 = _bn_scale_shift(St2[0], jnp.diag(Gt2), N, qx["gn"], qx["bn"])
    x2, Sx2, Gx2 = _ewstats_pass(t2, sc_t2[None, :], sh_t2[None, :], RN)

    sc_m2, sh_m2 = _bn_scale_shift(s_m2, ssq_m2, E, qx["ge"], qx["be"])
    y2, Sy2, Gy2 = _ewstats_pass(m2, sc_m2[None, :], sh_m2[None, :], R)

    qe = p["node_ex"]
    Wxf, bxf = _fold_linear_bn(Sx2, Gx2, N, qe["W"], qe["b"], qe["g"], qe["be"])
    xo = _expand_pass(x2, Wxf, bxf, x, RN)
    qe = p["pair_ex"]
    Wyf, byf = _fold_linear_bn(Sy2, Gy2, E, qe["W"], qe["b"], qe["g"], qe["be"])
    yo = _expand_pass(y2, Wyf, byf, y, R)
    qe = p["trip_ex"]
    Wzf, bzf = _fold_linear_bn(Sz2, Gz2, EL, qe["W"], qe["b"], qe["g"], qe["be"])
    zo = _expand_pass(z2, Wzf, bzf, z, R)

    return xo, yo, zo


# speedup vs baseline: 7.3600x; 1.0549x over previous
import functools

import jax
import jax.numpy as jnp
from jax import lax
from jax.experimental import pallas as pl
from jax.experimental.pallas import tpu as pltpu
from jax.experimental.pallas import tpu_sc as plsc

D = 128
H = 64
BT = 32

NC = 2
NS = 16
L = 16
NW = NC * NS

EPS_BN = 1e-5
EPS_DIV = 1e-6


def _silu(v):
    return v * (1.0 / (1.0 + jnp.exp(-v)))


def _gram_body(x_ref, s_out, g_out, acc_s, acc_g):
    i = pl.program_id(0)

    @pl.when(i == 0)
    def _init():
        acc_s[...] = jnp.zeros_like(acc_s)
        acc_g[...] = jnp.zeros_like(acc_g)

    x = x_ref[...]
    acc_s[...] += jnp.sum(x, axis=0, keepdims=True)
    acc_g[...] += lax.dot_general(x, x, (((0,), (0,)), ((), ())),
                                  preferred_element_type=jnp.float32)

    @pl.when(i == pl.num_programs(0) - 1)
    def _fin():
        s_out[...] = acc_s[...]
        g_out[...] = acc_g[...]


def _gram_pass(x, R):
    M, Dx = x.shape
    n = M // R
    return pl.pallas_call(
        _gram_body,
        grid=(n,),
        in_specs=[pl.BlockSpec((R, Dx), lambda i: (i, 0))],
        out_specs=[pl.BlockSpec((1, Dx), lambda i: (0, 0)),
                   pl.BlockSpec((Dx, Dx), lambda i: (0, 0))],
        out_shape=[jax.ShapeDtypeStruct((1, Dx), jnp.float32),
                   jax.ShapeDtypeStruct((Dx, Dx), jnp.float32)],
        scratch_shapes=[pltpu.VMEM((1, Dx), jnp.float32),
                        pltpu.VMEM((Dx, Dx), jnp.float32)],
    )(x)


def _mid_body(x_ref, w_ref, b_ref, a_out, s_out, g_out, acc_s, acc_g):
    i = pl.program_id(0)

    @pl.when(i == 0)
    def _init():
        acc_s[...] = jnp.zeros_like(acc_s)
        acc_g[...] = jnp.zeros_like(acc_g)

    a = _silu(jnp.dot(x_ref[...], w_ref[...],
                      preferred_element_type=jnp.float32) + b_ref[...])
    a_out[...] = a
    acc_s[...] += jnp.sum(a, axis=0, keepdims=True)
    acc_g[...] += lax.dot_general(a, a, (((0,), (0,)), ((), ())),
                                  preferred_element_type=jnp.float32)

    @pl.when(i == pl.num_programs(0) - 1)
    def _fin():
        s_out[...] = acc_s[...]
        g_out[...] = acc_g[...]


def _mid_pass(x, Wf, bf, R):
    M, Dx = x.shape
    Hx = Wf.shape[1]
    n = M // R
    return pl.pallas_call(
        _mid_body,
        grid=(n,),
        in_specs=[pl.BlockSpec((R, Dx), lambda i: (i, 0)),
                  pl.BlockSpec((Dx, Hx), lambda i: (0, 0)),
                  pl.BlockSpec((1, Hx), lambda i: (0, 0))],
        out_specs=[pl.BlockSpec((R, Hx), lambda i: (i, 0)),
                   pl.BlockSpec((1, Hx), lambda i: (0, 0)),
                   pl.BlockSpec((Hx, Hx), lambda i: (0, 0))],
        out_shape=[jax.ShapeDtypeStruct((M, Hx), jnp.float32),
                   jax.ShapeDtypeStruct((1, Hx), jnp.float32),
                   jax.ShapeDtypeStruct((Hx, Hx), jnp.float32)],
        scratch_shapes=[pltpu.VMEM((1, Hx), jnp.float32),
                        pltpu.VMEM((Hx, Hx), jnp.float32)],
    )(x, Wf, bf)


def _apply_post_pass(a, W2f, b2f, posts, R):
    M, Hx = a.shape
    Bx = W2f.shape[1]
    n = M // R
    widths = [p.shape[1] for p in posts]
    npost = len(posts)

    def body(*refs):
        a_ref = refs[0]
        w_ref = refs[1]
        b_ref = refs[2]
        post_refs = refs[3:3 + npost]
        out_refs = refs[3 + npost:3 + 2 * npost]
        v = _silu(jnp.dot(a_ref[...], w_ref[...],
                          preferred_element_type=jnp.float32) + b_ref[...])
        for pr, orf in zip(post_refs, out_refs):
            orf[...] = jnp.dot(v, pr[...], preferred_element_type=jnp.float32)

    in_specs = [pl.BlockSpec((R, Hx), lambda i: (i, 0)),
                pl.BlockSpec((Hx, Bx), lambda i: (0, 0)),
                pl.BlockSpec((1, Bx), lambda i: (0, 0))]
    for w in widths:
        in_specs.append(pl.BlockSpec((Bx, w), lambda i: (0, 0)))
    out_specs = [pl.BlockSpec((R, w), lambda i: (i, 0)) for w in widths]
    out_shape = [jax.ShapeDtypeStruct((M, w), jnp.float32) for w in widths]
    outs = pl.pallas_call(
        body,
        grid=(n,),
        in_specs=in_specs,
        out_specs=out_specs,
        out_shape=out_shape,
    )(a, W2f, b2f, *posts)
    return outs


def _ewstats_body(t_ref, sc_ref, sh_ref, v_out, s_out, g_out, acc_s, acc_g):
    i = pl.program_id(0)

    @pl.when(i == 0)
    def _init():
        acc_s[...] = jnp.zeros_like(acc_s)
        acc_g[...] = jnp.zeros_like(acc_g)

    v = _silu(t_ref[...] * sc_ref[...] + sh_ref[...])
    v_out[...] = v
    acc_s[...] += jnp.sum(v, axis=0, keepdims=True)
    acc_g[...] += lax.dot_general(v, v, (((0,), (0,)), ((), ())),
                                  preferred_element_type=jnp.float32)

    @pl.when(i == pl.num_programs(0) - 1)
    def _fin():
        s_out[...] = acc_s[...]
        g_out[...] = acc_g[...]


def _ewstats_pass(t, scale, shift, R):
    M, Bx = t.shape
    n = M // R
    return pl.pallas_call(
        _ewstats_body,
        grid=(n,),
        in_specs=[pl.BlockSpec((R, Bx), lambda i: (i, 0)),
                  pl.BlockSpec((1, Bx), lambda i: (0, 0)),
                  pl.BlockSpec((1, Bx), lambda i: (0, 0))],
        out_specs=[pl.BlockSpec((R, Bx), lambda i: (i, 0)),
                   pl.BlockSpec((1, Bx), lambda i: (0, 0)),
                   pl.BlockSpec((Bx, Bx), lambda i: (0, 0))],
        out_shape=[jax.ShapeDtypeStruct((M, Bx), jnp.float32),
                   jax.ShapeDtypeStruct((1, Bx), jnp.float32),
                   jax.ShapeDtypeStruct((Bx, Bx), jnp.float32)],
        scratch_shapes=[pltpu.VMEM((1, Bx), jnp.float32),
                        pltpu.VMEM((Bx, Bx), jnp.float32)],
    )(t, scale, shift)


def _expand_body(v_ref, w_ref, b_ref, r_ref, o_ref):
    o_ref[...] = _silu(jnp.dot(v_ref[...], w_ref[...],
                               preferred_element_type=jnp.float32)
                       + b_ref[...]) + r_ref[...]


def _expand_pass(v, Wf, bf, resid, R):
    M, Bx = v.shape
    Dx = Wf.shape[1]
    n = M // R
    return pl.pallas_call(
        _expand_body,
        grid=(n,),
        in_specs=[pl.BlockSpec((R, Bx), lambda i: (i, 0)),
                  pl.BlockSpec((Bx, Dx), lambda i: (0, 0)),
                  pl.BlockSpec((1, Dx), lambda i: (0, 0)),
                  pl.BlockSpec((R, Dx), lambda i: (i, 0))],
        out_specs=pl.BlockSpec((R, Dx), lambda i: (i, 0)),
        out_shape=jax.ShapeDtypeStruct((M, Dx), jnp.float32),
    )(v, Wf, bf, resid)


def _tw_body(t_ref, sc_ref, sh_ref, w_ref, o_ref):
    m = _silu(t_ref[...] * sc_ref[...] + sh_ref[...])
    o_ref[...] = jnp.dot(m, w_ref[...], preferred_element_type=jnp.float32)


def _tw_pass(t, scale, shift, Weg, R):
    M, Bx = t.shape
    n = M // R
    return pl.pallas_call(
        _tw_body,
        grid=(n,),
        in_specs=[pl.BlockSpec((R, Bx), lambda i: (i, 0)),
                  pl.BlockSpec((1, Bx), lambda i: (0, 0)),
                  pl.BlockSpec((1, Bx), lambda i: (0, 0)),
                  pl.BlockSpec((Bx, Bx), lambda i: (0, 0))],
        out_specs=pl.BlockSpec((R, Bx), lambda i: (i, 0)),
        out_shape=jax.ShapeDtypeStruct((M, Bx), jnp.float32),
    )(t, scale, shift, Weg)


def _fold_linear_bn(S, G, M, W, b, g, be):
    mu_in = S / M
    mean = mu_in @ W + b
    ex2 = jnp.sum(W * (G @ W), axis=0) / M
    var = ex2 - jnp.square(mu_in @ W)[0]
    scale = g / jnp.sqrt(var + EPS_BN)
    Wf = W * scale[None, :]
    bf = (b - mean[0]) * scale + be
    return Wf, bf[None, :]


def _bn_scale_shift(s, ss, M, g, be):
    mean = s / M
    var = ss / M - jnp.square(mean)
    scale = g / jnp.sqrt(var + EPS_BN)
    shift = be - mean * scale
    return scale, shift


def _bottleneck_heads(x, q, R, posts):
    M = x.shape[0]
    S1, G1 = _gram_pass(x, R)
    W1f, b1f = _fold_linear_bn(S1, G1, M, q["W1"], q["b1"], q["g1"], q["be1"])
    a, S2, G2 = _mid_pass(x, W1f, b1f, R)
    W2f, b2f = _fold_linear_bn(S2, G2, M, q["W2"], q["b2"], q["g2"], q["be2"])
    return _apply_post_pass(a, W2f, b2f, posts, R)


def _sc_edge_gate(src_idx, dst_idx, src_tab, dst_tab, ew, CH=80):
    EL = src_idx.shape[0]
    per_w = EL // NW
    nch = per_w // CH
    mesh = plsc.VectorSubcoreMesh(core_axis_name="c", subcore_axis_name="s")

    @functools.partial(
        pl.kernel,
        out_type=[jax.ShapeDtypeStruct((EL, BT), jnp.float32),
                  jax.ShapeDtypeStruct((EL + 8, 2 * BT), jnp.float32),
                  jax.ShapeDtypeStruct((NW, 2, BT), jnp.float32)],
        mesh=mesh,
        compiler_params=pltpu.CompilerParams(use_tc_tiling_on_sc=False),
        scratch_types=(
            [pltpu.VMEM((CH,), jnp.int32)] * 4
            + [pltpu.VMEM((CH, 2 * BT), jnp.float32)] * 2
            + [pltpu.VMEM((CH, BT), jnp.float32)] * 4
            + [pltpu.VMEM((CH, BT), jnp.float32)] * 2
            + [pltpu.VMEM((CH, 2 * BT), jnp.float32)] * 2
            + [pltpu.VMEM((2, BT), jnp.float32)]
            + [pltpu.SemaphoreType.DMA] * 2
        ),
    )
    def k(src_hbm, dst_hbm, stab_hbm, dtab_hbm, ew_hbm,
          m_hbm, p_hbm, st_hbm,
          idxs0, idxs1, idxd0, idxd1, srows0, srows1,
          drows0, drows1, erows0, erows1, mbuf0, mbuf1, pbuf0, pbuf1, stat_v,
          sem0, sem1):
        wid = lax.axis_index("s") * NC + lax.axis_index("c")
        base_w = wid * per_w
        zero = jnp.zeros((L,), jnp.float32)
        for r in range(2):
            for cv in range(BT // L):
                stat_v[r, pl.ds(cv * L, L)] = zero

        idxs = (idxs0, idxs1)
        idxd = (idxd0, idxd1)
        srows = (srows0, srows1)
        drows = (drows0, drows1)
        erows = (erows0, erows1)
        mbuf = (mbuf0, mbuf1)
        pbuf = (pbuf0, pbuf1)
        sems = (sem0, sem1)

        def start(ci, b):
            base = base_w + ci * CH
            pltpu.sync_copy(src_hbm.at[pl.ds(base, CH)], idxs[b])
            pltpu.sync_copy(dst_hbm.at[pl.ds(base, CH)], idxd[b])
            pltpu.async_copy(stab_hbm.at[idxs[b]], srows[b], sems[b])
            pltpu.async_copy(dtab_hbm.at[idxd[b]], drows[b], sems[b])
            pltpu.async_copy(ew_hbm.at[pl.ds(base, CH)], erows[b], sems[b])

        def finish(ci, b):
            pltpu.make_async_copy(stab_hbm.at[idxs[b]], srows[b], sems[b]).wait()
            pltpu.make_async_copy(dtab_hbm.at[idxd[b]], drows[b], sems[b]).wait()
            base = base_w + ci * CH
            pltpu.make_async_copy(ew_hbm.at[pl.ds(base, CH)], erows[b], sems[b]).wait()

        def compute(ci, b):
            base = base_w + ci * CH

            def row_body(j, carry):
                accs = list(carry)
                for dj in range(2):
                    jj = j * 2 + dj
                    for h in range(BT // L):
                        es = srows[b][jj, pl.ds(L * h, L)]
                        bh = srows[b][jj, pl.ds(BT + L * h, L)]
                        ed = drows[b][jj, pl.ds(L * h, L)]
                        zw = erows[b][jj, pl.ds(L * h, L)]
                        m = es + ed + zw
                        sg = 1.0 / (1.0 + jnp.exp(-m))
                        v = sg * bh
                        mbuf[b][jj, pl.ds(L * h, L)] = m
                        pbuf[b][jj, pl.ds(L * h, L)] = v
                        pbuf[b][jj, pl.ds(BT + L * h, L)] = sg
                        accs[h] = accs[h] + m
                        accs[2 + h] = accs[2 + h] + m * m
                return tuple(accs)

            accs = lax.fori_loop(0, CH // 2, row_body, (zero, zero, zero, zero))
            for h in range(BT // L):
                stat_v[0, pl.ds(L * h, L)] += accs[h]
                stat_v[1, pl.ds(L * h, L)] += accs[2 + h]
            pltpu.sync_copy(mbuf[b], m_hbm.at[pl.ds(base, CH)])
            pltpu.sync_copy(pbuf[b], p_hbm.at[pl.ds(base, CH)])

        start(0, 0)

        def pair_body(i, _):
            for b in range(2):
                ci = i * 2 + b

                @pl.when(ci + 1 < nch)
                def _():
                    start(ci + 1, 1 - b)
                finish(ci, b)
                compute(ci, b)
            return 0

        lax.fori_loop(0, nch // 2, pair_body, 0)
        if nch % 2:
            finish(nch - 1, (nch - 1) % 2)
            compute(nch - 1, (nch - 1) % 2)
        pltpu.sync_copy(stat_v, st_hbm.at[wid])

        @pl.when(wid == 0)
        def _():
            for j in range(8):
                for h in range(2 * BT // L):
                    pbuf0[j, pl.ds(L * h, L)] = zero
            pltpu.sync_copy(pbuf0.at[pl.ds(0, 8)], p_hbm.at[pl.ds(EL, 8)])

    return k(src_idx, dst_idx, src_tab, dst_tab, ew)


def _sc_node_conv(src_idx, dst_idx, src_tab, dst_tab, ew, NPAD, CH=80):
    E = src_idx.shape[0]
    per_w = E // NW
    nch = per_w // CH
    stripe = NPAD // NS
    nsch = stripe // CH
    mesh = plsc.VectorSubcoreMesh(core_axis_name="c", subcore_axis_name="s")

    @functools.partial(
        pl.kernel,
        out_type=[jax.ShapeDtypeStruct((E, BT), jnp.float32),
                  jax.ShapeDtypeStruct((NC, NPAD, 2 * BT), jnp.float32),
                  jax.ShapeDtypeStruct((NW, 2, BT), jnp.float32)],
        mesh=mesh,
        compiler_params=pltpu.CompilerParams(use_tc_tiling_on_sc=False),
        scratch_types=(
            [pltpu.VMEM((CH,), jnp.int32)] * 4
            + [pltpu.VMEM((CH, 2 * BT), jnp.float32)] * 2
            + [pltpu.VMEM((CH, BT), jnp.float32)] * 4
            + [pltpu.VMEM((CH, BT), jnp.float32)] * 2
            + [pltpu.VMEM((CH, 2 * BT), jnp.float32)] * 2
            + [pltpu.VMEM((2, BT), jnp.float32)]
            + [pltpu.VMEM_SHARED((NPAD, 2 * BT), jnp.float32)]
            + [pltpu.SemaphoreType.DMA] * 2
        ),
    )
    def k(src_hbm, dst_hbm, stab_hbm, dtab_hbm, ew_hbm,
          m_hbm, dump_hbm, st_hbm,
          idxs0, idxs1, idxd0, idxd1, srows0, srows1,
          drows0, drows1, erows0, erows1, mbuf0, mbuf1, pbuf0, pbuf1, stat_v,
          acc_sh, sem0, sem1):
        cid = lax.axis_index("c")
        sid = lax.axis_index("s")
        wid = sid * NC + cid
        base_w = wid * per_w
        zero = jnp.zeros((L,), jnp.float32)

        idxs = (idxs0, idxs1)
        idxd = (idxd0, idxd1)
        srows = (srows0, srows1)
        drows = (drows0, drows1)
        erows = (erows0, erows1)
        mbuf = (mbuf0, mbuf1)
        pbuf = (pbuf0, pbuf1)
        sems = (sem0, sem1)

        def zrow(j, _):
            for h in range(2 * BT // L):
                pbuf0[j, pl.ds(L * h, L)] = zero
            return 0
        lax.fori_loop(0, CH, zrow, 0)

        def zchunk(c, _):
            pltpu.sync_copy(pbuf0, acc_sh.at[pl.ds(sid * stripe + c * CH, CH)])
            return 0
        lax.fori_loop(0, nsch, zchunk, 0)
        for r in range(2):
            for cv in range(BT // L):
                stat_v[r, pl.ds(cv * L, L)] = zero
        plsc.subcore_barrier()

        def start(ci, b):
            base = base_w + ci * CH
            pltpu.sync_copy(src_hbm.at[pl.ds(base, CH)], idxs[b])
            pltpu.sync_copy(dst_hbm.at[pl.ds(base, CH)], idxd[b])
            pltpu.async_copy(stab_hbm.at[idxs[b]], srows[b], sems[b])
            pltpu.async_copy(dtab_hbm.at[idxd[b]], drows[b], sems[b])
            pltpu.async_copy(ew_hbm.at[pl.ds(base, CH)], erows[b], sems[b])

        def finish(ci, b):
            pltpu.make_async_copy(stab_hbm.at[idxs[b]], srows[b], sems[b]).wait()
            pltpu.make_async_copy(dtab_hbm.at[idxd[b]], drows[b], sems[b]).wait()
            base = base_w + ci * CH
            pltpu.make_async_copy(ew_hbm.at[pl.ds(base, CH)], erows[b], sems[b]).wait()

        def compute(ci, b):
            base = base_w + ci * CH

            def row_body(j, carry):
                accs = list(carry)
                for dj in range(2):
                    jj = j * 2 + dj
                    for h in range(BT // L):
                        es = srows[b][jj, pl.ds(L * h, L)]
                        bh = srows[b][jj, pl.ds(BT + L * h, L)]
                        ed = drows[b][jj, pl.ds(L * h, L)]
                        zw = erows[b][jj, pl.ds(L * h, L)]
                        m = es + ed + zw
                        sg = 1.0 / (1.0 + jnp.exp(-m))
                        v = sg * bh
                        mbuf[b][jj, pl.ds(L * h, L)] = m
                        pbuf[b][jj, pl.ds(L * h, L)] = v
                        pbuf[b][jj, pl.ds(BT + L * h, L)] = sg
                        accs[h] = accs[h] + m
                        accs[2 + h] = accs[2 + h] + m * m
                return tuple(accs)

            accs = lax.fori_loop(0, CH // 2, row_body, (zero, zero, zero, zero))
            for h in range(BT // L):
                stat_v[0, pl.ds(L * h, L)] += accs[h]
                stat_v[1, pl.ds(L * h, L)] += accs[2 + h]
            pltpu.sync_copy(mbuf[b], m_hbm.at[pl.ds(base, CH)])
            pltpu.sync_copy(pbuf[b], acc_sh.at[idxd[b]], add=True)

        start(0, 0)

        def pair_body(i, _):
            for b in range(2):
                ci = i * 2 + b

                @pl.when(ci + 1 < nch)
                def _():
                    start(ci + 1, 1 - b)
                finish(ci, b)
                compute(ci, b)
            return 0

        lax.fori_loop(0, nch // 2, pair_body, 0)
        if nch % 2:
            finish(nch - 1, (nch - 1) % 2)
            compute(nch - 1, (nch - 1) % 2)
        pltpu.sync_copy(stat_v, st_hbm.at[wid])
        plsc.subcore_barrier()

        def dchunk(c, _):
            off = sid * stripe + c * CH
            pltpu.sync_copy(acc_sh.at[pl.ds(off, CH)], pbuf0)
            pltpu.sync_copy(pbuf0, dump_hbm.at[cid, pl.ds(off, CH)])
            return 0
        lax.fori_loop(0, nsch, dchunk, 0)

    return k(src_idx, dst_idx, src_tab, dst_tab, ew)


def _sc_seg_sum(dst_idx, pay, zrows, E, K=8, SEGPAD=20480):
    EL = dst_idx.shape[0]
    SEGC = E // (K * NC)
    assert SEGC <= SEGPAD
    per_s = EL // NS
    SCH = 2000
    nstage = per_s // SCH
    nvec = SCH // L
    FB = 128
    RING = 1024
    stripe = SEGPAD // NS
    nzch = stripe // 80
    nfch = SEGC // 80
    nfk = (nfch + NS - 1) // NS
    mesh = plsc.VectorSubcoreMesh(core_axis_name="c", subcore_axis_name="s")

    @functools.partial(
        pl.kernel,
        out_type=[jax.ShapeDtypeStruct((E, 2 * BT), jnp.float32)],
        mesh=mesh,
        compiler_params=pltpu.CompilerParams(use_tc_tiling_on_sc=False,
                                             needs_layout_passes=False),
        scratch_types=[
            pltpu.VMEM((SCH,), jnp.int32),
            pltpu.VMEM((SCH,), jnp.int32),
            pltpu.VMEM((RING // FB, FB), jnp.int32),
            pltpu.VMEM((RING // FB, FB), jnp.int32),
            pltpu.VMEM((FB, 2 * BT), jnp.float32),
            pltpu.VMEM((80, 2 * BT), jnp.float32),
            pltpu.VMEM((80, 2 * BT), jnp.float32),
            pltpu.VMEM_SHARED((SEGPAD, 2 * BT), jnp.float32),
            pltpu.SemaphoreType.DMA,
            pltpu.SemaphoreType.DMA,
            pltpu.SemaphoreType.DMA,
        ],
    )
    def k(dst_hbm, pay_hbm, zrows_hbm, d_hbm,
          dbuf0, dbuf1, cidx_v, cseg_v, prow_v, zbuf_v, bnc_v,
          acc_sh, semf, dsem0, dsem1):
        cid = lax.axis_index("c")
        sid = lax.axis_index("s")
        iota = lax.iota(jnp.int32, L)
        pltpu.sync_copy(zrows_hbm, zbuf_v)
        dbuf = (dbuf0, dbuf1)
        dsem = (dsem0, dsem1)

        def stage_start(sc, b):
            sbase = sid * per_s + sc * SCH
            pltpu.async_copy(dst_hbm.at[pl.ds(sbase, SCH)], dbuf[b], dsem[b])

        def stage_wait(sc, b):
            sbase = sid * per_s + sc * SCH
            pltpu.make_async_copy(dst_hbm.at[pl.ds(sbase, SCH)],
                                  dbuf[b], dsem[b]).wait()

        def flush_one(tail):
            b = lax.rem(lax.div(tail, FB), RING // FB)
            pltpu.async_copy(pay_hbm.at[cidx_v.at[b]], prow_v, semf).wait()
            pltpu.sync_copy(prow_v, acc_sh.at[cseg_v.at[b]], add=True)
            return tail + FB

        def pass_body(p, _):
            lo = (p * NC + cid) * SEGC

            def zchunk(c, _):
                pltpu.sync_copy(zbuf_v, acc_sh.at[pl.ds(sid * stripe + c * 80, 80)])
                return 0
            lax.fori_loop(0, nzch, zchunk, 0)
            plsc.subcore_barrier()

            def vec_loop(dref, sbase, pos, tail):
                def vec_body(it, carry):
                    pos, tail = carry
                    dv = dref[pl.ds(it * L, L)]
                    msk = (dv >= lo) & (dv < lo + SEGC)
                    m01f = jnp.where(msk, 1.0, 0.0)
                    incl = plsc.cumsum(m01f).astype(jnp.int32)
                    cnt = jnp.sum(m01f).astype(jnp.int32)
                    posv = pos + incl - 1
                    rp = jnp.bitwise_and(posv, RING - 1)
                    row = jnp.right_shift(rp, 7)
                    col = jnp.bitwise_and(rp, FB - 1)
                    ids = sbase + it * L + iota
                    plsc.store_scatter(cidx_v, [row, col], ids, mask=msk)
                    plsc.store_scatter(cseg_v, [row, col], dv - lo, mask=msk)
                    pos = pos + cnt
                    pos, tail = lax.while_loop(
                        lambda c: c[0] - c[1] >= FB,
                        lambda c: (c[0], flush_one(c[1])),
                        (pos, tail))
                    return pos, tail

                return lax.fori_loop(0, nvec, vec_body, (pos, tail))

            stage_start(0, 0)

            def stage_pair(i, carry):
                pos, tail = carry
                for b in range(2):
                    sc = i * 2 + b

                    @pl.when(sc + 1 < nstage)
                    def _():
                        stage_start(sc + 1, 1 - b)
                    stage_wait(sc, b)
                    sbase = sid * per_s + sc * SCH
                    pos, tail = vec_loop(dbuf[b], sbase, pos, tail)
                return pos, tail

            pos, tail = lax.fori_loop(0, nstage // 2, stage_pair,
                                      (jnp.int32(0), jnp.int32(0)))

            nfill = jnp.bitwise_and(-pos, FB - 1)
            for h in range(FB // L):
                fpos = pos + h * L + iota
                fmask = (fpos - pos) < nfill
                rp = jnp.bitwise_and(fpos, RING - 1)
                row = jnp.right_shift(rp, 7)
                col = jnp.bitwise_and(rp, FB - 1)
                plsc.store_scatter(cidx_v, [row, col],
                                   jnp.full((L,), EL, jnp.int32), mask=fmask)
                plsc.store_scatter(cseg_v, [row, col],
                                   jnp.zeros((L,), jnp.int32), mask=fmask)
            pos = pos + nfill
            pos, tail = lax.while_loop(
                lambda c: c[0] - c[1] >= FB,
                lambda c: (c[0], flush_one(c[1])),
                (pos, tail))
            plsc.subcore_barrier()

            def fin_body(kk, _):
                c = kk * NS + sid

                @pl.when(c < nfch)
                def _():
                    off = c * 80
                    pltpu.sync_copy(acc_sh.at[pl.ds(off, 80)], bnc_v)
                    pltpu.sync_copy(bnc_v, d_hbm.at[pl.ds(lo + off, 80)])
                return 0

            lax.fori_loop(0, nfk, fin_body, 0)
            plsc.subcore_barrier()
            return 0

        lax.fori_loop(0, K, pass_body, 0)

    return k(dst_idx, pay, zrows)


def _segfin_body(d_ref, u_ref, t_out, s_out, q_out, acc_s, acc_q):
    i = pl.program_id(0)

    @pl.when(i == 0)
    def _init():
        acc_s[...] = jnp.zeros_like(acc_s)
        acc_q[...] = jnp.zeros_like(acc_q)

    d = d_ref[...]
    t = u_ref[...] + d[:, :BT] / (d[:, BT:] + EPS_DIV)
    t_out[...] = t
    acc_s[...] += jnp.sum(t, axis=0, keepdims=True)
    acc_q[...] += jnp.sum(t * t, axis=0, keepdims=True)

    @pl.when(i == pl.num_programs(0) - 1)
    def _fin():
        s_out[...] = acc_s[...]
        q_out[...] = acc_q[...]


def _segfin_pass(d, u, R):
    M = u.shape[0]
    n = M // R
    return pl.pallas_call(
        _segfin_body,
        grid=(n,),
        in_specs=[pl.BlockSpec((R, 2 * BT), lambda i: (i, 0)),
                  pl.BlockSpec((R, BT), lambda i: (i, 0))],
        out_specs=[pl.BlockSpec((R, BT), lambda i: (i, 0)),
                   pl.BlockSpec((1, BT), lambda i: (0, 0)),
                   pl.BlockSpec((1, BT), lambda i: (0, 0))],
        out_shape=[jax.ShapeDtypeStruct((M, BT), jnp.float32),
                   jax.ShapeDtypeStruct((1, BT), jnp.float32),
                   jax.ShapeDtypeStruct((1, BT), jnp.float32)],
        scratch_shapes=[pltpu.VMEM((1, BT), jnp.float32),
                        pltpu.VMEM((1, BT), jnp.float32)],
    )(d, u)


def _combine_body(a0_ref, a1_ref, u_ref, t_ref):
    ah = a0_ref[:, :BT] + a1_ref[:, :BT]
    as_ = a0_ref[:, BT:] + a1_ref[:, BT:]
    t_ref[...] = u_ref[...] + ah / (as_ + EPS_DIV)


def _combine_pass(a0, a1, u, R):
    M = u.shape[0]
    n = M // R
    return pl.pallas_call(
        _combine_body,
        grid=(n,),
        in_specs=[pl.BlockSpec((R, 2 * BT), lambda i: (i, 0)),
                  pl.BlockSpec((R, 2 * BT), lambda i: (i, 0)),
                  pl.BlockSpec((R, BT), lambda i: (i, 0))],
        out_specs=pl.BlockSpec((R, BT), lambda i: (i, 0)),
        out_shape=jax.ShapeDtypeStruct((M, BT), jnp.float32),
    )(a0, a1, u)


def kernel(x, y, z, edge_index, lg_edge_index, params):
    N = x.shape[0]
    E = y.shape[0]
    EL = z.shape[0]
    R = 8000
    RN = 2000
    p = params

    qy = p["edge_upd"]
    src_tab_y, dst_tab_y, u_y = _bottleneck_heads(
        y, p["pair_bn"], R,
        [jnp.concatenate([qy["Wsg"], qy["Wdu"]], axis=1), qy["Wdg"], qy["Wsu"]])
    (zweg,) = _bottleneck_heads(z, p["trip_bn"], R, [qy["Weg"]])
    qx = p["node_upd"]
    src_tab_x, dst_tab_x, u_x = _bottleneck_heads(
        x, p["node_bn"], RN,
        [jnp.concatenate([qx["Wsg"], qx["Wdu"]], axis=1), qx["Wdg"], qx["Wsu"]])

    src_l, dst_l = lg_edge_index[0], lg_edge_index[1]
    m_l, pay_l, st_l = _sc_edge_gate(src_l, dst_l, src_tab_y, dst_tab_y, zweg)
    stls = jnp.sum(st_l, axis=0)
    s_ml, ssq_ml = stls[0], stls[1]

    zrows = jnp.zeros((80, 2 * BT), jnp.float32)
    (segdump,) = _sc_seg_sum(dst_l, pay_l, zrows, E)
    t, St, Qt = _segfin_pass(segdump, u_y, R)
    sc_t, sh_t = _bn_scale_shift(St[0], Qt[0], E, qy["gn"], qy["bn"])
    w = _tw_pass(t, sc_t[None, :], sh_t[None, :], qx["Weg"], R)

    sc_m, sh_m = _bn_scale_shift(s_ml, ssq_ml, EL, qy["ge"], qy["be"])
    z2, Sz2, Gz2 = _ewstats_pass(m_l, sc_m[None, :], sh_m[None, :], R)

    src_g, dst_g = edge_index[0], edge_index[1]
    NPAD = 10240
    m2, dump, st2 = _sc_node_conv(src_g, dst_g, src_tab_x, dst_tab_x, w, NPAD)
    st2s = jnp.sum(st2, axis=0)
    s_m2, ssq_m2 = st2s[0], st2s[1]

    t2 = _combine_pass(dump[0, :N], dump[1, :N], u_x, RN)
    St2, Gt2 = _gram_pass(t2, RN)
    sc_t2, sh_t2 = _bn_scale_shift(St2[0], jnp.diag(Gt2), N, qx["gn"], qx["bn"])
    x2, Sx2, Gx2 = _ewstats_pass(t2, sc_t2[None, :], sh_t2[None, :], RN)

    sc_m2, sh_m2 = _bn_scale_shift(s_m2, ssq_m2, E, qx["ge"], qx["be"])
    y2, Sy2, Gy2 = _ewstats_pass(m2, sc_m2[None, :], sh_m2[None, :], R)

    qe = p["node_ex"]
    Wxf, bxf = _fold_linear_bn(Sx2, Gx2, N, qe["W"], qe["b"], qe["g"], qe["be"])
    xo = _expand_pass(x2, Wxf, bxf, x, RN)
    qe = p["pair_ex"]
    Wyf, byf = _fold_linear_bn(Sy2, Gy2, E, qe["W"], qe["b"], qe["g"], qe["be"])
    yo = _expand_pass(y2, Wyf, byf, y, R)
    qe = p["trip_ex"]
    Wzf, bzf = _fold_linear_bn(Sz2, Gz2, EL, qe["W"], qe["b"], qe["g"], qe["be"])
    zo = _expand_pass(z2, Wzf, bzf, z, R)

    return xo, yo, zo
